# baseline (speedup 1.0000x reference)
import functools

import jax
import jax.numpy as jnp
from jax import lax
from jax.experimental import pallas as pl
from jax.experimental.pallas import tpu as pltpu
from jax.experimental.pallas import tpu_sc as plsc

N_NODES = 10000
N_EDGES = 160000
D_FEAT = 300
DH = 100
DP = 128
DP2 = 64
NC, NS = 2, 16
NW = NC * NS
EPW = N_EDGES // NW
CHUNK = 128
NFULL = EPW // CHUNK
TAIL = EPW - NFULL * CHUNK
OWN = 6000
TROWS = 6016
TSTRIPE = TROWS // NS
TRASH = TROWS - 1
EPT = N_EDGES // NS
NF2 = EPT // CHUNK
TAIL2 = EPT - NF2 * CHUNK
BE = 2000
BN = 2000
EPS = 1e-5

f32 = jnp.float32
bf16 = jnp.bfloat16


def _swish(x):
    return x * lax.logistic(x)


def _pad2(w, r, c):
    return jnp.zeros((r, c), f32).at[: w.shape[0], : w.shape[1]].set(w)


def _pad1(b, n):
    return jnp.zeros((1, n), f32).at[0, : b.shape[0]].set(b)


def _node_pre_body(x_ref, wa_ref, wb_ref, b_ref, a_ref, bb_ref):
    xb = x_ref[...]
    a = jnp.dot(xb, wa_ref[...], preferred_element_type=f32) + b_ref[...]
    bb = jnp.dot(xb[:, :DH], wb_ref[...], preferred_element_type=f32)
    a_ref[...] = a.astype(bf16)
    bb_ref[...] = bb.astype(bf16)


def _node_pre(x, wa, wb, b):
    g = N_NODES // BN
    return pl.pallas_call(
        _node_pre_body,
        grid=(g,),
        in_specs=[
            pl.BlockSpec((BN, D_FEAT), lambda i: (i, 0)),
            pl.BlockSpec((D_FEAT, DP), lambda i: (0, 0)),
            pl.BlockSpec((DH, DP), lambda i: (0, 0)),
            pl.BlockSpec((1, DP), lambda i: (0, 0)),
        ],
        out_specs=[
            pl.BlockSpec((BN, DP), lambda i: (i, 0)),
            pl.BlockSpec((BN, DP), lambda i: (i, 0)),
        ],
        out_shape=[
            jax.ShapeDtypeStruct((N_NODES, DP), bf16),
            jax.ShapeDtypeStruct((N_NODES, DP), bf16),
        ],
    )(x, wa, wb, b)


def _s1_body(ea_ref, eb_ref, sum_ref, sq_ref):
    i = pl.program_id(0)
    s = _swish(ea_ref[...].astype(f32) + eb_ref[...].astype(f32))
    ps = jnp.sum(s, axis=0, keepdims=True)
    pq = jnp.sum(s * s, axis=0, keepdims=True)

    @pl.when(i == 0)
    def _():
        sum_ref[...] = ps
        sq_ref[...] = pq

    @pl.when(i != 0)
    def _():
        sum_ref[...] = sum_ref[...] + ps
        sq_ref[...] = sq_ref[...] + pq


def _s1_stats(ea, eb):
    g = N_EDGES // BE
    return pl.pallas_call(
        _s1_body,
        grid=(g,),
        in_specs=[pl.BlockSpec((BE, DP), lambda i: (i, 0)),
                  pl.BlockSpec((BE, DP), lambda i: (i, 0))],
        out_specs=[
            pl.BlockSpec((1, DP), lambda i: (0, 0)),
            pl.BlockSpec((1, DP), lambda i: (0, 0)),
        ],
        out_shape=[
            jax.ShapeDtypeStruct((1, DP), f32),
            jax.ShapeDtypeStruct((1, DP), f32),
        ],
    )(ea, eb)


def _stage(xs, stats, gamma, beta, w, b, *, pre_swish, track_stats, ones_col=False):
    g = N_EDGES // BE
    one = pl.BlockSpec((1, DP), lambda i: (0, 0))
    n_in = len(xs)

    def body(*refs):
        in_refs = refs[:n_in]
        sum_ref, sq_ref, g_ref, be_ref, w_ref, b_ref = refs[n_in:n_in + 6]
        outs = refs[n_in + 6:]
        i = pl.program_id(0)
        h = in_refs[0][...].astype(f32)
        for r in in_refs[1:]:
            h = h + r[...].astype(f32)
        if pre_swish:
            h = _swish(h)
        m = sum_ref[...] * (1.0 / N_EDGES)
        var = sq_ref[...] * (1.0 / N_EDGES) - m * m
        h = (h - m) * lax.rsqrt(var + EPS) * g_ref[...] + be_ref[...]
        z = jnp.dot(h.astype(bf16), w_ref[...], preferred_element_type=f32) + b_ref[...]
        s = _swish(z)
        if ones_col:
            col = lax.broadcasted_iota(jnp.int32, s.shape, 1)
            s = jnp.where(col == DP - 1, 1.0, s)
        outs[0][...] = s if ones_col else s.astype(bf16)
        if track_stats:
            ps = jnp.sum(s, axis=0, keepdims=True)
            pq = jnp.sum(s * s, axis=0, keepdims=True)

            @pl.when(i == 0)
            def _():
                outs[1][...] = ps
                outs[2][...] = pq

            @pl.when(i != 0)
            def _():
                outs[1][...] = outs[1][...] + ps
                outs[2][...] = outs[2][...] + pq

    out_specs = [pl.BlockSpec((BE, DP), lambda i: (i, 0)), one, one]
    out_shape = [
        jax.ShapeDtypeStruct((N_EDGES, DP), f32 if ones_col else bf16),
        jax.ShapeDtypeStruct((1, DP), f32),
        jax.ShapeDtypeStruct((1, DP), f32),
    ]
    if not track_stats:
        out_specs, out_shape = out_specs[:1], out_shape[:1]
    return pl.pallas_call(
        body,
        grid=(g,),
        in_specs=[pl.BlockSpec((BE, DP), lambda i: (i, 0))] * n_in
                 + [one, one, one, one,
                    pl.BlockSpec((DP, DP), lambda i: (0, 0)), one],
        out_specs=out_specs,
        out_shape=out_shape,
    )(*xs, stats[0], stats[1], gamma, beta, w, b)


def _update_body(x_ref, v_ref, a_ref, wv_ref, wx_ref, wa_ref, b1_ref,
                 w2_ref, b2_ref, w3_ref, b3_ref, w4_ref, b4_ref, out_ref):
    xb = x_ref[...]
    acc = a_ref[0]
    cnt = jnp.maximum(acc[:, DP - 1 : DP], 1.0)
    agg = acc / cnt
    h = (jnp.dot(v_ref[...], wv_ref[...], preferred_element_type=f32)
         + jnp.dot(xb, wx_ref[...], preferred_element_type=f32)
         + jnp.dot(agg, wa_ref[...], preferred_element_type=f32)
         + b1_ref[...])
    h = _swish(h)
    h = _swish(jnp.dot(h, w2_ref[...], preferred_element_type=f32) + b2_ref[...])
    h = _swish(jnp.dot(h, w3_ref[...], preferred_element_type=f32) + b3_ref[...])
    upd = _swish(jnp.dot(h, w4_ref[...], preferred_element_type=f32) + b4_ref[...])
    out_ref[...] = xb + upd


def _update(x, v, aggs, wv, wx, wa, b1, w2, b2, w3, b3, w4, b4):
    g = N_NODES // BN
    oneh = pl.BlockSpec((1, DH), lambda i: (0, 0))
    return pl.pallas_call(
        _update_body,
        grid=(g,),
        in_specs=[
            pl.BlockSpec((BN, D_FEAT), lambda i: (i, 0)),
            pl.BlockSpec((BN, DH), lambda i: (i, 0)),
            pl.BlockSpec((1, BN, DP), lambda i: (i // 3, i - 3 * (i // 3), 0)),
            pl.BlockSpec((DH, DH), lambda i: (0, 0)),
            pl.BlockSpec((D_FEAT, DH), lambda i: (0, 0)),
            pl.BlockSpec((DP, DH), lambda i: (0, 0)),
            oneh,
            pl.BlockSpec((DH, DH), lambda i: (0, 0)), oneh,
            pl.BlockSpec((DH, DH), lambda i: (0, 0)), oneh,
            pl.BlockSpec((DH, D_FEAT), lambda i: (0, 0)),
            pl.BlockSpec((1, D_FEAT), lambda i: (0, 0)),
        ],
        out_specs=[pl.BlockSpec((BN, D_FEAT), lambda i: (i, 0))],
        out_shape=[jax.ShapeDtypeStruct((N_NODES, D_FEAT), f32)],
    )(x, v, aggs, wv, wx, wa, b1, w2, b2, w3, b3, w4, b4)[0]


def _pre_pool_body(x_ref, w1_ref, b1_ref, w2_ref, b2_ref, w3_ref, b3_ref,
                   w4_ref, b4_ref, sum_ref):
    i = pl.program_id(0)
    h = _swish(jnp.dot(x_ref[...], w1_ref[...], preferred_element_type=f32) + b1_ref[...])
    h = _swish(jnp.dot(h, w2_ref[...], preferred_element_type=f32) + b2_ref[...])
    h = _swish(jnp.dot(h, w3_ref[...], preferred_element_type=f32) + b3_ref[...])
    h = jnp.dot(h, w4_ref[...], preferred_element_type=f32) + b4_ref[...]
    ps = jnp.sum(h, axis=0, keepdims=True)

    @pl.when(i == 0)
    def _():
        sum_ref[...] = ps

    @pl.when(i != 0)
    def _():
        sum_ref[...] = sum_ref[...] + ps


def _pre_pool(x, w1, b1, w2, b2, w3, b3, w4, b4):
    g = N_NODES // BN
    oneh = pl.BlockSpec((1, DH), lambda i: (0, 0))
    return pl.pallas_call(
        _pre_pool_body,
        grid=(g,),
        in_specs=[
            pl.BlockSpec((BN, D_FEAT), lambda i: (i, 0)),
            pl.BlockSpec((D_FEAT, DH), lambda i: (0, 0)), oneh,
            pl.BlockSpec((DH, DH), lambda i: (0, 0)), oneh,
            pl.BlockSpec((DH, DH), lambda i: (0, 0)), oneh,
            pl.BlockSpec((DH, DH), lambda i: (0, 0)), oneh,
        ],
        out_specs=[oneh],
        out_shape=[jax.ShapeDtypeStruct((1, DH), f32)],
    )(x, w1, b1, w2, b2, w3, b3, w4, b4)[0]


def _final_body(hsum_ref, v0_ref, w1_ref, b1_ref, w2_ref, b2_ref, out_ref):
    pooled = hsum_ref[...] * (1.0 / N_NODES)
    c = _swish(jnp.dot(pooled, w1_ref[...], preferred_element_type=f32) + b1_ref[...])
    coeff = jnp.dot(c, w2_ref[...], preferred_element_type=f32) + b2_ref[...]
    out_ref[...] = v0_ref[...] * coeff


def _final(hsum, v0, w1, b1, w2, b2):
    return pl.pallas_call(
        _final_body,
        out_shape=jax.ShapeDtypeStruct((1, DH), f32),
    )(hsum, v0, w1, b1, w2, b2)


@functools.cache
def _mesh():
    return plsc.VectorSubcoreMesh(core_axis_name="c", subcore_axis_name="s",
                                  num_cores=NC, num_subcores=NS)


NPAIR = (NFULL - 1) // 2


def _sc_gather_body(a_hbm, b_hbm, dst_hbm, src_hbm, oa_hbm, ob_hbm,
                    idxd0, idxs0, idxd1, idxs1, ra0, rb0, ra1, rb1, sem_g, sem_w):
    cid = lax.axis_index("c")
    sid = lax.axis_index("s")
    wid = sid * NC + cid
    base = wid * EPW
    sets = ((idxd0, idxs0, ra0, rb0, 0), (idxd1, idxs1, ra1, rb1, 1))

    def start(ci, s):
        idxd, idxs, ra, rb, b = s
        off = base + ci * CHUNK
        pltpu.sync_copy(dst_hbm.at[pl.ds(off, CHUNK)], idxd)
        pltpu.sync_copy(src_hbm.at[pl.ds(off, CHUNK)], idxs)
        pltpu.async_copy(a_hbm.at[idxd], ra, sem_g.at[b, 0])
        pltpu.async_copy(b_hbm.at[idxs], rb, sem_g.at[b, 1])

    def wait_gather(s):
        idxd, idxs, ra, rb, b = s
        pltpu.make_async_copy(a_hbm.at[idxd], ra, sem_g.at[b, 0]).wait()
        pltpu.make_async_copy(b_hbm.at[idxs], rb, sem_g.at[b, 1]).wait()

    def wstart(ci, s):
        _, _, ra, rb, b = s
        off = pl.ds(base + ci * CHUNK, CHUNK)
        pltpu.async_copy(ra, oa_hbm.at[off], sem_w.at[b, 0])
        pltpu.async_copy(rb, ob_hbm.at[off], sem_w.at[b, 1])

    def wwait(ci, s):
        _, _, ra, rb, b = s
        off = pl.ds(base + ci * CHUNK, CHUNK)
        pltpu.make_async_copy(ra, oa_hbm.at[off], sem_w.at[b, 0]).wait()
        pltpu.make_async_copy(rb, ob_hbm.at[off], sem_w.at[b, 1]).wait()

    start(0, sets[0])
    start(1, sets[1])

    def body(p, carry):
        c0 = 2 * p
        wait_gather(sets[0])
        wstart(c0, sets[0])
        wait_gather(sets[1])
        wstart(c0 + 1, sets[1])

        @pl.when(p < NPAIR - 1)
        def _():
            wwait(c0, sets[0])
            start(c0 + 2, sets[0])
            wwait(c0 + 1, sets[1])
            start(c0 + 3, sets[1])

        return carry

    lax.fori_loop(0, NPAIR, body, 0)
    last0 = 2 * (NPAIR - 1)
    wwait(last0, sets[0])
    wwait(last0 + 1, sets[1])

    start(NFULL - 1, sets[0])
    wait_gather(sets[0])
    off = pl.ds(base + (NFULL - 1) * CHUNK, CHUNK)
    pltpu.sync_copy(ra0, oa_hbm.at[off])
    pltpu.sync_copy(rb0, ob_hbm.at[off])

    if TAIL:
        offt = base + NFULL * CHUNK
        id_d = idxd1.at[pl.ds(0, TAIL)]
        id_s = idxs1.at[pl.ds(0, TAIL)]
        pltpu.sync_copy(dst_hbm.at[pl.ds(offt, TAIL)], id_d)
        pltpu.sync_copy(src_hbm.at[pl.ds(offt, TAIL)], id_s)
        pltpu.async_copy(a_hbm.at[id_d], ra1.at[pl.ds(0, TAIL)], sem_g.at[1, 0]).wait()
        pltpu.async_copy(b_hbm.at[id_s], rb1.at[pl.ds(0, TAIL)], sem_g.at[1, 1]).wait()
        pltpu.sync_copy(ra1.at[pl.ds(0, TAIL)], oa_hbm.at[pl.ds(offt, TAIL)])
        pltpu.sync_copy(rb1.at[pl.ds(0, TAIL)], ob_hbm.at[pl.ds(offt, TAIL)])


def _gather_edges(a, b, dst, src):
    a32 = lax.bitcast_convert_type(a.reshape(N_NODES, DP2, 2), jnp.int32)
    b32 = lax.bitcast_convert_type(b.reshape(N_NODES, DP2, 2), jnp.int32)
    oa, ob = pl.kernel(
        _sc_gather_body,
        out_type=[
            jax.ShapeDtypeStruct((N_EDGES, DP2), jnp.int32),
            jax.ShapeDtypeStruct((N_EDGES, DP2), jnp.int32),
        ],
        mesh=_mesh(),
        compiler_params=pltpu.CompilerParams(use_tc_tiling_on_sc=False),
        scratch_types=[
            pltpu.VMEM((CHUNK,), jnp.int32),
            pltpu.VMEM((CHUNK,), jnp.int32),
            pltpu.VMEM((CHUNK,), jnp.int32),
            pltpu.VMEM((CHUNK,), jnp.int32),
            pltpu.VMEM((CHUNK, DP2), jnp.int32),
            pltpu.VMEM((CHUNK, DP2), jnp.int32),
            pltpu.VMEM((CHUNK, DP2), jnp.int32),
            pltpu.VMEM((CHUNK, DP2), jnp.int32),
            pltpu.SemaphoreType.DMA((2, 2)),
            pltpu.SemaphoreType.DMA((2, 2)),
        ],
    )(a32, b32, dst, src)
    ea = lax.bitcast_convert_type(oa, bf16).reshape(N_EDGES, DP)
    eb = lax.bitcast_convert_type(ob, bf16).reshape(N_EDGES, DP)
    return ea, eb


def _sc_scatter_body(msg_hbm, dst_hbm, out_hbm, idx, idx_t, rows, zbuf, shared,
                     sem_r, sem_s):
    cid = lax.axis_index("c")
    sid = lax.axis_index("s")
    base = sid * EPT
    nbase = cid * OWN

    zvec = jnp.zeros((16,), f32)

    def zrow(r, carry):
        for j in range(DP // 16):
            zbuf[r, pl.ds(j * 16, 16)] = zvec
        return carry

    lax.fori_loop(0, TSTRIPE, zrow, 0)
    pltpu.sync_copy(zbuf, shared.at[pl.ds(sid * TSTRIPE, TSTRIPE)])
    plsc.subcore_barrier()

    def remap(id_buf, b, k):
        for j in range(k // 16):
            sl = (b, pl.ds(j * 16, 16))
            local = id_buf[sl] - nbase
            ok = (local >= 0) & (local < OWN)
            id_buf[sl] = jnp.where(ok, local, TRASH)

    def start_chunk(ci, b):
        off = base + ci * CHUNK
        pltpu.sync_copy(dst_hbm.at[pl.ds(off, CHUNK)], idx.at[b])
        pltpu.async_copy(msg_hbm.at[pl.ds(off, CHUNK)], rows.at[b], sem_r.at[b])
        remap(idx, b, CHUNK)

    start_chunk(0, 0)

    def body(i, carry):
        b = lax.rem(i, 2)
        nb = 1 - b

        @pl.when(i + 1 < NF2)
        def _():
            @pl.when(i >= 1)
            def _():
                pltpu.make_async_copy(rows.at[nb], shared.at[idx.at[nb]],
                                      sem_s.at[nb]).wait()

            start_chunk(i + 1, nb)

        pltpu.make_async_copy(msg_hbm.at[pl.ds(0, CHUNK)], rows.at[b],
                              sem_r.at[b]).wait()
        pltpu.async_copy(rows.at[b], shared.at[idx.at[b]], sem_s.at[b], add=True)
        return carry

    lax.fori_loop(0, NF2, body, 0)
    pb = (NF2 - 1) % 2
    pltpu.make_async_copy(rows.at[pb], shared.at[idx.at[pb]], sem_s.at[pb]).wait()
    pltpu.make_async_copy(rows.at[1 - pb], shared.at[idx.at[1 - pb]],
                          sem_s.at[1 - pb]).wait()

    if TAIL2:
        off = base + NF2 * CHUNK
        pltpu.sync_copy(dst_hbm.at[pl.ds(off, TAIL2)], idx_t)
        pltpu.sync_copy(msg_hbm.at[pl.ds(off, TAIL2)], rows.at[0, pl.ds(0, TAIL2)])
        for j in range(TAIL2 // 16):
            sl = pl.ds(j * 16, 16)
            local = idx_t[sl] - nbase
            ok = (local >= 0) & (local < OWN)
            idx_t[sl] = jnp.where(ok, local, TRASH)
        pltpu.sync_copy(rows.at[0, pl.ds(0, TAIL2)], shared.at[idx_t], add=True)

    plsc.subcore_barrier()
    pltpu.sync_copy(shared.at[pl.ds(sid * TSTRIPE, TSTRIPE)],
                    out_hbm.at[cid, pl.ds(sid * TSTRIPE, TSTRIPE)])


def _scatter_msgs(msg, dst):
    return pl.kernel(
        _sc_scatter_body,
        out_type=jax.ShapeDtypeStruct((NC, TROWS, DP), f32),
        mesh=_mesh(),
        scratch_types=[
            pltpu.VMEM((2, CHUNK), jnp.int32),
            pltpu.VMEM((TAIL2,), jnp.int32),
            pltpu.VMEM((2, CHUNK, DP), f32),
            pltpu.VMEM((TSTRIPE, DP), f32),
            pltpu.VMEM_SHARED((TROWS, DP), f32),
            pltpu.SemaphoreType.DMA((2,)),
            pltpu.SemaphoreType.DMA((2,)),
        ],
    )(msg, dst)


def _msg_weights(mp):
    w1 = mp["l1"]["w"]
    wa = _pad2(w1[:, :D_FEAT].T, D_FEAT, DP)
    wb = _pad2(w1[:, D_FEAT:].T, DH, DP)
    b1 = _pad1(mp["l1"]["b"], DP)
    out = {"wa": wa, "wb": wb, "b1": b1}
    for k in ("2", "3"):
        out["w" + k] = _pad2(mp["l" + k]["w"].T, DP, DP).astype(bf16)
        out["b" + k] = _pad1(mp["l" + k]["b"], DP)
        out["g" + k] = _pad1(mp["bn" + k]["gamma"], DP)
        out["be" + k] = _pad1(mp["bn" + k]["beta"], DP)
    out["g1"] = _pad1(mp["bn1"]["gamma"], DP)
    out["be1"] = _pad1(mp["bn1"]["beta"], DP)
    out["w4"] = _pad2(mp["l4"]["w"].T, DP, DP).astype(bf16)
    out["b4"] = _pad1(mp["l4"]["b"], DP)
    return out


def _gnn_layer(x, v, dst, src, lp):
    mw = _msg_weights(lp["msg"])
    a, b = _node_pre(x, mw["wa"], mw["wb"], mw["b1"])
    ea, eb = _gather_edges(a, b, dst, src)
    st1 = _s1_stats(ea, eb)
    s2, st2s, st2q = _stage((ea, eb), st1, mw["g1"], mw["be1"], mw["w2"], mw["b2"],
                            pre_swish=True, track_stats=True)
    s3, st3s, st3q = _stage((s2,), (st2s, st2q), mw["g2"], mw["be2"], mw["w3"], mw["b3"],
                            pre_swish=False, track_stats=True)
    (msg,) = _stage((s3,), (st3s, st3q), mw["g3"], mw["be3"], mw["w4"], mw["b4"],
                    pre_swish=False, track_stats=False, ones_col=True)
    aggs = _scatter_msgs(msg, dst)

    up = lp["upd"]
    w1u = up["l1"]["w"]
    wv = w1u[:, :DH].T
    wx = w1u[:, DH:DH + D_FEAT].T
    wa = _pad2(w1u[:, DH + D_FEAT:].T, DP, DH)
    return _update(
        x, v, aggs,
        wv, wx, wa, _pad1(up["l1"]["b"], DH),
        up["l2"]["w"].T, _pad1(up["l2"]["b"], DH),
        up["l3"]["w"].T, _pad1(up["l3"]["b"], DH),
        up["l4"]["w"].T, _pad1(up["l4"]["b"], D_FEAT),
    )


def kernel(node_feature, edge_index, vectors, params):
    x0 = node_feature[0]
    src = edge_index[0, 0]
    dst = edge_index[0, 1]
    v = x0[:, :DH]
    stacked = jax.tree.map(lambda *a: jnp.stack(a), *params["layers"])

    def _layer_step(xc, lp):
        return _gnn_layer(xc, v, dst, src, lp), None

    x, _ = lax.scan(_layer_step, x0, stacked)

    pp = params["pre"]
    hsum = _pre_pool(
        x,
        pp["l1"]["w"].T, _pad1(pp["l1"]["b"], DH),
        pp["l2"]["w"].T, _pad1(pp["l2"]["b"], DH),
        pp["l3"]["w"].T, _pad1(pp["l3"]["b"], DH),
        pp["l4"]["w"].T, _pad1(pp["l4"]["b"], DH),
    )
    qp = params["post"]
    out = _final(hsum, x0[0:1, :DH],
                 qp["l1"]["w"].T, _pad1(qp["l1"]["b"], DH),
                 qp["l2"]["w"].T, _pad1(qp["l2"]["b"], DH))
    return out.reshape((DH,))


# speedup vs baseline: 2.1224x; 2.1224x over previous
import functools

import jax
import jax.numpy as jnp
from jax import lax
from jax.experimental import pallas as pl
from jax.experimental.pallas import tpu as pltpu
from jax.experimental.pallas import tpu_sc as plsc

N_NODES = 10000
N_EDGES = 160000
D_FEAT = 300
DH = 100
DP = 128
DP2 = 64
NC, NS = 2, 16
NW = NC * NS
EPW = N_EDGES // NW
CHUNK = 128
NFULL = EPW // CHUNK
TAIL = EPW - NFULL * CHUNK
OWN = 6000
TROWS = 6016
TSTRIPE = TROWS // NS
TRASH = TROWS - 1
EPT = N_EDGES // NS
NF2 = EPT // CHUNK
TAIL2 = EPT - NF2 * CHUNK
BE = 2000
BN = 2000
EPS = 1e-5

f32 = jnp.float32
bf16 = jnp.bfloat16


def _swish(x):
    return x * lax.logistic(x)


def _pad2(w, r, c):
    return jnp.zeros((r, c), f32).at[: w.shape[0], : w.shape[1]].set(w)


def _pad1(b, n):
    return jnp.zeros((1, n), f32).at[0, : b.shape[0]].set(b)


def _node_pre_body(x_ref, wa_ref, wb_ref, b_ref, a_ref, bb_ref):
    xb = x_ref[...]
    a_ref[...] = jnp.dot(xb, wa_ref[...], preferred_element_type=f32) + b_ref[...]
    bb_ref[...] = jnp.dot(xb[:, :DH], wb_ref[...], preferred_element_type=f32)


def _node_pre(x, wa, wb, b):
    g = N_NODES // BN
    return pl.pallas_call(
        _node_pre_body,
        grid=(g,),
        in_specs=[
            pl.BlockSpec((BN, D_FEAT), lambda i: (i, 0)),
            pl.BlockSpec((D_FEAT, DP), lambda i: (0, 0)),
            pl.BlockSpec((DH, DP), lambda i: (0, 0)),
            pl.BlockSpec((1, DP), lambda i: (0, 0)),
        ],
        out_specs=[
            pl.BlockSpec((BN, DP), lambda i: (i, 0)),
            pl.BlockSpec((BN, DP), lambda i: (i, 0)),
        ],
        out_shape=[
            jax.ShapeDtypeStruct((N_NODES, DP), f32),
            jax.ShapeDtypeStruct((N_NODES, DP), f32),
        ],
    )(x, wa, wb, b)


def _s1_body(ea_ref, eb_ref, sum_ref, sq_ref):
    i = pl.program_id(0)
    s = _swish(ea_ref[...].astype(f32) + eb_ref[...].astype(f32))
    ps = jnp.sum(s, axis=0, keepdims=True)
    pq = jnp.sum(s * s, axis=0, keepdims=True)

    @pl.when(i == 0)
    def _():
        sum_ref[...] = ps
        sq_ref[...] = pq

    @pl.when(i != 0)
    def _():
        sum_ref[...] = sum_ref[...] + ps
        sq_ref[...] = sq_ref[...] + pq


def _s1_stats(ea, eb):
    g = N_EDGES // BE
    return pl.pallas_call(
        _s1_body,
        grid=(g,),
        in_specs=[pl.BlockSpec((BE, DP), lambda i: (i, 0)),
                  pl.BlockSpec((BE, DP), lambda i: (i, 0))],
        out_specs=[
            pl.BlockSpec((1, DP), lambda i: (0, 0)),
            pl.BlockSpec((1, DP), lambda i: (0, 0)),
        ],
        out_shape=[
            jax.ShapeDtypeStruct((1, DP), f32),
            jax.ShapeDtypeStruct((1, DP), f32),
        ],
    )(ea, eb)


def _stage(xs, stats, gamma, beta, w, b, *, pre_swish, track_stats, ones_col=False):
    g = N_EDGES // BE
    one = pl.BlockSpec((1, DP), lambda i: (0, 0))
    n_in = len(xs)

    def body(*refs):
        in_refs = refs[:n_in]
        sum_ref, sq_ref, g_ref, be_ref, w_ref, b_ref = refs[n_in:n_in + 6]
        outs = refs[n_in + 6:]
        i = pl.program_id(0)
        h = in_refs[0][...].astype(f32)
        for r in in_refs[1:]:
            h = h + r[...].astype(f32)
        if pre_swish:
            h = _swish(h)
        m = sum_ref[...] * (1.0 / N_EDGES)
        var = sq_ref[...] * (1.0 / N_EDGES) - m * m
        h = (h - m) * lax.rsqrt(var + EPS) * g_ref[...] + be_ref[...]
        z = jnp.dot(h.astype(bf16), w_ref[...], preferred_element_type=f32) + b_ref[...]
        s = _swish(z)
        if ones_col:
            col = lax.broadcasted_iota(jnp.int32, s.shape, 1)
            s = jnp.where(col == DP - 1, 1.0, s)
        outs[0][...] = s if ones_col else s.astype(bf16)
        if track_stats:
            ps = jnp.sum(s, axis=0, keepdims=True)
            pq = jnp.sum(s * s, axis=0, keepdims=True)

            @pl.when(i == 0)
            def _():
                outs[1][...] = ps
                outs[2][...] = pq

            @pl.when(i != 0)
            def _():
                outs[1][...] = outs[1][...] + ps
                outs[2][...] = outs[2][...] + pq

    out_specs = [pl.BlockSpec((BE, DP), lambda i: (i, 0)), one, one]
    out_shape = [
        jax.ShapeDtypeStruct((N_EDGES, DP), f32 if ones_col else bf16),
        jax.ShapeDtypeStruct((1, DP), f32),
        jax.ShapeDtypeStruct((1, DP), f32),
    ]
    if not track_stats:
        out_specs, out_shape = out_specs[:1], out_shape[:1]
    return pl.pallas_call(
        body,
        grid=(g,),
        in_specs=[pl.BlockSpec((BE, DP), lambda i: (i, 0))] * n_in
                 + [one, one, one, one,
                    pl.BlockSpec((DP, DP), lambda i: (0, 0)), one],
        out_specs=out_specs,
        out_shape=out_shape,
    )(*xs, stats[0], stats[1], gamma, beta, w, b)


def _update_body(x_ref, v_ref, a_ref, wv_ref, wx_ref, wa_ref, b1_ref,
                 w2_ref, b2_ref, w3_ref, b3_ref, w4_ref, b4_ref, out_ref):
    xb = x_ref[...]
    acc = a_ref[0]
    cnt = jnp.maximum(acc[:, DP - 1 : DP], 1.0)
    agg = acc / cnt
    h = (jnp.dot(v_ref[...], wv_ref[...], preferred_element_type=f32)
         + jnp.dot(xb, wx_ref[...], preferred_element_type=f32)
         + jnp.dot(agg, wa_ref[...], preferred_element_type=f32)
         + b1_ref[...])
    h = _swish(h)
    h = _swish(jnp.dot(h, w2_ref[...], preferred_element_type=f32) + b2_ref[...])
    h = _swish(jnp.dot(h, w3_ref[...], preferred_element_type=f32) + b3_ref[...])
    upd = _swish(jnp.dot(h, w4_ref[...], preferred_element_type=f32) + b4_ref[...])
    out_ref[...] = xb + upd


def _update(x, v, aggs, wv, wx, wa, b1, w2, b2, w3, b3, w4, b4):
    g = N_NODES // BN
    oneh = pl.BlockSpec((1, DH), lambda i: (0, 0))
    return pl.pallas_call(
        _update_body,
        grid=(g,),
        in_specs=[
            pl.BlockSpec((BN, D_FEAT), lambda i: (i, 0)),
            pl.BlockSpec((BN, DH), lambda i: (i, 0)),
            pl.BlockSpec((1, BN, DP), lambda i: (i // 3, i - 3 * (i // 3), 0)),
            pl.BlockSpec((DH, DH), lambda i: (0, 0)),
            pl.BlockSpec((D_FEAT, DH), lambda i: (0, 0)),
            pl.BlockSpec((DP, DH), lambda i: (0, 0)),
            oneh,
            pl.BlockSpec((DH, DH), lambda i: (0, 0)), oneh,
            pl.BlockSpec((DH, DH), lambda i: (0, 0)), oneh,
            pl.BlockSpec((DH, D_FEAT), lambda i: (0, 0)),
            pl.BlockSpec((1, D_FEAT), lambda i: (0, 0)),
        ],
        out_specs=[pl.BlockSpec((BN, D_FEAT), lambda i: (i, 0))],
        out_shape=[jax.ShapeDtypeStruct((N_NODES, D_FEAT), f32)],
    )(x, v, aggs, wv, wx, wa, b1, w2, b2, w3, b3, w4, b4)[0]


def _pre_pool_body(x_ref, w1_ref, b1_ref, w2_ref, b2_ref, w3_ref, b3_ref,
                   w4_ref, b4_ref, sum_ref):
    i = pl.program_id(0)
    h = _swish(jnp.dot(x_ref[...], w1_ref[...], preferred_element_type=f32) + b1_ref[...])
    h = _swish(jnp.dot(h, w2_ref[...], preferred_element_type=f32) + b2_ref[...])
    h = _swish(jnp.dot(h, w3_ref[...], preferred_element_type=f32) + b3_ref[...])
    h = jnp.dot(h, w4_ref[...], preferred_element_type=f32) + b4_ref[...]
    ps = jnp.sum(h, axis=0, keepdims=True)

    @pl.when(i == 0)
    def _():
        sum_ref[...] = ps

    @pl.when(i != 0)
    def _():
        sum_ref[...] = sum_ref[...] + ps


def _pre_pool(x, w1, b1, w2, b2, w3, b3, w4, b4):
    g = N_NODES // BN
    oneh = pl.BlockSpec((1, DH), lambda i: (0, 0))
    return pl.pallas_call(
        _pre_pool_body,
        grid=(g,),
        in_specs=[
            pl.BlockSpec((BN, D_FEAT), lambda i: (i, 0)),
            pl.BlockSpec((D_FEAT, DH), lambda i: (0, 0)), oneh,
            pl.BlockSpec((DH, DH), lambda i: (0, 0)), oneh,
            pl.BlockSpec((DH, DH), lambda i: (0, 0)), oneh,
            pl.BlockSpec((DH, DH), lambda i: (0, 0)), oneh,
        ],
        out_specs=[oneh],
        out_shape=[jax.ShapeDtypeStruct((1, DH), f32)],
    )(x, w1, b1, w2, b2, w3, b3, w4, b4)[0]


def _final_body(hsum_ref, v0_ref, w1_ref, b1_ref, w2_ref, b2_ref, out_ref):
    pooled = hsum_ref[...] * (1.0 / N_NODES)
    c = _swish(jnp.dot(pooled, w1_ref[...], preferred_element_type=f32) + b1_ref[...])
    coeff = jnp.dot(c, w2_ref[...], preferred_element_type=f32) + b2_ref[...]
    out_ref[...] = v0_ref[...] * coeff


def _final(hsum, v0, w1, b1, w2, b2):
    return pl.pallas_call(
        _final_body,
        out_shape=jax.ShapeDtypeStruct((1, DH), f32),
    )(hsum, v0, w1, b1, w2, b2)


@functools.cache
def _mesh():
    return plsc.VectorSubcoreMesh(core_axis_name="c", subcore_axis_name="s",
                                  num_cores=NC, num_subcores=NS)


NPAIR = (NFULL - 1) // 2


def _sc_gather_body(a_hbm, b_hbm, dst_hbm, src_hbm, oa_hbm, ob_hbm,
                    idxd0, idxs0, idxd1, idxs1, ra0, rb0, ra1, rb1, sem_g, sem_w):
    cid = lax.axis_index("c")
    sid = lax.axis_index("s")
    wid = sid * NC + cid
    base = wid * EPW
    sets = ((idxd0, idxs0, ra0, rb0, 0), (idxd1, idxs1, ra1, rb1, 1))

    def start(ci, s):
        idxd, idxs, ra, rb, b = s
        off = base + ci * CHUNK
        pltpu.sync_copy(dst_hbm.at[pl.ds(off, CHUNK)], idxd)
        pltpu.sync_copy(src_hbm.at[pl.ds(off, CHUNK)], idxs)
        pltpu.async_copy(a_hbm.at[idxd], ra, sem_g.at[b, 0])
        pltpu.async_copy(b_hbm.at[idxs], rb, sem_g.at[b, 1])

    def wait_gather(s):
        idxd, idxs, ra, rb, b = s
        pltpu.make_async_copy(a_hbm.at[idxd], ra, sem_g.at[b, 0]).wait()
        pltpu.make_async_copy(b_hbm.at[idxs], rb, sem_g.at[b, 1]).wait()

    def wstart(ci, s):
        _, _, ra, rb, b = s
        off = pl.ds(base + ci * CHUNK, CHUNK)
        pltpu.async_copy(ra, oa_hbm.at[off], sem_w.at[b, 0])
        pltpu.async_copy(rb, ob_hbm.at[off], sem_w.at[b, 1])

    def wwait(ci, s):
        _, _, ra, rb, b = s
        off = pl.ds(base + ci * CHUNK, CHUNK)
        pltpu.make_async_copy(ra, oa_hbm.at[off], sem_w.at[b, 0]).wait()
        pltpu.make_async_copy(rb, ob_hbm.at[off], sem_w.at[b, 1]).wait()

    start(0, sets[0])
    start(1, sets[1])

    def body(p, carry):
        c0 = 2 * p
        wait_gather(sets[0])
        wstart(c0, sets[0])
        wait_gather(sets[1])
        wstart(c0 + 1, sets[1])

        @pl.when(p < NPAIR - 1)
        def _():
            wwait(c0, sets[0])
            start(c0 + 2, sets[0])
            wwait(c0 + 1, sets[1])
            start(c0 + 3, sets[1])

        return carry

    lax.fori_loop(0, NPAIR, body, 0)
    last0 = 2 * (NPAIR - 1)
    wwait(last0, sets[0])
    wwait(last0 + 1, sets[1])

    start(NFULL - 1, sets[0])
    wait_gather(sets[0])
    off = pl.ds(base + (NFULL - 1) * CHUNK, CHUNK)
    pltpu.sync_copy(ra0, oa_hbm.at[off])
    pltpu.sync_copy(rb0, ob_hbm.at[off])

    if TAIL:
        offt = base + NFULL * CHUNK
        id_d = idxd1.at[pl.ds(0, TAIL)]
        id_s = idxs1.at[pl.ds(0, TAIL)]
        pltpu.sync_copy(dst_hbm.at[pl.ds(offt, TAIL)], id_d)
        pltpu.sync_copy(src_hbm.at[pl.ds(offt, TAIL)], id_s)
        pltpu.async_copy(a_hbm.at[id_d], ra1.at[pl.ds(0, TAIL)], sem_g.at[1, 0]).wait()
        pltpu.async_copy(b_hbm.at[id_s], rb1.at[pl.ds(0, TAIL)], sem_g.at[1, 1]).wait()
        pltpu.sync_copy(ra1.at[pl.ds(0, TAIL)], oa_hbm.at[pl.ds(offt, TAIL)])
        pltpu.sync_copy(rb1.at[pl.ds(0, TAIL)], ob_hbm.at[pl.ds(offt, TAIL)])


def _gather_edges(a, b, dst, src):
    return pl.kernel(
        _sc_gather_body,
        out_type=[
            jax.ShapeDtypeStruct((N_EDGES, DP), f32),
            jax.ShapeDtypeStruct((N_EDGES, DP), f32),
        ],
        mesh=_mesh(),
        scratch_types=[
            pltpu.VMEM((CHUNK,), jnp.int32),
            pltpu.VMEM((CHUNK,), jnp.int32),
            pltpu.VMEM((CHUNK,), jnp.int32),
            pltpu.VMEM((CHUNK,), jnp.int32),
            pltpu.VMEM((CHUNK, DP), f32),
            pltpu.VMEM((CHUNK, DP), f32),
            pltpu.VMEM((CHUNK, DP), f32),
            pltpu.VMEM((CHUNK, DP), f32),
            pltpu.SemaphoreType.DMA((2, 2)),
            pltpu.SemaphoreType.DMA((2, 2)),
        ],
    )(a, b, dst, src)


def _sc_scatter_body(msg_hbm, dst_hbm, out_hbm, idx, idx_t, rows, zbuf, shared,
                     sem_r, sem_s):
    cid = lax.axis_index("c")
    sid = lax.axis_index("s")
    base = sid * EPT
    nbase = cid * OWN

    zvec = jnp.zeros((16,), f32)

    def zrow(r, carry):
        for j in range(DP // 16):
            zbuf[r, pl.ds(j * 16, 16)] = zvec
        return carry

    lax.fori_loop(0, TSTRIPE, zrow, 0)
    pltpu.sync_copy(zbuf, shared.at[pl.ds(sid * TSTRIPE, TSTRIPE)])
    plsc.subcore_barrier()

    def remap(id_buf, b, k):
        for j in range(k // 16):
            sl = (b, pl.ds(j * 16, 16))
            local = id_buf[sl] - nbase
            ok = (local >= 0) & (local < OWN)
            id_buf[sl] = jnp.where(ok, local, TRASH)

    def start_chunk(ci, b):
        off = base + ci * CHUNK
        pltpu.sync_copy(dst_hbm.at[pl.ds(off, CHUNK)], idx.at[b])
        pltpu.async_copy(msg_hbm.at[pl.ds(off, CHUNK)], rows.at[b], sem_r.at[b])
        remap(idx, b, CHUNK)

    start_chunk(0, 0)

    def body(i, carry):
        b = lax.rem(i, 2)
        nb = 1 - b

        @pl.when(i + 1 < NF2)
        def _():
            @pl.when(i >= 1)
            def _():
                pltpu.make_async_copy(rows.at[nb], shared.at[idx.at[nb]],
                                      sem_s.at[nb]).wait()

            start_chunk(i + 1, nb)

        pltpu.make_async_copy(msg_hbm.at[pl.ds(0, CHUNK)], rows.at[b],
                              sem_r.at[b]).wait()
        pltpu.async_copy(rows.at[b], shared.at[idx.at[b]], sem_s.at[b], add=True)
        return carry

    lax.fori_loop(0, NF2, body, 0)
    pb = (NF2 - 1) % 2
    pltpu.make_async_copy(rows.at[pb], shared.at[idx.at[pb]], sem_s.at[pb]).wait()
    pltpu.make_async_copy(rows.at[1 - pb], shared.at[idx.at[1 - pb]],
                          sem_s.at[1 - pb]).wait()

    if TAIL2:
        off = base + NF2 * CHUNK
        pltpu.sync_copy(dst_hbm.at[pl.ds(off, TAIL2)], idx_t)
        pltpu.sync_copy(msg_hbm.at[pl.ds(off, TAIL2)], rows.at[0, pl.ds(0, TAIL2)])
        for j in range(TAIL2 // 16):
            sl = pl.ds(j * 16, 16)
            local = idx_t[sl] - nbase
            ok = (local >= 0) & (local < OWN)
            idx_t[sl] = jnp.where(ok, local, TRASH)
        pltpu.sync_copy(rows.at[0, pl.ds(0, TAIL2)], shared.at[idx_t], add=True)

    plsc.subcore_barrier()
    pltpu.sync_copy(shared.at[pl.ds(sid * TSTRIPE, TSTRIPE)],
                    out_hbm.at[cid, pl.ds(sid * TSTRIPE, TSTRIPE)])


def _scatter_msgs(msg, dst):
    return pl.kernel(
        _sc_scatter_body,
        out_type=jax.ShapeDtypeStruct((NC, TROWS, DP), f32),
        mesh=_mesh(),
        scratch_types=[
            pltpu.VMEM((2, CHUNK), jnp.int32),
            pltpu.VMEM((TAIL2,), jnp.int32),
            pltpu.VMEM((2, CHUNK, DP), f32),
            pltpu.VMEM((TSTRIPE, DP), f32),
            pltpu.VMEM_SHARED((TROWS, DP), f32),
            pltpu.SemaphoreType.DMA((2,)),
            pltpu.SemaphoreType.DMA((2,)),
        ],
    )(msg, dst)


def _msg_weights(mp):
    w1 = mp["l1"]["w"]
    wa = _pad2(w1[:, :D_FEAT].T, D_FEAT, DP)
    wb = _pad2(w1[:, D_FEAT:].T, DH, DP)
    b1 = _pad1(mp["l1"]["b"], DP)
    out = {"wa": wa, "wb": wb, "b1": b1}
    for k in ("2", "3"):
        out["w" + k] = _pad2(mp["l" + k]["w"].T, DP, DP).astype(bf16)
        out["b" + k] = _pad1(mp["l" + k]["b"], DP)
        out["g" + k] = _pad1(mp["bn" + k]["gamma"], DP)
        out["be" + k] = _pad1(mp["bn" + k]["beta"], DP)
    out["g1"] = _pad1(mp["bn1"]["gamma"], DP)
    out["be1"] = _pad1(mp["bn1"]["beta"], DP)
    out["w4"] = _pad2(mp["l4"]["w"].T, DP, DP).astype(bf16)
    out["b4"] = _pad1(mp["l4"]["b"], DP)
    return out


def _gnn_layer(x, v, dst, src, lp):
    mw = _msg_weights(lp["msg"])
    a, b = _node_pre(x, mw["wa"], mw["wb"], mw["b1"])
    ea, eb = _gather_edges(a, b, dst, src)
    st1 = _s1_stats(ea, eb)
    s2, st2s, st2q = _stage((ea, eb), st1, mw["g1"], mw["be1"], mw["w2"], mw["b2"],
                            pre_swish=True, track_stats=True)
    s3, st3s, st3q = _stage((s2,), (st2s, st2q), mw["g2"], mw["be2"], mw["w3"], mw["b3"],
                            pre_swish=False, track_stats=True)
    (msg,) = _stage((s3,), (st3s, st3q), mw["g3"], mw["be3"], mw["w4"], mw["b4"],
                    pre_swish=False, track_stats=False, ones_col=True)
    aggs = _scatter_msgs(msg, dst)

    up = lp["upd"]
    w1u = up["l1"]["w"]
    wv = w1u[:, :DH].T
    wx = w1u[:, DH:DH + D_FEAT].T
    wa = _pad2(w1u[:, DH + D_FEAT:].T, DP, DH)
    return _update(
        x, v, aggs,
        wv, wx, wa, _pad1(up["l1"]["b"], DH),
        up["l2"]["w"].T, _pad1(up["l2"]["b"], DH),
        up["l3"]["w"].T, _pad1(up["l3"]["b"], DH),
        up["l4"]["w"].T, _pad1(up["l4"]["b"], D_FEAT),
    )


def kernel(node_feature, edge_index, vectors, params):
    x0 = node_feature[0]
    src = edge_index[0, 0]
    dst = edge_index[0, 1]
    v = x0[:, :DH]
    stacked = jax.tree.map(lambda *a: jnp.stack(a), *params["layers"])

    def _layer_step(xc, lp):
        return _gnn_layer(xc, v, dst, src, lp), None

    x, _ = lax.scan(_layer_step, x0, stacked)

    pp = params["pre"]
    hsum = _pre_pool(
        x,
        pp["l1"]["w"].T, _pad1(pp["l1"]["b"], DH),
        pp["l2"]["w"].T, _pad1(pp["l2"]["b"], DH),
        pp["l3"]["w"].T, _pad1(pp["l3"]["b"], DH),
        pp["l4"]["w"].T, _pad1(pp["l4"]["b"], DH),
    )
    qp = params["post"]
    out = _final(hsum, x0[0:1, :DH],
                 qp["l1"]["w"].T, _pad1(qp["l1"]["b"], DH),
                 qp["l2"]["w"].T, _pad1(qp["l2"]["b"], DH))
    return out.reshape((DH,))


# speedup vs baseline: 2.3506x; 1.1075x over previous
import functools

import jax
import jax.numpy as jnp
from jax import lax
from jax.experimental import pallas as pl
from jax.experimental.pallas import tpu as pltpu
from jax.experimental.pallas import tpu_sc as plsc

N_NODES = 10000
N_EDGES = 160000
D_FEAT = 300
DH = 100
DP = 128
DP2 = 64
NC, NS = 2, 16
NW = NC * NS
EPW = N_EDGES // NW
CHUNK = 128
NFULL = EPW // CHUNK
TAIL = EPW - NFULL * CHUNK
OWN = 6000
TROWS = 6016
TSTRIPE = TROWS // NS
TRASH = TROWS - 1
EPT = N_EDGES // NS
NF2 = EPT // CHUNK
TAIL2 = EPT - NF2 * CHUNK
BE = 2000
BN = 2000
EPS = 1e-5

f32 = jnp.float32
bf16 = jnp.bfloat16


def _swish(x):
    return x * lax.logistic(x)


def _pad2(w, r, c):
    return jnp.zeros((r, c), f32).at[: w.shape[0], : w.shape[1]].set(w)


def _pad1(b, n):
    return jnp.zeros((1, n), f32).at[0, : b.shape[0]].set(b)


def _node_pre_body(x_ref, wa_ref, wb_ref, b_ref, a_ref, bb_ref):
    xb = x_ref[...]
    a_ref[...] = jnp.dot(xb, wa_ref[...], preferred_element_type=f32) + b_ref[...]
    bb_ref[...] = jnp.dot(xb[:, :DH], wb_ref[...], preferred_element_type=f32)


def _node_pre(x, wa, wb, b):
    g = N_NODES // BN
    return pl.pallas_call(
        _node_pre_body,
        grid=(g,),
        in_specs=[
            pl.BlockSpec((BN, D_FEAT), lambda i: (i, 0)),
            pl.BlockSpec((D_FEAT, DP), lambda i: (0, 0)),
            pl.BlockSpec((DH, DP), lambda i: (0, 0)),
            pl.BlockSpec((1, DP), lambda i: (0, 0)),
        ],
        out_specs=[
            pl.BlockSpec((BN, DP), lambda i: (i, 0)),
            pl.BlockSpec((BN, DP), lambda i: (i, 0)),
        ],
        out_shape=[
            jax.ShapeDtypeStruct((N_NODES, DP), f32),
            jax.ShapeDtypeStruct((N_NODES, DP), f32),
        ],
    )(x, wa, wb, b)


NBLK = N_EDGES // BE


def _edge_mlp_body(ea_ref, eb_ref, g_ref, be_ref, w_ref, b_ref, out_ref,
                   scr, sums, sqs):
    s = pl.program_id(0)
    b = pl.program_id(1)

    @pl.when(s == 0)
    def _():
        s1 = _swish(ea_ref[...] + eb_ref[...])
        scr[b] = s1.astype(bf16)
        ps = jnp.sum(s1, axis=0, keepdims=True)
        pq = jnp.sum(s1 * s1, axis=0, keepdims=True)

        @pl.when(b == 0)
        def _():
            sums[0] = ps
            sqs[0] = pq

        @pl.when(b != 0)
        def _():
            sums[0] = sums[0] + ps
            sqs[0] = sqs[0] + pq

    @pl.when(s > 0)
    def _():
        sm1 = s - 1
        h = scr[b].astype(f32)
        m = sums[sm1] * (1.0 / N_EDGES)
        var = sqs[sm1] * (1.0 / N_EDGES) - m * m
        h = (h - m) * lax.rsqrt(var + EPS) * g_ref[0] + be_ref[0]
        z = jnp.dot(h.astype(bf16), w_ref[0], preferred_element_type=f32) + b_ref[0]
        sv = _swish(z)

        @pl.when(s < 3)
        def _():
            scr[b] = sv.astype(bf16)
            ps = jnp.sum(sv, axis=0, keepdims=True)
            pq = jnp.sum(sv * sv, axis=0, keepdims=True)

            @pl.when(b == 0)
            def _():
                sums[s] = ps
                sqs[s] = pq

            @pl.when(b != 0)
            def _():
                sums[s] = sums[s] + ps
                sqs[s] = sqs[s] + pq

        @pl.when(s == 3)
        def _():
            col = lax.broadcasted_iota(jnp.int32, sv.shape, 1)
            out_ref[...] = jnp.where(col == DP - 1, 1.0, sv)


def _edge_mlp(ea, eb, gs, bes, ws, bs):
    emap = lambda s, b: (jnp.where(s == 0, b, 0), 0)
    wm = lambda s, b: (jnp.maximum(s - 1, 0), 0, 0)
    return pl.pallas_call(
        _edge_mlp_body,
        grid=(4, NBLK),
        in_specs=[
            pl.BlockSpec((BE, DP), emap),
            pl.BlockSpec((BE, DP), emap),
            pl.BlockSpec((1, 1, DP), wm),
            pl.BlockSpec((1, 1, DP), wm),
            pl.BlockSpec((1, DP, DP), wm),
            pl.BlockSpec((1, 1, DP), wm),
        ],
        out_specs=[pl.BlockSpec((BE, DP), lambda s, b: (jnp.where(s == 3, b, 0), 0))],
        out_shape=[jax.ShapeDtypeStruct((N_EDGES, DP), f32)],
        scratch_shapes=[
            pltpu.VMEM((NBLK, BE, DP), bf16),
            pltpu.VMEM((3, 1, DP), f32),
            pltpu.VMEM((3, 1, DP), f32),
        ],
        compiler_params=pltpu.CompilerParams(vmem_limit_bytes=100 * 1024 * 1024),
    )(ea, eb, gs, bes, ws, bs)[0]


def _s1_body(ea_ref, eb_ref, sum_ref, sq_ref):
    i = pl.program_id(0)
    s = _swish(ea_ref[...].astype(f32) + eb_ref[...].astype(f32))
    ps = jnp.sum(s, axis=0, keepdims=True)
    pq = jnp.sum(s * s, axis=0, keepdims=True)

    @pl.when(i == 0)
    def _():
        sum_ref[...] = ps
        sq_ref[...] = pq

    @pl.when(i != 0)
    def _():
        sum_ref[...] = sum_ref[...] + ps
        sq_ref[...] = sq_ref[...] + pq


def _s1_stats(ea, eb):
    g = N_EDGES // BE
    return pl.pallas_call(
        _s1_body,
        grid=(g,),
        in_specs=[pl.BlockSpec((BE, DP), lambda i: (i, 0)),
                  pl.BlockSpec((BE, DP), lambda i: (i, 0))],
        out_specs=[
            pl.BlockSpec((1, DP), lambda i: (0, 0)),
            pl.BlockSpec((1, DP), lambda i: (0, 0)),
        ],
        out_shape=[
            jax.ShapeDtypeStruct((1, DP), f32),
            jax.ShapeDtypeStruct((1, DP), f32),
        ],
    )(ea, eb)


def _stage(xs, stats, gamma, beta, w, b, *, pre_swish, track_stats, ones_col=False):
    g = N_EDGES // BE
    one = pl.BlockSpec((1, DP), lambda i: (0, 0))
    n_in = len(xs)

    def body(*refs):
        in_refs = refs[:n_in]
        sum_ref, sq_ref, g_ref, be_ref, w_ref, b_ref = refs[n_in:n_in + 6]
        outs = refs[n_in + 6:]
        i = pl.program_id(0)
        h = in_refs[0][...].astype(f32)
        for r in in_refs[1:]:
            h = h + r[...].astype(f32)
        if pre_swish:
            h = _swish(h)
        m = sum_ref[...] * (1.0 / N_EDGES)
        var = sq_ref[...] * (1.0 / N_EDGES) - m * m
        h = (h - m) * lax.rsqrt(var + EPS) * g_ref[...] + be_ref[...]
        z = jnp.dot(h.astype(bf16), w_ref[...], preferred_element_type=f32) + b_ref[...]
        s = _swish(z)
        if ones_col:
            col = lax.broadcasted_iota(jnp.int32, s.shape, 1)
            s = jnp.where(col == DP - 1, 1.0, s)
        outs[0][...] = s if ones_col else s.astype(bf16)
        if track_stats:
            ps = jnp.sum(s, axis=0, keepdims=True)
            pq = jnp.sum(s * s, axis=0, keepdims=True)

            @pl.when(i == 0)
            def _():
                outs[1][...] = ps
                outs[2][...] = pq

            @pl.when(i != 0)
            def _():
                outs[1][...] = outs[1][...] + ps
                outs[2][...] = outs[2][...] + pq

    out_specs = [pl.BlockSpec((BE, DP), lambda i: (i, 0)), one, one]
    out_shape = [
        jax.ShapeDtypeStruct((N_EDGES, DP), f32 if ones_col else bf16),
        jax.ShapeDtypeStruct((1, DP), f32),
        jax.ShapeDtypeStruct((1, DP), f32),
    ]
    if not track_stats:
        out_specs, out_shape = out_specs[:1], out_shape[:1]
    return pl.pallas_call(
        body,
        grid=(g,),
        in_specs=[pl.BlockSpec((BE, DP), lambda i: (i, 0))] * n_in
                 + [one, one, one, one,
                    pl.BlockSpec((DP, DP), lambda i: (0, 0)), one],
        out_specs=out_specs,
        out_shape=out_shape,
    )(*xs, stats[0], stats[1], gamma, beta, w, b)


def _update_body(x_ref, v_ref, a_ref, wv_ref, wx_ref, wa_ref, b1_ref,
                 w2_ref, b2_ref, w3_ref, b3_ref, w4_ref, b4_ref, out_ref):
    xb = x_ref[...]
    acc = a_ref[0]
    cnt = jnp.maximum(acc[:, DP - 1 : DP], 1.0)
    agg = acc / cnt
    h = (jnp.dot(v_ref[...], wv_ref[...], preferred_element_type=f32)
         + jnp.dot(xb, wx_ref[...], preferred_element_type=f32)
         + jnp.dot(agg, wa_ref[...], preferred_element_type=f32)
         + b1_ref[...])
    h = _swish(h)
    h = _swish(jnp.dot(h, w2_ref[...], preferred_element_type=f32) + b2_ref[...])
    h = _swish(jnp.dot(h, w3_ref[...], preferred_element_type=f32) + b3_ref[...])
    upd = _swish(jnp.dot(h, w4_ref[...], preferred_element_type=f32) + b4_ref[...])
    out_ref[...] = xb + upd


def _update(x, v, aggs, wv, wx, wa, b1, w2, b2, w3, b3, w4, b4):
    g = N_NODES // BN
    oneh = pl.BlockSpec((1, DH), lambda i: (0, 0))
    return pl.pallas_call(
        _update_body,
        grid=(g,),
        in_specs=[
            pl.BlockSpec((BN, D_FEAT), lambda i: (i, 0)),
            pl.BlockSpec((BN, DH), lambda i: (i, 0)),
            pl.BlockSpec((1, BN, DP), lambda i: (i // 3, i - 3 * (i // 3), 0)),
            pl.BlockSpec((DH, DH), lambda i: (0, 0)),
            pl.BlockSpec((D_FEAT, DH), lambda i: (0, 0)),
            pl.BlockSpec((DP, DH), lambda i: (0, 0)),
            oneh,
            pl.BlockSpec((DH, DH), lambda i: (0, 0)), oneh,
            pl.BlockSpec((DH, DH), lambda i: (0, 0)), oneh,
            pl.BlockSpec((DH, D_FEAT), lambda i: (0, 0)),
            pl.BlockSpec((1, D_FEAT), lambda i: (0, 0)),
        ],
        out_specs=[pl.BlockSpec((BN, D_FEAT), lambda i: (i, 0))],
        out_shape=[jax.ShapeDtypeStruct((N_NODES, D_FEAT), f32)],
    )(x, v, aggs, wv, wx, wa, b1, w2, b2, w3, b3, w4, b4)[0]


def _pre_pool_body(x_ref, w1_ref, b1_ref, w2_ref, b2_ref, w3_ref, b3_ref,
                   w4_ref, b4_ref, sum_ref):
    i = pl.program_id(0)
    h = _swish(jnp.dot(x_ref[...], w1_ref[...], preferred_element_type=f32) + b1_ref[...])
    h = _swish(jnp.dot(h, w2_ref[...], preferred_element_type=f32) + b2_ref[...])
    h = _swish(jnp.dot(h, w3_ref[...], preferred_element_type=f32) + b3_ref[...])
    h = jnp.dot(h, w4_ref[...], preferred_element_type=f32) + b4_ref[...]
    ps = jnp.sum(h, axis=0, keepdims=True)

    @pl.when(i == 0)
    def _():
        sum_ref[...] = ps

    @pl.when(i != 0)
    def _():
        sum_ref[...] = sum_ref[...] + ps


def _pre_pool(x, w1, b1, w2, b2, w3, b3, w4, b4):
    g = N_NODES // BN
    oneh = pl.BlockSpec((1, DH), lambda i: (0, 0))
    return pl.pallas_call(
        _pre_pool_body,
        grid=(g,),
        in_specs=[
            pl.BlockSpec((BN, D_FEAT), lambda i: (i, 0)),
            pl.BlockSpec((D_FEAT, DH), lambda i: (0, 0)), oneh,
            pl.BlockSpec((DH, DH), lambda i: (0, 0)), oneh,
            pl.BlockSpec((DH, DH), lambda i: (0, 0)), oneh,
            pl.BlockSpec((DH, DH), lambda i: (0, 0)), oneh,
        ],
        out_specs=[oneh],
        out_shape=[jax.ShapeDtypeStruct((1, DH), f32)],
    )(x, w1, b1, w2, b2, w3, b3, w4, b4)[0]


def _final_body(hsum_ref, v0_ref, w1_ref, b1_ref, w2_ref, b2_ref, out_ref):
    pooled = hsum_ref[...] * (1.0 / N_NODES)
    c = _swish(jnp.dot(pooled, w1_ref[...], preferred_element_type=f32) + b1_ref[...])
    coeff = jnp.dot(c, w2_ref[...], preferred_element_type=f32) + b2_ref[...]
    out_ref[...] = v0_ref[...] * coeff


def _final(hsum, v0, w1, b1, w2, b2):
    return pl.pallas_call(
        _final_body,
        out_shape=jax.ShapeDtypeStruct((1, DH), f32),
    )(hsum, v0, w1, b1, w2, b2)


@functools.cache
def _mesh():
    return plsc.VectorSubcoreMesh(core_axis_name="c", subcore_axis_name="s",
                                  num_cores=NC, num_subcores=NS)


NPAIR = (NFULL - 1) // 2


def _sc_gather_body(a_hbm, b_hbm, dst_hbm, src_hbm, oa_hbm, ob_hbm,
                    idxd0, idxs0, idxd1, idxs1, ra0, rb0, ra1, rb1, sem_g, sem_w):
    cid = lax.axis_index("c")
    sid = lax.axis_index("s")
    wid = sid * NC + cid
    base = wid * EPW
    sets = ((idxd0, idxs0, ra0, rb0, 0), (idxd1, idxs1, ra1, rb1, 1))

    def start(ci, s):
        idxd, idxs, ra, rb, b = s
        off = base + ci * CHUNK
        pltpu.sync_copy(dst_hbm.at[pl.ds(off, CHUNK)], idxd)
        pltpu.sync_copy(src_hbm.at[pl.ds(off, CHUNK)], idxs)
        pltpu.async_copy(a_hbm.at[idxd], ra, sem_g.at[b, 0])
        pltpu.async_copy(b_hbm.at[idxs], rb, sem_g.at[b, 1])

    def wait_gather(s):
        idxd, idxs, ra, rb, b = s
        pltpu.make_async_copy(a_hbm.at[idxd], ra, sem_g.at[b, 0]).wait()
        pltpu.make_async_copy(b_hbm.at[idxs], rb, sem_g.at[b, 1]).wait()

    def wstart(ci, s):
        _, _, ra, rb, b = s
        off = pl.ds(base + ci * CHUNK, CHUNK)
        pltpu.async_copy(ra, oa_hbm.at[off], sem_w.at[b, 0])
        pltpu.async_copy(rb, ob_hbm.at[off], sem_w.at[b, 1])

    def wwait(ci, s):
        _, _, ra, rb, b = s
        off = pl.ds(base + ci * CHUNK, CHUNK)
        pltpu.make_async_copy(ra, oa_hbm.at[off], sem_w.at[b, 0]).wait()
        pltpu.make_async_copy(rb, ob_hbm.at[off], sem_w.at[b, 1]).wait()

    start(0, sets[0])
    start(1, sets[1])

    def body(p, carry):
        c0 = 2 * p
        wait_gather(sets[0])
        wstart(c0, sets[0])
        wait_gather(sets[1])
        wstart(c0 + 1, sets[1])

        @pl.when(p < NPAIR - 1)
        def _():
            wwait(c0, sets[0])
            start(c0 + 2, sets[0])
            wwait(c0 + 1, sets[1])
            start(c0 + 3, sets[1])

        return carry

    lax.fori_loop(0, NPAIR, body, 0)
    last0 = 2 * (NPAIR - 1)
    wwait(last0, sets[0])
    wwait(last0 + 1, sets[1])

    start(NFULL - 1, sets[0])
    wait_gather(sets[0])
    off = pl.ds(base + (NFULL - 1) * CHUNK, CHUNK)
    pltpu.sync_copy(ra0, oa_hbm.at[off])
    pltpu.sync_copy(rb0, ob_hbm.at[off])

    if TAIL:
        offt = base + NFULL * CHUNK
        id_d = idxd1.at[pl.ds(0, TAIL)]
        id_s = idxs1.at[pl.ds(0, TAIL)]
        pltpu.sync_copy(dst_hbm.at[pl.ds(offt, TAIL)], id_d)
        pltpu.sync_copy(src_hbm.at[pl.ds(offt, TAIL)], id_s)
        pltpu.async_copy(a_hbm.at[id_d], ra1.at[pl.ds(0, TAIL)], sem_g.at[1, 0]).wait()
        pltpu.async_copy(b_hbm.at[id_s], rb1.at[pl.ds(0, TAIL)], sem_g.at[1, 1]).wait()
        pltpu.sync_copy(ra1.at[pl.ds(0, TAIL)], oa_hbm.at[pl.ds(offt, TAIL)])
        pltpu.sync_copy(rb1.at[pl.ds(0, TAIL)], ob_hbm.at[pl.ds(offt, TAIL)])


def _gather_edges(a, b, dst, src):
    return pl.kernel(
        _sc_gather_body,
        out_type=[
            jax.ShapeDtypeStruct((N_EDGES, DP), f32),
            jax.ShapeDtypeStruct((N_EDGES, DP), f32),
        ],
        mesh=_mesh(),
        scratch_types=[
            pltpu.VMEM((CHUNK,), jnp.int32),
            pltpu.VMEM((CHUNK,), jnp.int32),
            pltpu.VMEM((CHUNK,), jnp.int32),
            pltpu.VMEM((CHUNK,), jnp.int32),
            pltpu.VMEM((CHUNK, DP), f32),
            pltpu.VMEM((CHUNK, DP), f32),
            pltpu.VMEM((CHUNK, DP), f32),
            pltpu.VMEM((CHUNK, DP), f32),
            pltpu.SemaphoreType.DMA((2, 2)),
            pltpu.SemaphoreType.DMA((2, 2)),
        ],
    )(a, b, dst, src)


def _sc_scatter_body(msg_hbm, dst_hbm, out_hbm, idx, idx_t, rows, zbuf, shared,
                     sem_r, sem_s):
    cid = lax.axis_index("c")
    sid = lax.axis_index("s")
    base = sid * EPT
    nbase = cid * OWN

    zvec = jnp.zeros((16,), f32)

    def zrow(r, carry):
        for j in range(DP // 16):
            zbuf[r, pl.ds(j * 16, 16)] = zvec
        return carry

    lax.fori_loop(0, TSTRIPE, zrow, 0)
    pltpu.sync_copy(zbuf, shared.at[pl.ds(sid * TSTRIPE, TSTRIPE)])
    plsc.subcore_barrier()

    def remap(id_buf, b, k):
        for j in range(k // 16):
            sl = (b, pl.ds(j * 16, 16))
            local = id_buf[sl] - nbase
            ok = (local >= 0) & (local < OWN)
            id_buf[sl] = jnp.where(ok, local, TRASH)

    def start_chunk(ci, b):
        off = base + ci * CHUNK
        pltpu.sync_copy(dst_hbm.at[pl.ds(off, CHUNK)], idx.at[b])
        pltpu.async_copy(msg_hbm.at[pl.ds(off, CHUNK)], rows.at[b], sem_r.at[b])
        remap(idx, b, CHUNK)

    start_chunk(0, 0)

    def body(i, carry):
        b = lax.rem(i, 2)
        nb = 1 - b

        @pl.when(i + 1 < NF2)
        def _():
            @pl.when(i >= 1)
            def _():
                pltpu.make_async_copy(rows.at[nb], shared.at[idx.at[nb]],
                                      sem_s.at[nb]).wait()

            start_chunk(i + 1, nb)

        pltpu.make_async_copy(msg_hbm.at[pl.ds(0, CHUNK)], rows.at[b],
                              sem_r.at[b]).wait()
        pltpu.async_copy(rows.at[b], shared.at[idx.at[b]], sem_s.at[b], add=True)
        return carry

    lax.fori_loop(0, NF2, body, 0)
    pb = (NF2 - 1) % 2
    pltpu.make_async_copy(rows.at[pb], shared.at[idx.at[pb]], sem_s.at[pb]).wait()
    pltpu.make_async_copy(rows.at[1 - pb], shared.at[idx.at[1 - pb]],
                          sem_s.at[1 - pb]).wait()

    if TAIL2:
        off = base + NF2 * CHUNK
        pltpu.sync_copy(dst_hbm.at[pl.ds(off, TAIL2)], idx_t)
        pltpu.sync_copy(msg_hbm.at[pl.ds(off, TAIL2)], rows.at[0, pl.ds(0, TAIL2)])
        for j in range(TAIL2 // 16):
            sl = pl.ds(j * 16, 16)
            local = idx_t[sl] - nbase
            ok = (local >= 0) & (local < OWN)
            idx_t[sl] = jnp.where(ok, local, TRASH)
        pltpu.sync_copy(rows.at[0, pl.ds(0, TAIL2)], shared.at[idx_t], add=True)

    plsc.subcore_barrier()
    pltpu.sync_copy(shared.at[pl.ds(sid * TSTRIPE, TSTRIPE)],
                    out_hbm.at[cid, pl.ds(sid * TSTRIPE, TSTRIPE)])


def _scatter_msgs(msg, dst):
    return pl.kernel(
        _sc_scatter_body,
        out_type=jax.ShapeDtypeStruct((NC, TROWS, DP), f32),
        mesh=_mesh(),
        scratch_types=[
            pltpu.VMEM((2, CHUNK), jnp.int32),
            pltpu.VMEM((TAIL2,), jnp.int32),
            pltpu.VMEM((2, CHUNK, DP), f32),
            pltpu.VMEM((TSTRIPE, DP), f32),
            pltpu.VMEM_SHARED((TROWS, DP), f32),
            pltpu.SemaphoreType.DMA((2,)),
            pltpu.SemaphoreType.DMA((2,)),
        ],
    )(msg, dst)


def _msg_weights(mp):
    w1 = mp["l1"]["w"]
    wa = _pad2(w1[:, :D_FEAT].T, D_FEAT, DP)
    wb = _pad2(w1[:, D_FEAT:].T, DH, DP)
    b1 = _pad1(mp["l1"]["b"], DP)
    out = {"wa": wa, "wb": wb, "b1": b1}
    for k in ("2", "3"):
        out["w" + k] = _pad2(mp["l" + k]["w"].T, DP, DP).astype(bf16)
        out["b" + k] = _pad1(mp["l" + k]["b"], DP)
        out["g" + k] = _pad1(mp["bn" + k]["gamma"], DP)
        out["be" + k] = _pad1(mp["bn" + k]["beta"], DP)
    out["g1"] = _pad1(mp["bn1"]["gamma"], DP)
    out["be1"] = _pad1(mp["bn1"]["beta"], DP)
    out["w4"] = _pad2(mp["l4"]["w"].T, DP, DP).astype(bf16)
    out["b4"] = _pad1(mp["l4"]["b"], DP)
    return out


def _gnn_layer(x, v, dst, src, lp):
    mw = _msg_weights(lp["msg"])
    a, b = _node_pre(x, mw["wa"], mw["wb"], mw["b1"])
    ea, eb = _gather_edges(a, b, dst, src)
    gs = jnp.stack([mw["g1"], mw["g2"], mw["g3"]])
    bes = jnp.stack([mw["be1"], mw["be2"], mw["be3"]])
    ws = jnp.stack([mw["w2"], mw["w3"], mw["w4"]])
    bs = jnp.stack([mw["b2"], mw["b3"], mw["b4"]])
    msg = _edge_mlp(ea, eb, gs, bes, ws, bs)
    aggs = _scatter_msgs(msg, dst)

    up = lp["upd"]
    w1u = up["l1"]["w"]
    wv = w1u[:, :DH].T
    wx = w1u[:, DH:DH + D_FEAT].T
    wa = _pad2(w1u[:, DH + D_FEAT:].T, DP, DH)
    return _update(
        x, v, aggs,
        wv, wx, wa, _pad1(up["l1"]["b"], DH),
        up["l2"]["w"].T, _pad1(up["l2"]["b"], DH),
        up["l3"]["w"].T, _pad1(up["l3"]["b"], DH),
        up["l4"]["w"].T, _pad1(up["l4"]["b"], D_FEAT),
    )


def kernel(node_feature, edge_index, vectors, params):
    x0 = node_feature[0]
    src = edge_index[0, 0]
    dst = edge_index[0, 1]
    v = x0[:, :DH]
    stacked = jax.tree.map(lambda *a: jnp.stack(a), *params["layers"])

    def _layer_step(xc, lp):
        return _gnn_layer(xc, v, dst, src, lp), None

    x, _ = lax.scan(_layer_step, x0, stacked)

    pp = params["pre"]
    hsum = _pre_pool(
        x,
        pp["l1"]["w"].T, _pad1(pp["l1"]["b"], DH),
        pp["l2"]["w"].T, _pad1(pp["l2"]["b"], DH),
        pp["l3"]["w"].T, _pad1(pp["l3"]["b"], DH),
        pp["l4"]["w"].T, _pad1(pp["l4"]["b"], DH),
    )
    qp = params["post"]
    out = _final(hsum, x0[0:1, :DH],
                 qp["l1"]["w"].T, _pad1(qp["l1"]["b"], DH),
                 qp["l2"]["w"].T, _pad1(qp["l2"]["b"], DH))
    return out.reshape((DH,))


# speedup vs baseline: 2.5280x; 1.0755x over previous
import functools

import jax
import jax.numpy as jnp
from jax import lax
from jax.experimental import pallas as pl
from jax.experimental.pallas import tpu as pltpu
from jax.experimental.pallas import tpu_sc as plsc

N_NODES = 10000
N_EDGES = 160000
D_FEAT = 300
DH = 100
DP = 128
DP2 = 64
NC, NS = 2, 16
NW = NC * NS
EPW = N_EDGES // NW
CHUNK = 128
NFULL = EPW // CHUNK
TAIL = EPW - NFULL * CHUNK
OWN = 6000
TROWS = 6016
TSTRIPE = TROWS // NS
TRASH = TROWS - 1
EPT = N_EDGES // NS
NF2 = EPT // CHUNK
TAIL2 = EPT - NF2 * CHUNK
BE = 4000
BN = 2000
EPS = 1e-5

f32 = jnp.float32
bf16 = jnp.bfloat16


def _swish(x):
    return x * lax.logistic(x)


def _pad2(w, r, c):
    return jnp.zeros((r, c), f32).at[: w.shape[0], : w.shape[1]].set(w)


def _pad1(b, n):
    return jnp.zeros((1, n), f32).at[0, : b.shape[0]].set(b)


def _node_pre_body(x_ref, wa_ref, wb_ref, b_ref, a_ref, bb_ref):
    xb = x_ref[...]
    a_ref[...] = jnp.dot(xb, wa_ref[...], preferred_element_type=f32) + b_ref[...]
    bb_ref[...] = jnp.dot(xb[:, :DH], wb_ref[...], preferred_element_type=f32)


def _node_pre(x, wa, wb, b):
    g = N_NODES // BN
    return pl.pallas_call(
        _node_pre_body,
        grid=(g,),
        in_specs=[
            pl.BlockSpec((BN, D_FEAT), lambda i: (i, 0)),
            pl.BlockSpec((D_FEAT, DP), lambda i: (0, 0)),
            pl.BlockSpec((DH, DP), lambda i: (0, 0)),
            pl.BlockSpec((1, DP), lambda i: (0, 0)),
        ],
        out_specs=[
            pl.BlockSpec((BN, DP), lambda i: (i, 0)),
            pl.BlockSpec((BN, DP), lambda i: (i, 0)),
        ],
        out_shape=[
            jax.ShapeDtypeStruct((N_NODES, DP), f32),
            jax.ShapeDtypeStruct((N_NODES, DP), f32),
        ],
    )(x, wa, wb, b)


NBLK = N_EDGES // BE


def _edge_mlp_body(ea_ref, eb_ref, g_ref, be_ref, w_ref, b_ref, out_ref,
                   scr, sums, sqs):
    s = pl.program_id(0)
    b = pl.program_id(1)

    @pl.when(s == 0)
    def _():
        e16 = (ea_ref[...] + eb_ref[...]).astype(bf16)
        s1 = (e16 * lax.logistic(e16)).astype(f32)
        scr[b] = s1.astype(bf16)
        ps = jnp.sum(s1, axis=0, keepdims=True)
        pq = jnp.sum(s1 * s1, axis=0, keepdims=True)

        @pl.when(b == 0)
        def _():
            sums[0] = ps
            sqs[0] = pq

        @pl.when(b != 0)
        def _():
            sums[0] = sums[0] + ps
            sqs[0] = sqs[0] + pq

    @pl.when(s > 0)
    def _():
        sm1 = s - 1
        h = scr[b].astype(f32)
        m = sums[sm1] * (1.0 / N_EDGES)
        var = sqs[sm1] * (1.0 / N_EDGES) - m * m
        h = (h - m) * lax.rsqrt(var + EPS) * g_ref[0] + be_ref[0]
        z = jnp.dot(h.astype(bf16), w_ref[0], preferred_element_type=f32) + b_ref[0]
        z16 = z.astype(bf16)
        sv = (z16 * lax.logistic(z16)).astype(f32)

        @pl.when(s < 3)
        def _():
            scr[b] = sv.astype(bf16)
            ps = jnp.sum(sv, axis=0, keepdims=True)
            pq = jnp.sum(sv * sv, axis=0, keepdims=True)

            @pl.when(b == 0)
            def _():
                sums[s] = ps
                sqs[s] = pq

            @pl.when(b != 0)
            def _():
                sums[s] = sums[s] + ps
                sqs[s] = sqs[s] + pq

        @pl.when(s == 3)
        def _():
            col = lax.broadcasted_iota(jnp.int32, sv.shape, 1)
            out_ref[...] = jnp.where(col == DP - 1, 1.0, sv)


def _edge_mlp(ea, eb, gs, bes, ws, bs):
    emap = lambda s, b: (jnp.where(s == 0, b, 0), 0)
    wm = lambda s, b: (jnp.maximum(s - 1, 0), 0, 0)
    return pl.pallas_call(
        _edge_mlp_body,
        grid=(4, NBLK),
        in_specs=[
            pl.BlockSpec((BE, DP), emap),
            pl.BlockSpec((BE, DP), emap),
            pl.BlockSpec((1, 1, DP), wm),
            pl.BlockSpec((1, 1, DP), wm),
            pl.BlockSpec((1, DP, DP), wm),
            pl.BlockSpec((1, 1, DP), wm),
        ],
        out_specs=[pl.BlockSpec((BE, DP), lambda s, b: (jnp.where(s == 3, b, 0), 0))],
        out_shape=[jax.ShapeDtypeStruct((N_EDGES, DP), f32)],
        scratch_shapes=[
            pltpu.VMEM((NBLK, BE, DP), bf16),
            pltpu.VMEM((3, 1, DP), f32),
            pltpu.VMEM((3, 1, DP), f32),
        ],
        compiler_params=pltpu.CompilerParams(vmem_limit_bytes=100 * 1024 * 1024),
    )(ea, eb, gs, bes, ws, bs)[0]


def _s1_body(ea_ref, eb_ref, sum_ref, sq_ref):
    i = pl.program_id(0)
    s = _swish(ea_ref[...].astype(f32) + eb_ref[...].astype(f32))
    ps = jnp.sum(s, axis=0, keepdims=True)
    pq = jnp.sum(s * s, axis=0, keepdims=True)

    @pl.when(i == 0)
    def _():
        sum_ref[...] = ps
        sq_ref[...] = pq

    @pl.when(i != 0)
    def _():
        sum_ref[...] = sum_ref[...] + ps
        sq_ref[...] = sq_ref[...] + pq


def _s1_stats(ea, eb):
    g = N_EDGES // BE
    return pl.pallas_call(
        _s1_body,
        grid=(g,),
        in_specs=[pl.BlockSpec((BE, DP), lambda i: (i, 0)),
                  pl.BlockSpec((BE, DP), lambda i: (i, 0))],
        out_specs=[
            pl.BlockSpec((1, DP), lambda i: (0, 0)),
            pl.BlockSpec((1, DP), lambda i: (0, 0)),
        ],
        out_shape=[
            jax.ShapeDtypeStruct((1, DP), f32),
            jax.ShapeDtypeStruct((1, DP), f32),
        ],
    )(ea, eb)


def _stage(xs, stats, gamma, beta, w, b, *, pre_swish, track_stats, ones_col=False):
    g = N_EDGES // BE
    one = pl.BlockSpec((1, DP), lambda i: (0, 0))
    n_in = len(xs)

    def body(*refs):
        in_refs = refs[:n_in]
        sum_ref, sq_ref, g_ref, be_ref, w_ref, b_ref = refs[n_in:n_in + 6]
        outs = refs[n_in + 6:]
        i = pl.program_id(0)
        h = in_refs[0][...].astype(f32)
        for r in in_refs[1:]:
            h = h + r[...].astype(f32)
        if pre_swish:
            h = _swish(h)
        m = sum_ref[...] * (1.0 / N_EDGES)
        var = sq_ref[...] * (1.0 / N_EDGES) - m * m
        h = (h - m) * lax.rsqrt(var + EPS) * g_ref[...] + be_ref[...]
        z = jnp.dot(h.astype(bf16), w_ref[...], preferred_element_type=f32) + b_ref[...]
        s = _swish(z)
        if ones_col:
            col = lax.broadcasted_iota(jnp.int32, s.shape, 1)
            s = jnp.where(col == DP - 1, 1.0, s)
        outs[0][...] = s if ones_col else s.astype(bf16)
        if track_stats:
            ps = jnp.sum(s, axis=0, keepdims=True)
            pq = jnp.sum(s * s, axis=0, keepdims=True)

            @pl.when(i == 0)
            def _():
                outs[1][...] = ps
                outs[2][...] = pq

            @pl.when(i != 0)
            def _():
                outs[1][...] = outs[1][...] + ps
                outs[2][...] = outs[2][...] + pq

    out_specs = [pl.BlockSpec((BE, DP), lambda i: (i, 0)), one, one]
    out_shape = [
        jax.ShapeDtypeStruct((N_EDGES, DP), f32 if ones_col else bf16),
        jax.ShapeDtypeStruct((1, DP), f32),
        jax.ShapeDtypeStruct((1, DP), f32),
    ]
    if not track_stats:
        out_specs, out_shape = out_specs[:1], out_shape[:1]
    return pl.pallas_call(
        body,
        grid=(g,),
        in_specs=[pl.BlockSpec((BE, DP), lambda i: (i, 0))] * n_in
                 + [one, one, one, one,
                    pl.BlockSpec((DP, DP), lambda i: (0, 0)), one],
        out_specs=out_specs,
        out_shape=out_shape,
    )(*xs, stats[0], stats[1], gamma, beta, w, b)


def _update_body(x_ref, v_ref, a_ref, wv_ref, wx_ref, wa_ref, b1_ref,
                 w2_ref, b2_ref, w3_ref, b3_ref, w4_ref, b4_ref, out_ref):
    xb = x_ref[...]
    acc = a_ref[0]
    cnt = jnp.maximum(acc[:, DP - 1 : DP], 1.0)
    agg = acc / cnt
    h = (jnp.dot(v_ref[...], wv_ref[...], preferred_element_type=f32)
         + jnp.dot(xb, wx_ref[...], preferred_element_type=f32)
         + jnp.dot(agg, wa_ref[...], preferred_element_type=f32)
         + b1_ref[...])
    h = _swish(h)
    h = _swish(jnp.dot(h, w2_ref[...], preferred_element_type=f32) + b2_ref[...])
    h = _swish(jnp.dot(h, w3_ref[...], preferred_element_type=f32) + b3_ref[...])
    upd = _swish(jnp.dot(h, w4_ref[...], preferred_element_type=f32) + b4_ref[...])
    out_ref[...] = xb + upd


def _update(x, v, aggs, wv, wx, wa, b1, w2, b2, w3, b3, w4, b4):
    g = N_NODES // BN
    oneh = pl.BlockSpec((1, DH), lambda i: (0, 0))
    return pl.pallas_call(
        _update_body,
        grid=(g,),
        in_specs=[
            pl.BlockSpec((BN, D_FEAT), lambda i: (i, 0)),
            pl.BlockSpec((BN, DH), lambda i: (i, 0)),
            pl.BlockSpec((1, BN, DP), lambda i: (i // 3, i - 3 * (i // 3), 0)),
            pl.BlockSpec((DH, DH), lambda i: (0, 0)),
            pl.BlockSpec((D_FEAT, DH), lambda i: (0, 0)),
            pl.BlockSpec((DP, DH), lambda i: (0, 0)),
            oneh,
            pl.BlockSpec((DH, DH), lambda i: (0, 0)), oneh,
            pl.BlockSpec((DH, DH), lambda i: (0, 0)), oneh,
            pl.BlockSpec((DH, D_FEAT), lambda i: (0, 0)),
            pl.BlockSpec((1, D_FEAT), lambda i: (0, 0)),
        ],
        out_specs=[pl.BlockSpec((BN, D_FEAT), lambda i: (i, 0))],
        out_shape=[jax.ShapeDtypeStruct((N_NODES, D_FEAT), f32)],
    )(x, v, aggs, wv, wx, wa, b1, w2, b2, w3, b3, w4, b4)[0]


def _pre_pool_body(x_ref, w1_ref, b1_ref, w2_ref, b2_ref, w3_ref, b3_ref,
                   w4_ref, b4_ref, sum_ref):
    i = pl.program_id(0)
    h = _swish(jnp.dot(x_ref[...], w1_ref[...], preferred_element_type=f32) + b1_ref[...])
    h = _swish(jnp.dot(h, w2_ref[...], preferred_element_type=f32) + b2_ref[...])
    h = _swish(jnp.dot(h, w3_ref[...], preferred_element_type=f32) + b3_ref[...])
    h = jnp.dot(h, w4_ref[...], preferred_element_type=f32) + b4_ref[...]
    ps = jnp.sum(h, axis=0, keepdims=True)

    @pl.when(i == 0)
    def _():
        sum_ref[...] = ps

    @pl.when(i != 0)
    def _():
        sum_ref[...] = sum_ref[...] + ps


def _pre_pool(x, w1, b1, w2, b2, w3, b3, w4, b4):
    g = N_NODES // BN
    oneh = pl.BlockSpec((1, DH), lambda i: (0, 0))
    return pl.pallas_call(
        _pre_pool_body,
        grid=(g,),
        in_specs=[
            pl.BlockSpec((BN, D_FEAT), lambda i: (i, 0)),
            pl.BlockSpec((D_FEAT, DH), lambda i: (0, 0)), oneh,
            pl.BlockSpec((DH, DH), lambda i: (0, 0)), oneh,
            pl.BlockSpec((DH, DH), lambda i: (0, 0)), oneh,
            pl.BlockSpec((DH, DH), lambda i: (0, 0)), oneh,
        ],
        out_specs=[oneh],
        out_shape=[jax.ShapeDtypeStruct((1, DH), f32)],
    )(x, w1, b1, w2, b2, w3, b3, w4, b4)[0]


def _final_body(hsum_ref, v0_ref, w1_ref, b1_ref, w2_ref, b2_ref, out_ref):
    pooled = hsum_ref[...] * (1.0 / N_NODES)
    c = _swish(jnp.dot(pooled, w1_ref[...], preferred_element_type=f32) + b1_ref[...])
    coeff = jnp.dot(c, w2_ref[...], preferred_element_type=f32) + b2_ref[...]
    out_ref[...] = v0_ref[...] * coeff


def _final(hsum, v0, w1, b1, w2, b2):
    return pl.pallas_call(
        _final_body,
        out_shape=jax.ShapeDtypeStruct((1, DH), f32),
    )(hsum, v0, w1, b1, w2, b2)


@functools.cache
def _mesh():
    return plsc.VectorSubcoreMesh(core_axis_name="c", subcore_axis_name="s",
                                  num_cores=NC, num_subcores=NS)


NPAIR = (NFULL - 1) // 2


def _sc_gather_body(a_hbm, b_hbm, dst_hbm, src_hbm, oa_hbm, ob_hbm,
                    idxd0, idxs0, idxd1, idxs1, ra0, rb0, ra1, rb1, sem_g, sem_w):
    cid = lax.axis_index("c")
    sid = lax.axis_index("s")
    wid = sid * NC + cid
    base = wid * EPW
    sets = ((idxd0, idxs0, ra0, rb0, 0), (idxd1, idxs1, ra1, rb1, 1))

    def start(ci, s):
        idxd, idxs, ra, rb, b = s
        off = base + ci * CHUNK
        pltpu.sync_copy(dst_hbm.at[pl.ds(off, CHUNK)], idxd)
        pltpu.sync_copy(src_hbm.at[pl.ds(off, CHUNK)], idxs)
        pltpu.async_copy(a_hbm.at[idxd], ra, sem_g.at[b, 0])
        pltpu.async_copy(b_hbm.at[idxs], rb, sem_g.at[b, 1])

    def wait_gather(s):
        idxd, idxs, ra, rb, b = s
        pltpu.make_async_copy(a_hbm.at[idxd], ra, sem_g.at[b, 0]).wait()
        pltpu.make_async_copy(b_hbm.at[idxs], rb, sem_g.at[b, 1]).wait()

    def wstart(ci, s):
        _, _, ra, rb, b = s
        off = pl.ds(base + ci * CHUNK, CHUNK)
        pltpu.async_copy(ra, oa_hbm.at[off], sem_w.at[b, 0])
        pltpu.async_copy(rb, ob_hbm.at[off], sem_w.at[b, 1])

    def wwait(ci, s):
        _, _, ra, rb, b = s
        off = pl.ds(base + ci * CHUNK, CHUNK)
        pltpu.make_async_copy(ra, oa_hbm.at[off], sem_w.at[b, 0]).wait()
        pltpu.make_async_copy(rb, ob_hbm.at[off], sem_w.at[b, 1]).wait()

    start(0, sets[0])
    start(1, sets[1])

    def body(p, carry):
        c0 = 2 * p
        wait_gather(sets[0])
        wstart(c0, sets[0])
        wait_gather(sets[1])
        wstart(c0 + 1, sets[1])

        @pl.when(p < NPAIR - 1)
        def _():
            wwait(c0, sets[0])
            start(c0 + 2, sets[0])
            wwait(c0 + 1, sets[1])
            start(c0 + 3, sets[1])

        return carry

    lax.fori_loop(0, NPAIR, body, 0)
    last0 = 2 * (NPAIR - 1)
    wwait(last0, sets[0])
    wwait(last0 + 1, sets[1])

    start(NFULL - 1, sets[0])
    wait_gather(sets[0])
    off = pl.ds(base + (NFULL - 1) * CHUNK, CHUNK)
    pltpu.sync_copy(ra0, oa_hbm.at[off])
    pltpu.sync_copy(rb0, ob_hbm.at[off])

    if TAIL:
        offt = base + NFULL * CHUNK
        id_d = idxd1.at[pl.ds(0, TAIL)]
        id_s = idxs1.at[pl.ds(0, TAIL)]
        pltpu.sync_copy(dst_hbm.at[pl.ds(offt, TAIL)], id_d)
        pltpu.sync_copy(src_hbm.at[pl.ds(offt, TAIL)], id_s)
        pltpu.async_copy(a_hbm.at[id_d], ra1.at[pl.ds(0, TAIL)], sem_g.at[1, 0]).wait()
        pltpu.async_copy(b_hbm.at[id_s], rb1.at[pl.ds(0, TAIL)], sem_g.at[1, 1]).wait()
        pltpu.sync_copy(ra1.at[pl.ds(0, TAIL)], oa_hbm.at[pl.ds(offt, TAIL)])
        pltpu.sync_copy(rb1.at[pl.ds(0, TAIL)], ob_hbm.at[pl.ds(offt, TAIL)])


def _gather_edges(a, b, dst, src):
    return pl.kernel(
        _sc_gather_body,
        out_type=[
            jax.ShapeDtypeStruct((N_EDGES, DP), f32),
            jax.ShapeDtypeStruct((N_EDGES, DP), f32),
        ],
        mesh=_mesh(),
        scratch_types=[
            pltpu.VMEM((CHUNK,), jnp.int32),
            pltpu.VMEM((CHUNK,), jnp.int32),
            pltpu.VMEM((CHUNK,), jnp.int32),
            pltpu.VMEM((CHUNK,), jnp.int32),
            pltpu.VMEM((CHUNK, DP), f32),
            pltpu.VMEM((CHUNK, DP), f32),
            pltpu.VMEM((CHUNK, DP), f32),
            pltpu.VMEM((CHUNK, DP), f32),
            pltpu.SemaphoreType.DMA((2, 2)),
            pltpu.SemaphoreType.DMA((2, 2)),
        ],
    )(a, b, dst, src)


def _sc_scatter_body(msg_hbm, dst_hbm, out_hbm, idx, idx_t, rows, zbuf, shared,
                     sem_r, sem_s):
    cid = lax.axis_index("c")
    sid = lax.axis_index("s")
    base = sid * EPT
    nbase = cid * OWN

    zvec = jnp.zeros((16,), f32)

    def zrow(r, carry):
        for j in range(DP // 16):
            zbuf[r, pl.ds(j * 16, 16)] = zvec
        return carry

    lax.fori_loop(0, TSTRIPE, zrow, 0)
    pltpu.sync_copy(zbuf, shared.at[pl.ds(sid * TSTRIPE, TSTRIPE)])
    plsc.subcore_barrier()

    def remap(id_buf, b, k):
        for j in range(k // 16):
            sl = (b, pl.ds(j * 16, 16))
            local = id_buf[sl] - nbase
            ok = (local >= 0) & (local < OWN)
            id_buf[sl] = jnp.where(ok, local, TRASH)

    def start_chunk(ci, b):
        off = base + ci * CHUNK
        pltpu.sync_copy(dst_hbm.at[pl.ds(off, CHUNK)], idx.at[b])
        pltpu.async_copy(msg_hbm.at[pl.ds(off, CHUNK)], rows.at[b], sem_r.at[b])
        remap(idx, b, CHUNK)

    start_chunk(0, 0)

    def body(i, carry):
        b = lax.rem(i, 2)
        nb = 1 - b

        @pl.when(i + 1 < NF2)
        def _():
            @pl.when(i >= 1)
            def _():
                pltpu.make_async_copy(rows.at[nb], shared.at[idx.at[nb]],
                                      sem_s.at[nb]).wait()

            start_chunk(i + 1, nb)

        pltpu.make_async_copy(msg_hbm.at[pl.ds(0, CHUNK)], rows.at[b],
                              sem_r.at[b]).wait()
        pltpu.async_copy(rows.at[b], shared.at[idx.at[b]], sem_s.at[b], add=True)
        return carry

    lax.fori_loop(0, NF2, body, 0)
    pb = (NF2 - 1) % 2
    pltpu.make_async_copy(rows.at[pb], shared.at[idx.at[pb]], sem_s.at[pb]).wait()
    pltpu.make_async_copy(rows.at[1 - pb], shared.at[idx.at[1 - pb]],
                          sem_s.at[1 - pb]).wait()

    if TAIL2:
        off = base + NF2 * CHUNK
        pltpu.sync_copy(dst_hbm.at[pl.ds(off, TAIL2)], idx_t)
        pltpu.sync_copy(msg_hbm.at[pl.ds(off, TAIL2)], rows.at[0, pl.ds(0, TAIL2)])
        for j in range(TAIL2 // 16):
            sl = pl.ds(j * 16, 16)
            local = idx_t[sl] - nbase
            ok = (local >= 0) & (local < OWN)
            idx_t[sl] = jnp.where(ok, local, TRASH)
        pltpu.sync_copy(rows.at[0, pl.ds(0, TAIL2)], shared.at[idx_t], add=True)

    plsc.subcore_barrier()
    pltpu.sync_copy(shared.at[pl.ds(sid * TSTRIPE, TSTRIPE)],
                    out_hbm.at[cid, pl.ds(sid * TSTRIPE, TSTRIPE)])


def _scatter_msgs(msg, dst):
    return pl.kernel(
        _sc_scatter_body,
        out_type=jax.ShapeDtypeStruct((NC, TROWS, DP), f32),
        mesh=_mesh(),
        scratch_types=[
            pltpu.VMEM((2, CHUNK), jnp.int32),
            pltpu.VMEM((TAIL2,), jnp.int32),
            pltpu.VMEM((2, CHUNK, DP), f32),
            pltpu.VMEM((TSTRIPE, DP), f32),
            pltpu.VMEM_SHARED((TROWS, DP), f32),
            pltpu.SemaphoreType.DMA((2,)),
            pltpu.SemaphoreType.DMA((2,)),
        ],
    )(msg, dst)


def _msg_weights(mp):
    w1 = mp["l1"]["w"]
    wa = _pad2(w1[:, :D_FEAT].T, D_FEAT, DP)
    wb = _pad2(w1[:, D_FEAT:].T, DH, DP)
    b1 = _pad1(mp["l1"]["b"], DP)
    out = {"wa": wa, "wb": wb, "b1": b1}
    for k in ("2", "3"):
        out["w" + k] = _pad2(mp["l" + k]["w"].T, DP, DP).astype(bf16)
        out["b" + k] = _pad1(mp["l" + k]["b"], DP)
        out["g" + k] = _pad1(mp["bn" + k]["gamma"], DP)
        out["be" + k] = _pad1(mp["bn" + k]["beta"], DP)
    out["g1"] = _pad1(mp["bn1"]["gamma"], DP)
    out["be1"] = _pad1(mp["bn1"]["beta"], DP)
    out["w4"] = _pad2(mp["l4"]["w"].T, DP, DP).astype(bf16)
    out["b4"] = _pad1(mp["l4"]["b"], DP)
    return out


def _gnn_layer(x, v, dst, src, lp):
    mw = _msg_weights(lp["msg"])
    a, b = _node_pre(x, mw["wa"], mw["wb"], mw["b1"])
    ea, eb = _gather_edges(a, b, dst, src)
    gs = jnp.stack([mw["g1"], mw["g2"], mw["g3"]])
    bes = jnp.stack([mw["be1"], mw["be2"], mw["be3"]])
    ws = jnp.stack([mw["w2"], mw["w3"], mw["w4"]])
    bs = jnp.stack([mw["b2"], mw["b3"], mw["b4"]])
    msg = _edge_mlp(ea, eb, gs, bes, ws, bs)
    aggs = _scatter_msgs(msg, dst)

    up = lp["upd"]
    w1u = up["l1"]["w"]
    wv = w1u[:, :DH].T
    wx = w1u[:, DH:DH + D_FEAT].T
    wa = _pad2(w1u[:, DH + D_FEAT:].T, DP, DH)
    return _update(
        x, v, aggs,
        wv, wx, wa, _pad1(up["l1"]["b"], DH),
        up["l2"]["w"].T, _pad1(up["l2"]["b"], DH),
        up["l3"]["w"].T, _pad1(up["l3"]["b"], DH),
        up["l4"]["w"].T, _pad1(up["l4"]["b"], D_FEAT),
    )


def kernel(node_feature, edge_index, vectors, params):
    x0 = node_feature[0]
    src = edge_index[0, 0]
    dst = edge_index[0, 1]
    v = x0[:, :DH]
    stacked = jax.tree.map(lambda *a: jnp.stack(a), *params["layers"])

    def _layer_step(xc, lp):
        return _gnn_layer(xc, v, dst, src, lp), None

    x, _ = lax.scan(_layer_step, x0, stacked)

    pp = params["pre"]
    hsum = _pre_pool(
        x,
        pp["l1"]["w"].T, _pad1(pp["l1"]["b"], DH),
        pp["l2"]["w"].T, _pad1(pp["l2"]["b"], DH),
        pp["l3"]["w"].T, _pad1(pp["l3"]["b"], DH),
        pp["l4"]["w"].T, _pad1(pp["l4"]["b"], DH),
    )
    qp = params["post"]
    out = _final(hsum, x0[0:1, :DH],
                 qp["l1"]["w"].T, _pad1(qp["l1"]["b"], DH),
                 qp["l2"]["w"].T, _pad1(qp["l2"]["b"], DH))
    return out.reshape((DH,))


# speedup vs baseline: 2.6684x; 1.0555x over previous
import functools

import jax
import jax.numpy as jnp
from jax import lax
from jax.experimental import pallas as pl
from jax.experimental.pallas import tpu as pltpu
from jax.experimental.pallas import tpu_sc as plsc

N_NODES = 10000
N_EDGES = 160000
D_FEAT = 300
DH = 100
DP = 128
DP2 = 64
NC, NS = 2, 16
NW = NC * NS
EPW = N_EDGES // NW
CHUNK = 128
NFULL = EPW // CHUNK
TAIL = EPW - NFULL * CHUNK
OWN = 6000
TROWS = 6016
TSTRIPE = TROWS // NS
TRASH = TROWS - 1
EPT = N_EDGES // NS
NF2 = EPT // CHUNK
TAIL2 = EPT - NF2 * CHUNK
BE = 4000
BN = 2000
EPS = 1e-5

f32 = jnp.float32
bf16 = jnp.bfloat16


def _swish(x):
    return x * lax.logistic(x)


def _pad2(w, r, c):
    return jnp.zeros((r, c), f32).at[: w.shape[0], : w.shape[1]].set(w)


def _pad1(b, n):
    return jnp.zeros((1, n), f32).at[0, : b.shape[0]].set(b)


def _node_pre_body(x_ref, wa_ref, wb_ref, b_ref, a_ref, bb_ref):
    xb = x_ref[...]
    a_ref[...] = jnp.dot(xb, wa_ref[...], preferred_element_type=f32) + b_ref[...]
    bb_ref[...] = jnp.dot(xb[:, :DH], wb_ref[...], preferred_element_type=f32)


def _node_pre(x, wa, wb, b):
    g = N_NODES // BN
    return pl.pallas_call(
        _node_pre_body,
        grid=(g,),
        in_specs=[
            pl.BlockSpec((BN, D_FEAT), lambda i: (i, 0)),
            pl.BlockSpec((D_FEAT, DP), lambda i: (0, 0)),
            pl.BlockSpec((DH, DP), lambda i: (0, 0)),
            pl.BlockSpec((1, DP), lambda i: (0, 0)),
        ],
        out_specs=[
            pl.BlockSpec((BN, DP), lambda i: (i, 0)),
            pl.BlockSpec((BN, DP), lambda i: (i, 0)),
        ],
        out_shape=[
            jax.ShapeDtypeStruct((N_NODES, DP), f32),
            jax.ShapeDtypeStruct((N_NODES, DP), f32),
        ],
    )(x, wa, wb, b)


NBLK = N_EDGES // BE


def _edge_mlp_body(ea_ref, g_ref, be_ref, w_ref, b_ref, out_ref,
                   scr, sums, sqs):
    s = pl.program_id(0)
    b = pl.program_id(1)

    @pl.when(s == 0)
    def _():
        e16 = ea_ref[...].astype(bf16)
        s1 = (e16 * lax.logistic(e16)).astype(f32)
        scr[b] = s1.astype(bf16)
        ps = jnp.sum(s1, axis=0, keepdims=True)
        pq = jnp.sum(s1 * s1, axis=0, keepdims=True)

        @pl.when(b == 0)
        def _():
            sums[0] = ps
            sqs[0] = pq

        @pl.when(b != 0)
        def _():
            sums[0] = sums[0] + ps
            sqs[0] = sqs[0] + pq

    @pl.when(s > 0)
    def _():
        sm1 = s - 1
        h = scr[b].astype(f32)
        m = sums[sm1] * (1.0 / N_EDGES)
        var = sqs[sm1] * (1.0 / N_EDGES) - m * m
        h = (h - m) * lax.rsqrt(var + EPS) * g_ref[0] + be_ref[0]
        z = jnp.dot(h.astype(bf16), w_ref[0], preferred_element_type=f32) + b_ref[0]
        z16 = z.astype(bf16)
        sv = (z16 * lax.logistic(z16)).astype(f32)

        @pl.when(s < 3)
        def _():
            scr[b] = sv.astype(bf16)
            ps = jnp.sum(sv, axis=0, keepdims=True)
            pq = jnp.sum(sv * sv, axis=0, keepdims=True)

            @pl.when(b == 0)
            def _():
                sums[s] = ps
                sqs[s] = pq

            @pl.when(b != 0)
            def _():
                sums[s] = sums[s] + ps
                sqs[s] = sqs[s] + pq

        @pl.when(s == 3)
        def _():
            col = lax.broadcasted_iota(jnp.int32, sv.shape, 1)
            out_ref[...] = jnp.where(col == DP - 1, 1.0, sv)


def _edge_mlp(ea, gs, bes, ws, bs):
    emap = lambda s, b: (jnp.where(s == 0, b, 0), 0)
    wm = lambda s, b: (jnp.maximum(s - 1, 0), 0, 0)
    return pl.pallas_call(
        _edge_mlp_body,
        grid=(4, NBLK),
        in_specs=[
            pl.BlockSpec((BE, DP), emap),
            pl.BlockSpec((1, 1, DP), wm),
            pl.BlockSpec((1, 1, DP), wm),
            pl.BlockSpec((1, DP, DP), wm),
            pl.BlockSpec((1, 1, DP), wm),
        ],
        out_specs=[pl.BlockSpec((BE, DP), lambda s, b: (jnp.where(s == 3, b, 0), 0))],
        out_shape=[jax.ShapeDtypeStruct((N_EDGES, DP), f32)],
        scratch_shapes=[
            pltpu.VMEM((NBLK, BE, DP), bf16),
            pltpu.VMEM((3, 1, DP), f32),
            pltpu.VMEM((3, 1, DP), f32),
        ],
        compiler_params=pltpu.CompilerParams(vmem_limit_bytes=100 * 1024 * 1024),
    )(ea, gs, bes, ws, bs)[0]


def _s1_body(ea_ref, eb_ref, sum_ref, sq_ref):
    i = pl.program_id(0)
    s = _swish(ea_ref[...].astype(f32) + eb_ref[...].astype(f32))
    ps = jnp.sum(s, axis=0, keepdims=True)
    pq = jnp.sum(s * s, axis=0, keepdims=True)

    @pl.when(i == 0)
    def _():
        sum_ref[...] = ps
        sq_ref[...] = pq

    @pl.when(i != 0)
    def _():
        sum_ref[...] = sum_ref[...] + ps
        sq_ref[...] = sq_ref[...] + pq


def _s1_stats(ea, eb):
    g = N_EDGES // BE
    return pl.pallas_call(
        _s1_body,
        grid=(g,),
        in_specs=[pl.BlockSpec((BE, DP), lambda i: (i, 0)),
                  pl.BlockSpec((BE, DP), lambda i: (i, 0))],
        out_specs=[
            pl.BlockSpec((1, DP), lambda i: (0, 0)),
            pl.BlockSpec((1, DP), lambda i: (0, 0)),
        ],
        out_shape=[
            jax.ShapeDtypeStruct((1, DP), f32),
            jax.ShapeDtypeStruct((1, DP), f32),
        ],
    )(ea, eb)


def _stage(xs, stats, gamma, beta, w, b, *, pre_swish, track_stats, ones_col=False):
    g = N_EDGES // BE
    one = pl.BlockSpec((1, DP), lambda i: (0, 0))
    n_in = len(xs)

    def body(*refs):
        in_refs = refs[:n_in]
        sum_ref, sq_ref, g_ref, be_ref, w_ref, b_ref = refs[n_in:n_in + 6]
        outs = refs[n_in + 6:]
        i = pl.program_id(0)
        h = in_refs[0][...].astype(f32)
        for r in in_refs[1:]:
            h = h + r[...].astype(f32)
        if pre_swish:
            h = _swish(h)
        m = sum_ref[...] * (1.0 / N_EDGES)
        var = sq_ref[...] * (1.0 / N_EDGES) - m * m
        h = (h - m) * lax.rsqrt(var + EPS) * g_ref[...] + be_ref[...]
        z = jnp.dot(h.astype(bf16), w_ref[...], preferred_element_type=f32) + b_ref[...]
        s = _swish(z)
        if ones_col:
            col = lax.broadcasted_iota(jnp.int32, s.shape, 1)
            s = jnp.where(col == DP - 1, 1.0, s)
        outs[0][...] = s if ones_col else s.astype(bf16)
        if track_stats:
            ps = jnp.sum(s, axis=0, keepdims=True)
            pq = jnp.sum(s * s, axis=0, keepdims=True)

            @pl.when(i == 0)
            def _():
                outs[1][...] = ps
                outs[2][...] = pq

            @pl.when(i != 0)
            def _():
                outs[1][...] = outs[1][...] + ps
                outs[2][...] = outs[2][...] + pq

    out_specs = [pl.BlockSpec((BE, DP), lambda i: (i, 0)), one, one]
    out_shape = [
        jax.ShapeDtypeStruct((N_EDGES, DP), f32 if ones_col else bf16),
        jax.ShapeDtypeStruct((1, DP), f32),
        jax.ShapeDtypeStruct((1, DP), f32),
    ]
    if not track_stats:
        out_specs, out_shape = out_specs[:1], out_shape[:1]
    return pl.pallas_call(
        body,
        grid=(g,),
        in_specs=[pl.BlockSpec((BE, DP), lambda i: (i, 0))] * n_in
                 + [one, one, one, one,
                    pl.BlockSpec((DP, DP), lambda i: (0, 0)), one],
        out_specs=out_specs,
        out_shape=out_shape,
    )(*xs, stats[0], stats[1], gamma, beta, w, b)


def _update_body(x_ref, v_ref, a_ref, wv_ref, wx_ref, wa_ref, b1_ref,
                 w2_ref, b2_ref, w3_ref, b3_ref, w4_ref, b4_ref, out_ref):
    xb = x_ref[...]
    acc = a_ref[0]
    cnt = jnp.maximum(acc[:, DP - 1 : DP], 1.0)
    agg = acc / cnt
    h = (jnp.dot(v_ref[...], wv_ref[...], preferred_element_type=f32)
         + jnp.dot(xb, wx_ref[...], preferred_element_type=f32)
         + jnp.dot(agg, wa_ref[...], preferred_element_type=f32)
         + b1_ref[...])
    h = _swish(h)
    h = _swish(jnp.dot(h, w2_ref[...], preferred_element_type=f32) + b2_ref[...])
    h = _swish(jnp.dot(h, w3_ref[...], preferred_element_type=f32) + b3_ref[...])
    upd = _swish(jnp.dot(h, w4_ref[...], preferred_element_type=f32) + b4_ref[...])
    out_ref[...] = xb + upd


def _update(x, v, aggs, wv, wx, wa, b1, w2, b2, w3, b3, w4, b4):
    g = N_NODES // BN
    oneh = pl.BlockSpec((1, DH), lambda i: (0, 0))
    return pl.pallas_call(
        _update_body,
        grid=(g,),
        in_specs=[
            pl.BlockSpec((BN, D_FEAT), lambda i: (i, 0)),
            pl.BlockSpec((BN, DH), lambda i: (i, 0)),
            pl.BlockSpec((1, BN, DP), lambda i: (i // 3, i - 3 * (i // 3), 0)),
            pl.BlockSpec((DH, DH), lambda i: (0, 0)),
            pl.BlockSpec((D_FEAT, DH), lambda i: (0, 0)),
            pl.BlockSpec((DP, DH), lambda i: (0, 0)),
            oneh,
            pl.BlockSpec((DH, DH), lambda i: (0, 0)), oneh,
            pl.BlockSpec((DH, DH), lambda i: (0, 0)), oneh,
            pl.BlockSpec((DH, D_FEAT), lambda i: (0, 0)),
            pl.BlockSpec((1, D_FEAT), lambda i: (0, 0)),
        ],
        out_specs=[pl.BlockSpec((BN, D_FEAT), lambda i: (i, 0))],
        out_shape=[jax.ShapeDtypeStruct((N_NODES, D_FEAT), f32)],
    )(x, v, aggs, wv, wx, wa, b1, w2, b2, w3, b3, w4, b4)[0]


def _pre_pool_body(x_ref, w1_ref, b1_ref, w2_ref, b2_ref, w3_ref, b3_ref,
                   w4_ref, b4_ref, sum_ref):
    i = pl.program_id(0)
    h = _swish(jnp.dot(x_ref[...], w1_ref[...], preferred_element_type=f32) + b1_ref[...])
    h = _swish(jnp.dot(h, w2_ref[...], preferred_element_type=f32) + b2_ref[...])
    h = _swish(jnp.dot(h, w3_ref[...], preferred_element_type=f32) + b3_ref[...])
    h = jnp.dot(h, w4_ref[...], preferred_element_type=f32) + b4_ref[...]
    ps = jnp.sum(h, axis=0, keepdims=True)

    @pl.when(i == 0)
    def _():
        sum_ref[...] = ps

    @pl.when(i != 0)
    def _():
        sum_ref[...] = sum_ref[...] + ps


def _pre_pool(x, w1, b1, w2, b2, w3, b3, w4, b4):
    g = N_NODES // BN
    oneh = pl.BlockSpec((1, DH), lambda i: (0, 0))
    return pl.pallas_call(
        _pre_pool_body,
        grid=(g,),
        in_specs=[
            pl.BlockSpec((BN, D_FEAT), lambda i: (i, 0)),
            pl.BlockSpec((D_FEAT, DH), lambda i: (0, 0)), oneh,
            pl.BlockSpec((DH, DH), lambda i: (0, 0)), oneh,
            pl.BlockSpec((DH, DH), lambda i: (0, 0)), oneh,
            pl.BlockSpec((DH, DH), lambda i: (0, 0)), oneh,
        ],
        out_specs=[oneh],
        out_shape=[jax.ShapeDtypeStruct((1, DH), f32)],
    )(x, w1, b1, w2, b2, w3, b3, w4, b4)[0]


def _final_body(hsum_ref, v0_ref, w1_ref, b1_ref, w2_ref, b2_ref, out_ref):
    pooled = hsum_ref[...] * (1.0 / N_NODES)
    c = _swish(jnp.dot(pooled, w1_ref[...], preferred_element_type=f32) + b1_ref[...])
    coeff = jnp.dot(c, w2_ref[...], preferred_element_type=f32) + b2_ref[...]
    out_ref[...] = v0_ref[...] * coeff


def _final(hsum, v0, w1, b1, w2, b2):
    return pl.pallas_call(
        _final_body,
        out_shape=jax.ShapeDtypeStruct((1, DH), f32),
    )(hsum, v0, w1, b1, w2, b2)


@functools.cache
def _mesh():
    return plsc.VectorSubcoreMesh(core_axis_name="c", subcore_axis_name="s",
                                  num_cores=NC, num_subcores=NS)


NPAIR = (NFULL - 1) // 2


def _sc_gather_body(a_hbm, b_hbm, dst_hbm, src_hbm, oa_hbm,
                    idxd0, idxs0, idxd1, idxs1, ra0, rb0, ra1, rb1, sem_g, sem_w):
    cid = lax.axis_index("c")
    sid = lax.axis_index("s")
    wid = sid * NC + cid
    base = wid * EPW
    sets = ((idxd0, idxs0, ra0, rb0, 0), (idxd1, idxs1, ra1, rb1, 1))

    def start(ci, s):
        idxd, idxs, ra, rb, b = s
        off = base + ci * CHUNK
        pltpu.sync_copy(dst_hbm.at[pl.ds(off, CHUNK)], idxd)
        pltpu.sync_copy(src_hbm.at[pl.ds(off, CHUNK)], idxs)
        pltpu.async_copy(a_hbm.at[idxd], ra, sem_g.at[b, 0])
        pltpu.async_copy(b_hbm.at[idxs], rb, sem_g.at[b, 1])

    def wait_gather(s):
        idxd, idxs, ra, rb, b = s
        pltpu.make_async_copy(a_hbm.at[idxd], ra, sem_g.at[b, 0]).wait()
        pltpu.make_async_copy(b_hbm.at[idxs], rb, sem_g.at[b, 1]).wait()

    def add_rows(s, k):
        _, _, ra, rb, _ = s

        @functools.partial(plsc.parallel_loop, 0, k, unroll=8)
        def _(r):
            for j in range(DP // 16):
                sl = (r, pl.ds(j * 16, 16))
                ra[sl] = ra[sl] + rb[sl]

    def wstart(ci, s):
        _, _, ra, _, b = s
        off = pl.ds(base + ci * CHUNK, CHUNK)
        pltpu.async_copy(ra, oa_hbm.at[off], sem_w.at[b])

    def wwait(ci, s):
        _, _, ra, _, b = s
        off = pl.ds(base + ci * CHUNK, CHUNK)
        pltpu.make_async_copy(ra, oa_hbm.at[off], sem_w.at[b]).wait()

    start(0, sets[0])
    start(1, sets[1])

    def body(p, carry):
        c0 = 2 * p
        wait_gather(sets[0])
        add_rows(sets[0], CHUNK)
        wstart(c0, sets[0])
        wait_gather(sets[1])
        add_rows(sets[1], CHUNK)
        wstart(c0 + 1, sets[1])

        @pl.when(p < NPAIR - 1)
        def _():
            wwait(c0, sets[0])
            start(c0 + 2, sets[0])
            wwait(c0 + 1, sets[1])
            start(c0 + 3, sets[1])

        return carry

    lax.fori_loop(0, NPAIR, body, 0)
    last0 = 2 * (NPAIR - 1)
    wwait(last0, sets[0])
    wwait(last0 + 1, sets[1])

    start(NFULL - 1, sets[0])
    wait_gather(sets[0])
    add_rows(sets[0], CHUNK)
    off = pl.ds(base + (NFULL - 1) * CHUNK, CHUNK)
    pltpu.sync_copy(ra0, oa_hbm.at[off])

    if TAIL:
        offt = base + NFULL * CHUNK
        id_d = idxd1.at[pl.ds(0, TAIL)]
        id_s = idxs1.at[pl.ds(0, TAIL)]
        pltpu.sync_copy(dst_hbm.at[pl.ds(offt, TAIL)], id_d)
        pltpu.sync_copy(src_hbm.at[pl.ds(offt, TAIL)], id_s)
        pltpu.async_copy(a_hbm.at[id_d], ra1.at[pl.ds(0, TAIL)], sem_g.at[1, 0]).wait()
        pltpu.async_copy(b_hbm.at[id_s], rb1.at[pl.ds(0, TAIL)], sem_g.at[1, 1]).wait()
        add_rows(sets[1], TAIL)
        pltpu.sync_copy(ra1.at[pl.ds(0, TAIL)], oa_hbm.at[pl.ds(offt, TAIL)])


def _gather_edges(a, b, dst, src):
    return pl.kernel(
        _sc_gather_body,
        out_type=jax.ShapeDtypeStruct((N_EDGES, DP), f32),
        mesh=_mesh(),
        scratch_types=[
            pltpu.VMEM((CHUNK,), jnp.int32),
            pltpu.VMEM((CHUNK,), jnp.int32),
            pltpu.VMEM((CHUNK,), jnp.int32),
            pltpu.VMEM((CHUNK,), jnp.int32),
            pltpu.VMEM((CHUNK, DP), f32),
            pltpu.VMEM((CHUNK, DP), f32),
            pltpu.VMEM((CHUNK, DP), f32),
            pltpu.VMEM((CHUNK, DP), f32),
            pltpu.SemaphoreType.DMA((2, 2)),
            pltpu.SemaphoreType.DMA((2,)),
        ],
    )(a, b, dst, src)


def _sc_scatter_body(msg_hbm, dst_hbm, out_hbm, idx, idx_t, rows, zbuf, shared,
                     sem_r, sem_s):
    cid = lax.axis_index("c")
    sid = lax.axis_index("s")
    base = sid * EPT
    nbase = cid * OWN

    zvec = jnp.zeros((16,), f32)

    def zrow(r, carry):
        for j in range(DP // 16):
            zbuf[r, pl.ds(j * 16, 16)] = zvec
        return carry

    lax.fori_loop(0, TSTRIPE, zrow, 0)
    pltpu.sync_copy(zbuf, shared.at[pl.ds(sid * TSTRIPE, TSTRIPE)])
    plsc.subcore_barrier()

    def remap(id_buf, b, k):
        for j in range(k // 16):
            sl = (b, pl.ds(j * 16, 16))
            local = id_buf[sl] - nbase
            ok = (local >= 0) & (local < OWN)
            id_buf[sl] = jnp.where(ok, local, TRASH)

    def start_chunk(ci, b):
        off = base + ci * CHUNK
        pltpu.sync_copy(dst_hbm.at[pl.ds(off, CHUNK)], idx.at[b])
        pltpu.async_copy(msg_hbm.at[pl.ds(off, CHUNK)], rows.at[b], sem_r.at[b])
        remap(idx, b, CHUNK)

    start_chunk(0, 0)

    def body(i, carry):
        b = lax.rem(i, 2)
        nb = 1 - b

        @pl.when(i + 1 < NF2)
        def _():
            @pl.when(i >= 1)
            def _():
                pltpu.make_async_copy(rows.at[nb], shared.at[idx.at[nb]],
                                      sem_s.at[nb]).wait()

            start_chunk(i + 1, nb)

        pltpu.make_async_copy(msg_hbm.at[pl.ds(0, CHUNK)], rows.at[b],
                              sem_r.at[b]).wait()
        pltpu.async_copy(rows.at[b], shared.at[idx.at[b]], sem_s.at[b], add=True)
        return carry

    lax.fori_loop(0, NF2, body, 0)
    pb = (NF2 - 1) % 2
    pltpu.make_async_copy(rows.at[pb], shared.at[idx.at[pb]], sem_s.at[pb]).wait()
    pltpu.make_async_copy(rows.at[1 - pb], shared.at[idx.at[1 - pb]],
                          sem_s.at[1 - pb]).wait()

    if TAIL2:
        off = base + NF2 * CHUNK
        pltpu.sync_copy(dst_hbm.at[pl.ds(off, TAIL2)], idx_t)
        pltpu.sync_copy(msg_hbm.at[pl.ds(off, TAIL2)], rows.at[0, pl.ds(0, TAIL2)])
        for j in range(TAIL2 // 16):
            sl = pl.ds(j * 16, 16)
            local = idx_t[sl] - nbase
            ok = (local >= 0) & (local < OWN)
            idx_t[sl] = jnp.where(ok, local, TRASH)
        pltpu.sync_copy(rows.at[0, pl.ds(0, TAIL2)], shared.at[idx_t], add=True)

    plsc.subcore_barrier()
    pltpu.sync_copy(shared.at[pl.ds(sid * TSTRIPE, TSTRIPE)],
                    out_hbm.at[cid, pl.ds(sid * TSTRIPE, TSTRIPE)])


def _scatter_msgs(msg, dst):
    return pl.kernel(
        _sc_scatter_body,
        out_type=jax.ShapeDtypeStruct((NC, TROWS, DP), f32),
        mesh=_mesh(),
        scratch_types=[
            pltpu.VMEM((2, CHUNK), jnp.int32),
            pltpu.VMEM((TAIL2,), jnp.int32),
            pltpu.VMEM((2, CHUNK, DP), f32),
            pltpu.VMEM((TSTRIPE, DP), f32),
            pltpu.VMEM_SHARED((TROWS, DP), f32),
            pltpu.SemaphoreType.DMA((2,)),
            pltpu.SemaphoreType.DMA((2,)),
        ],
    )(msg, dst)


def _msg_weights(mp):
    w1 = mp["l1"]["w"]
    wa = _pad2(w1[:, :D_FEAT].T, D_FEAT, DP)
    wb = _pad2(w1[:, D_FEAT:].T, DH, DP)
    b1 = _pad1(mp["l1"]["b"], DP)
    out = {"wa": wa, "wb": wb, "b1": b1}
    for k in ("2", "3"):
        out["w" + k] = _pad2(mp["l" + k]["w"].T, DP, DP).astype(bf16)
        out["b" + k] = _pad1(mp["l" + k]["b"], DP)
        out["g" + k] = _pad1(mp["bn" + k]["gamma"], DP)
        out["be" + k] = _pad1(mp["bn" + k]["beta"], DP)
    out["g1"] = _pad1(mp["bn1"]["gamma"], DP)
    out["be1"] = _pad1(mp["bn1"]["beta"], DP)
    out["w4"] = _pad2(mp["l4"]["w"].T, DP, DP).astype(bf16)
    out["b4"] = _pad1(mp["l4"]["b"], DP)
    return out


def _gnn_layer(x, v, dst, src, lp):
    mw = _msg_weights(lp["msg"])
    a, b = _node_pre(x, mw["wa"], mw["wb"], mw["b1"])
    e1 = _gather_edges(a, b, dst, src)
    gs = jnp.stack([mw["g1"], mw["g2"], mw["g3"]])
    bes = jnp.stack([mw["be1"], mw["be2"], mw["be3"]])
    ws = jnp.stack([mw["w2"], mw["w3"], mw["w4"]])
    bs = jnp.stack([mw["b2"], mw["b3"], mw["b4"]])
    msg = _edge_mlp(e1, gs, bes, ws, bs)
    aggs = _scatter_msgs(msg, dst)

    up = lp["upd"]
    w1u = up["l1"]["w"]
    wv = w1u[:, :DH].T
    wx = w1u[:, DH:DH + D_FEAT].T
    wa = _pad2(w1u[:, DH + D_FEAT:].T, DP, DH)
    return _update(
        x, v, aggs,
        wv, wx, wa, _pad1(up["l1"]["b"], DH),
        up["l2"]["w"].T, _pad1(up["l2"]["b"], DH),
        up["l3"]["w"].T, _pad1(up["l3"]["b"], DH),
        up["l4"]["w"].T, _pad1(up["l4"]["b"], D_FEAT),
    )


def kernel(node_feature, edge_index, vectors, params):
    x0 = node_feature[0]
    src = edge_index[0, 0]
    dst = edge_index[0, 1]
    v = x0[:, :DH]
    stacked = jax.tree.map(lambda *a: jnp.stack(a), *params["layers"])

    def _layer_step(xc, lp):
        return _gnn_layer(xc, v, dst, src, lp), None

    x, _ = lax.scan(_layer_step, x0, stacked)

    pp = params["pre"]
    hsum = _pre_pool(
        x,
        pp["l1"]["w"].T, _pad1(pp["l1"]["b"], DH),
        pp["l2"]["w"].T, _pad1(pp["l2"]["b"], DH),
        pp["l3"]["w"].T, _pad1(pp["l3"]["b"], DH),
        pp["l4"]["w"].T, _pad1(pp["l4"]["b"], DH),
    )
    qp = params["post"]
    out = _final(hsum, x0[0:1, :DH],
                 qp["l1"]["w"].T, _pad1(qp["l1"]["b"], DH),
                 qp["l2"]["w"].T, _pad1(qp["l2"]["b"], DH))
    return out.reshape((DH,))


# speedup vs baseline: 2.7265x; 1.0218x over previous
import functools

import jax
import jax.numpy as jnp
from jax import lax
from jax.experimental import pallas as pl
from jax.experimental.pallas import tpu as pltpu
from jax.experimental.pallas import tpu_sc as plsc

N_NODES = 10000
N_EDGES = 160000
D_FEAT = 300
DH = 100
DP = 128
DP2 = 64
NC, NS = 2, 16
NW = NC * NS
EPW = N_EDGES // NW
CHUNK = 128
NFULL = EPW // CHUNK
TAIL = EPW - NFULL * CHUNK
OWN = 6000
TROWS = 6016
TSTRIPE = TROWS // NS
TRASH = TROWS - 1
EPT = N_EDGES // NS
NF2 = EPT // CHUNK
TAIL2 = EPT - NF2 * CHUNK
BE = 4000
BN = 2000
EPS = 1e-5

f32 = jnp.float32
bf16 = jnp.bfloat16


def _swish(x):
    return x * lax.logistic(x)


def _pad2(w, r, c):
    return jnp.zeros((r, c), f32).at[: w.shape[0], : w.shape[1]].set(w)


def _pad1(b, n):
    return jnp.zeros((1, n), f32).at[0, : b.shape[0]].set(b)


def _node_pre_body(x_ref, wa_ref, wb_ref, b_ref, a_ref, bb_ref):
    xb = x_ref[...]
    a_ref[...] = jnp.dot(xb, wa_ref[...], preferred_element_type=f32) + b_ref[...]
    bb_ref[...] = jnp.dot(xb[:, :DH], wb_ref[...], preferred_element_type=f32)


def _node_pre(x, wa, wb, b):
    g = N_NODES // BN
    return pl.pallas_call(
        _node_pre_body,
        grid=(g,),
        in_specs=[
            pl.BlockSpec((BN, D_FEAT), lambda i: (i, 0)),
            pl.BlockSpec((D_FEAT, DP), lambda i: (0, 0)),
            pl.BlockSpec((DH, DP), lambda i: (0, 0)),
            pl.BlockSpec((1, DP), lambda i: (0, 0)),
        ],
        out_specs=[
            pl.BlockSpec((BN, DP), lambda i: (i, 0)),
            pl.BlockSpec((BN, DP), lambda i: (i, 0)),
        ],
        out_shape=[
            jax.ShapeDtypeStruct((N_NODES, DP), f32),
            jax.ShapeDtypeStruct((N_NODES, DP), f32),
        ],
    )(x, wa, wb, b)


NBLK = N_EDGES // BE


def _edge_mlp_body(ea_ref, g_ref, be_ref, w_ref, b_ref, out_ref,
                   scr, sums, sqs):
    s = pl.program_id(0)
    b = pl.program_id(1)

    @pl.when(s == 0)
    def _():
        e16 = ea_ref[...].astype(bf16)
        s1 = (e16 * lax.logistic(e16)).astype(f32)
        scr[b] = s1.astype(bf16)
        ps = jnp.sum(s1, axis=0, keepdims=True)
        pq = jnp.sum(s1 * s1, axis=0, keepdims=True)

        @pl.when(b == 0)
        def _():
            sums[0] = ps
            sqs[0] = pq

        @pl.when(b != 0)
        def _():
            sums[0] = sums[0] + ps
            sqs[0] = sqs[0] + pq

    @pl.when(s > 0)
    def _():
        sm1 = s - 1
        h = scr[b].astype(f32)
        m = sums[sm1] * (1.0 / N_EDGES)
        var = sqs[sm1] * (1.0 / N_EDGES) - m * m
        h = (h - m) * lax.rsqrt(var + EPS) * g_ref[0] + be_ref[0]
        z = jnp.dot(h.astype(bf16), w_ref[0], preferred_element_type=f32) + b_ref[0]
        z16 = z.astype(bf16)
        sv = (z16 * lax.logistic(z16)).astype(f32)

        @pl.when(s < 3)
        def _():
            scr[b] = sv.astype(bf16)
            ps = jnp.sum(sv, axis=0, keepdims=True)
            pq = jnp.sum(sv * sv, axis=0, keepdims=True)

            @pl.when(b == 0)
            def _():
                sums[s] = ps
                sqs[s] = pq

            @pl.when(b != 0)
            def _():
                sums[s] = sums[s] + ps
                sqs[s] = sqs[s] + pq

        @pl.when(s == 3)
        def _():
            col = lax.broadcasted_iota(jnp.int32, sv.shape, 1)
            out_ref[...] = jnp.where(col == DP - 1, 1.0, sv)


def _edge_mlp(ea, gs, bes, ws, bs):
    emap = lambda s, b: (jnp.where(s == 0, b, 0), 0)
    wm = lambda s, b: (jnp.maximum(s - 1, 0), 0, 0)
    return pl.pallas_call(
        _edge_mlp_body,
        grid=(4, NBLK),
        in_specs=[
            pl.BlockSpec((BE, DP), emap),
            pl.BlockSpec((1, 1, DP), wm),
            pl.BlockSpec((1, 1, DP), wm),
            pl.BlockSpec((1, DP, DP), wm),
            pl.BlockSpec((1, 1, DP), wm),
        ],
        out_specs=[pl.BlockSpec((BE, DP), lambda s, b: (jnp.where(s == 3, b, 0), 0))],
        out_shape=[jax.ShapeDtypeStruct((N_EDGES, DP), f32)],
        scratch_shapes=[
            pltpu.VMEM((NBLK, BE, DP), bf16),
            pltpu.VMEM((3, 1, DP), f32),
            pltpu.VMEM((3, 1, DP), f32),
        ],
        compiler_params=pltpu.CompilerParams(vmem_limit_bytes=100 * 1024 * 1024),
    )(ea, gs, bes, ws, bs)[0]


def _s1_body(ea_ref, eb_ref, sum_ref, sq_ref):
    i = pl.program_id(0)
    s = _swish(ea_ref[...].astype(f32) + eb_ref[...].astype(f32))
    ps = jnp.sum(s, axis=0, keepdims=True)
    pq = jnp.sum(s * s, axis=0, keepdims=True)

    @pl.when(i == 0)
    def _():
        sum_ref[...] = ps
        sq_ref[...] = pq

    @pl.when(i != 0)
    def _():
        sum_ref[...] = sum_ref[...] + ps
        sq_ref[...] = sq_ref[...] + pq


def _s1_stats(ea, eb):
    g = N_EDGES // BE
    return pl.pallas_call(
        _s1_body,
        grid=(g,),
        in_specs=[pl.BlockSpec((BE, DP), lambda i: (i, 0)),
                  pl.BlockSpec((BE, DP), lambda i: (i, 0))],
        out_specs=[
            pl.BlockSpec((1, DP), lambda i: (0, 0)),
            pl.BlockSpec((1, DP), lambda i: (0, 0)),
        ],
        out_shape=[
            jax.ShapeDtypeStruct((1, DP), f32),
            jax.ShapeDtypeStruct((1, DP), f32),
        ],
    )(ea, eb)


def _stage(xs, stats, gamma, beta, w, b, *, pre_swish, track_stats, ones_col=False):
    g = N_EDGES // BE
    one = pl.BlockSpec((1, DP), lambda i: (0, 0))
    n_in = len(xs)

    def body(*refs):
        in_refs = refs[:n_in]
        sum_ref, sq_ref, g_ref, be_ref, w_ref, b_ref = refs[n_in:n_in + 6]
        outs = refs[n_in + 6:]
        i = pl.program_id(0)
        h = in_refs[0][...].astype(f32)
        for r in in_refs[1:]:
            h = h + r[...].astype(f32)
        if pre_swish:
            h = _swish(h)
        m = sum_ref[...] * (1.0 / N_EDGES)
        var = sq_ref[...] * (1.0 / N_EDGES) - m * m
        h = (h - m) * lax.rsqrt(var + EPS) * g_ref[...] + be_ref[...]
        z = jnp.dot(h.astype(bf16), w_ref[...], preferred_element_type=f32) + b_ref[...]
        s = _swish(z)
        if ones_col:
            col = lax.broadcasted_iota(jnp.int32, s.shape, 1)
            s = jnp.where(col == DP - 1, 1.0, s)
        outs[0][...] = s if ones_col else s.astype(bf16)
        if track_stats:
            ps = jnp.sum(s, axis=0, keepdims=True)
            pq = jnp.sum(s * s, axis=0, keepdims=True)

            @pl.when(i == 0)
            def _():
                outs[1][...] = ps
                outs[2][...] = pq

            @pl.when(i != 0)
            def _():
                outs[1][...] = outs[1][...] + ps
                outs[2][...] = outs[2][...] + pq

    out_specs = [pl.BlockSpec((BE, DP), lambda i: (i, 0)), one, one]
    out_shape = [
        jax.ShapeDtypeStruct((N_EDGES, DP), f32 if ones_col else bf16),
        jax.ShapeDtypeStruct((1, DP), f32),
        jax.ShapeDtypeStruct((1, DP), f32),
    ]
    if not track_stats:
        out_specs, out_shape = out_specs[:1], out_shape[:1]
    return pl.pallas_call(
        body,
        grid=(g,),
        in_specs=[pl.BlockSpec((BE, DP), lambda i: (i, 0))] * n_in
                 + [one, one, one, one,
                    pl.BlockSpec((DP, DP), lambda i: (0, 0)), one],
        out_specs=out_specs,
        out_shape=out_shape,
    )(*xs, stats[0], stats[1], gamma, beta, w, b)


def _update_body(x_ref, v_ref, a_ref, wv_ref, wx_ref, wa_ref, b1_ref,
                 w2_ref, b2_ref, w3_ref, b3_ref, w4_ref, b4_ref, out_ref):
    xb = x_ref[...]
    acc = a_ref[0]
    cnt = jnp.maximum(acc[:, DP - 1 : DP], 1.0)
    agg = acc / cnt
    h = (jnp.dot(v_ref[...], wv_ref[...], preferred_element_type=f32)
         + jnp.dot(xb, wx_ref[...], preferred_element_type=f32)
         + jnp.dot(agg, wa_ref[...], preferred_element_type=f32)
         + b1_ref[...])
    h = _swish(h)
    h = _swish(jnp.dot(h, w2_ref[...], preferred_element_type=f32) + b2_ref[...])
    h = _swish(jnp.dot(h, w3_ref[...], preferred_element_type=f32) + b3_ref[...])
    upd = _swish(jnp.dot(h, w4_ref[...], preferred_element_type=f32) + b4_ref[...])
    out_ref[...] = xb + upd


def _update(x, v, aggs, wv, wx, wa, b1, w2, b2, w3, b3, w4, b4):
    g = N_NODES // BN
    oneh = pl.BlockSpec((1, DH), lambda i: (0, 0))
    return pl.pallas_call(
        _update_body,
        grid=(g,),
        in_specs=[
            pl.BlockSpec((BN, D_FEAT), lambda i: (i, 0)),
            pl.BlockSpec((BN, DH), lambda i: (i, 0)),
            pl.BlockSpec((1, BN, DP), lambda i: (i // 3, i - 3 * (i // 3), 0)),
            pl.BlockSpec((DH, DH), lambda i: (0, 0)),
            pl.BlockSpec((D_FEAT, DH), lambda i: (0, 0)),
            pl.BlockSpec((DP, DH), lambda i: (0, 0)),
            oneh,
            pl.BlockSpec((DH, DH), lambda i: (0, 0)), oneh,
            pl.BlockSpec((DH, DH), lambda i: (0, 0)), oneh,
            pl.BlockSpec((DH, D_FEAT), lambda i: (0, 0)),
            pl.BlockSpec((1, D_FEAT), lambda i: (0, 0)),
        ],
        out_specs=[pl.BlockSpec((BN, D_FEAT), lambda i: (i, 0))],
        out_shape=[jax.ShapeDtypeStruct((N_NODES, D_FEAT), f32)],
        input_output_aliases={0: 0},
    )(x, v, aggs, wv, wx, wa, b1, w2, b2, w3, b3, w4, b4)[0]


def _pre_pool_body(x_ref, w1_ref, b1_ref, w2_ref, b2_ref, w3_ref, b3_ref,
                   w4_ref, b4_ref, sum_ref):
    i = pl.program_id(0)
    h = _swish(jnp.dot(x_ref[...], w1_ref[...], preferred_element_type=f32) + b1_ref[...])
    h = _swish(jnp.dot(h, w2_ref[...], preferred_element_type=f32) + b2_ref[...])
    h = _swish(jnp.dot(h, w3_ref[...], preferred_element_type=f32) + b3_ref[...])
    h = jnp.dot(h, w4_ref[...], preferred_element_type=f32) + b4_ref[...]
    ps = jnp.sum(h, axis=0, keepdims=True)

    @pl.when(i == 0)
    def _():
        sum_ref[...] = ps

    @pl.when(i != 0)
    def _():
        sum_ref[...] = sum_ref[...] + ps


def _pre_pool(x, w1, b1, w2, b2, w3, b3, w4, b4):
    g = N_NODES // BN
    oneh = pl.BlockSpec((1, DH), lambda i: (0, 0))
    return pl.pallas_call(
        _pre_pool_body,
        grid=(g,),
        in_specs=[
            pl.BlockSpec((BN, D_FEAT), lambda i: (i, 0)),
            pl.BlockSpec((D_FEAT, DH), lambda i: (0, 0)), oneh,
            pl.BlockSpec((DH, DH), lambda i: (0, 0)), oneh,
            pl.BlockSpec((DH, DH), lambda i: (0, 0)), oneh,
            pl.BlockSpec((DH, DH), lambda i: (0, 0)), oneh,
        ],
        out_specs=[oneh],
        out_shape=[jax.ShapeDtypeStruct((1, DH), f32)],
    )(x, w1, b1, w2, b2, w3, b3, w4, b4)[0]


def _final_body(hsum_ref, v0_ref, w1_ref, b1_ref, w2_ref, b2_ref, out_ref):
    pooled = hsum_ref[...] * (1.0 / N_NODES)
    c = _swish(jnp.dot(pooled, w1_ref[...], preferred_element_type=f32) + b1_ref[...])
    coeff = jnp.dot(c, w2_ref[...], preferred_element_type=f32) + b2_ref[...]
    out_ref[...] = v0_ref[...] * coeff


def _final(hsum, v0, w1, b1, w2, b2):
    return pl.pallas_call(
        _final_body,
        out_shape=jax.ShapeDtypeStruct((1, DH), f32),
    )(hsum, v0, w1, b1, w2, b2)


@functools.cache
def _mesh():
    return plsc.VectorSubcoreMesh(core_axis_name="c", subcore_axis_name="s",
                                  num_cores=NC, num_subcores=NS)


NPAIR = (NFULL - 1) // 2


def _sc_gather_body(a_hbm, b_hbm, dst_hbm, src_hbm, oa_hbm,
                    idxd0, idxs0, idxd1, idxs1, ra0, rb0, ra1, rb1, sem_g, sem_w):
    cid = lax.axis_index("c")
    sid = lax.axis_index("s")
    wid = sid * NC + cid
    base = wid * EPW
    sets = ((idxd0, idxs0, ra0, rb0, 0), (idxd1, idxs1, ra1, rb1, 1))

    def start(ci, s):
        idxd, idxs, ra, rb, b = s
        off = base + ci * CHUNK
        pltpu.sync_copy(dst_hbm.at[pl.ds(off, CHUNK)], idxd)
        pltpu.sync_copy(src_hbm.at[pl.ds(off, CHUNK)], idxs)
        pltpu.async_copy(a_hbm.at[idxd], ra, sem_g.at[b, 0])
        pltpu.async_copy(b_hbm.at[idxs], rb, sem_g.at[b, 1])

    def wait_gather(s):
        idxd, idxs, ra, rb, b = s
        pltpu.make_async_copy(a_hbm.at[idxd], ra, sem_g.at[b, 0]).wait()
        pltpu.make_async_copy(b_hbm.at[idxs], rb, sem_g.at[b, 1]).wait()

    def add_rows(s, k):
        _, _, ra, rb, _ = s

        @functools.partial(plsc.parallel_loop, 0, k, unroll=8)
        def _(r):
            for j in range(DP // 16):
                sl = (r, pl.ds(j * 16, 16))
                ra[sl] = ra[sl] + rb[sl]

    def wstart(ci, s):
        _, _, ra, _, b = s
        off = pl.ds(base + ci * CHUNK, CHUNK)
        pltpu.async_copy(ra, oa_hbm.at[off], sem_w.at[b])

    def wwait(ci, s):
        _, _, ra, _, b = s
        off = pl.ds(base + ci * CHUNK, CHUNK)
        pltpu.make_async_copy(ra, oa_hbm.at[off], sem_w.at[b]).wait()

    start(0, sets[0])
    start(1, sets[1])

    def body(p, carry):
        c0 = 2 * p
        wait_gather(sets[0])
        add_rows(sets[0], CHUNK)
        wstart(c0, sets[0])
        wait_gather(sets[1])
        add_rows(sets[1], CHUNK)
        wstart(c0 + 1, sets[1])

        @pl.when(p < NPAIR - 1)
        def _():
            wwait(c0, sets[0])
            start(c0 + 2, sets[0])
            wwait(c0 + 1, sets[1])
            start(c0 + 3, sets[1])

        return carry

    lax.fori_loop(0, NPAIR, body, 0)
    last0 = 2 * (NPAIR - 1)
    wwait(last0, sets[0])
    wwait(last0 + 1, sets[1])

    start(NFULL - 1, sets[0])
    wait_gather(sets[0])
    add_rows(sets[0], CHUNK)
    off = pl.ds(base + (NFULL - 1) * CHUNK, CHUNK)
    pltpu.sync_copy(ra0, oa_hbm.at[off])

    if TAIL:
        offt = base + NFULL * CHUNK
        id_d = idxd1.at[pl.ds(0, TAIL)]
        id_s = idxs1.at[pl.ds(0, TAIL)]
        pltpu.sync_copy(dst_hbm.at[pl.ds(offt, TAIL)], id_d)
        pltpu.sync_copy(src_hbm.at[pl.ds(offt, TAIL)], id_s)
        pltpu.async_copy(a_hbm.at[id_d], ra1.at[pl.ds(0, TAIL)], sem_g.at[1, 0]).wait()
        pltpu.async_copy(b_hbm.at[id_s], rb1.at[pl.ds(0, TAIL)], sem_g.at[1, 1]).wait()
        add_rows(sets[1], TAIL)
        pltpu.sync_copy(ra1.at[pl.ds(0, TAIL)], oa_hbm.at[pl.ds(offt, TAIL)])


def _gather_edges(a, b, dst, src):
    return pl.kernel(
        _sc_gather_body,
        out_type=jax.ShapeDtypeStruct((N_EDGES, DP), f32),
        mesh=_mesh(),
        scratch_types=[
            pltpu.VMEM((CHUNK,), jnp.int32),
            pltpu.VMEM((CHUNK,), jnp.int32),
            pltpu.VMEM((CHUNK,), jnp.int32),
            pltpu.VMEM((CHUNK,), jnp.int32),
            pltpu.VMEM((CHUNK, DP), f32),
            pltpu.VMEM((CHUNK, DP), f32),
            pltpu.VMEM((CHUNK, DP), f32),
            pltpu.VMEM((CHUNK, DP), f32),
            pltpu.SemaphoreType.DMA((2, 2)),
            pltpu.SemaphoreType.DMA((2,)),
        ],
    )(a, b, dst, src)


def _sc_scatter_body(msg_hbm, dst_hbm, out_hbm, idx, idx_t, rows, zbuf, shared,
                     sem_r, sem_s):
    cid = lax.axis_index("c")
    sid = lax.axis_index("s")
    base = sid * EPT
    nbase = cid * OWN

    zvec = jnp.zeros((16,), f32)

    def zrow(r, carry):
        for j in range(DP // 16):
            zbuf[r, pl.ds(j * 16, 16)] = zvec
        return carry

    lax.fori_loop(0, TSTRIPE, zrow, 0)
    pltpu.sync_copy(zbuf, shared.at[pl.ds(sid * TSTRIPE, TSTRIPE)])
    plsc.subcore_barrier()

    def remap(id_buf, b, k):
        for j in range(k // 16):
            sl = (b, pl.ds(j * 16, 16))
            local = id_buf[sl] - nbase
            ok = (local >= 0) & (local < OWN)
            id_buf[sl] = jnp.where(ok, local, TRASH)

    def start_chunk(ci, b):
        off = base + ci * CHUNK
        pltpu.sync_copy(dst_hbm.at[pl.ds(off, CHUNK)], idx.at[b])
        pltpu.async_copy(msg_hbm.at[pl.ds(off, CHUNK)], rows.at[b], sem_r.at[b])
        remap(idx, b, CHUNK)

    start_chunk(0, 0)

    def body(i, carry):
        b = lax.rem(i, 2)
        nb = 1 - b

        @pl.when(i + 1 < NF2)
        def _():
            @pl.when(i >= 1)
            def _():
                pltpu.make_async_copy(rows.at[nb], shared.at[idx.at[nb]],
                                      sem_s.at[nb]).wait()

            start_chunk(i + 1, nb)

        pltpu.make_async_copy(msg_hbm.at[pl.ds(0, CHUNK)], rows.at[b],
                              sem_r.at[b]).wait()
        pltpu.async_copy(rows.at[b], shared.at[idx.at[b]], sem_s.at[b], add=True)
        return carry

    lax.fori_loop(0, NF2, body, 0)
    pb = (NF2 - 1) % 2
    pltpu.make_async_copy(rows.at[pb], shared.at[idx.at[pb]], sem_s.at[pb]).wait()
    pltpu.make_async_copy(rows.at[1 - pb], shared.at[idx.at[1 - pb]],
                          sem_s.at[1 - pb]).wait()

    if TAIL2:
        off = base + NF2 * CHUNK
        pltpu.sync_copy(dst_hbm.at[pl.ds(off, TAIL2)], idx_t)
        pltpu.sync_copy(msg_hbm.at[pl.ds(off, TAIL2)], rows.at[0, pl.ds(0, TAIL2)])
        for j in range(TAIL2 // 16):
            sl = pl.ds(j * 16, 16)
            local = idx_t[sl] - nbase
            ok = (local >= 0) & (local < OWN)
            idx_t[sl] = jnp.where(ok, local, TRASH)
        pltpu.sync_copy(rows.at[0, pl.ds(0, TAIL2)], shared.at[idx_t], add=True)

    plsc.subcore_barrier()
    pltpu.sync_copy(shared.at[pl.ds(sid * TSTRIPE, TSTRIPE)],
                    out_hbm.at[cid, pl.ds(sid * TSTRIPE, TSTRIPE)])


def _scatter_msgs(msg, dst):
    return pl.kernel(
        _sc_scatter_body,
        out_type=jax.ShapeDtypeStruct((NC, TROWS, DP), f32),
        mesh=_mesh(),
        scratch_types=[
            pltpu.VMEM((2, CHUNK), jnp.int32),
            pltpu.VMEM((TAIL2,), jnp.int32),
            pltpu.VMEM((2, CHUNK, DP), f32),
            pltpu.VMEM((TSTRIPE, DP), f32),
            pltpu.VMEM_SHARED((TROWS, DP), f32),
            pltpu.SemaphoreType.DMA((2,)),
            pltpu.SemaphoreType.DMA((2,)),
        ],
    )(msg, dst)


def _msg_weights(mp):
    w1 = mp["l1"]["w"]
    wa = _pad2(w1[:, :D_FEAT].T, D_FEAT, DP)
    wb = _pad2(w1[:, D_FEAT:].T, DH, DP)
    b1 = _pad1(mp["l1"]["b"], DP)
    out = {"wa": wa, "wb": wb, "b1": b1}
    for k in ("2", "3"):
        out["w" + k] = _pad2(mp["l" + k]["w"].T, DP, DP).astype(bf16)
        out["b" + k] = _pad1(mp["l" + k]["b"], DP)
        out["g" + k] = _pad1(mp["bn" + k]["gamma"], DP)
        out["be" + k] = _pad1(mp["bn" + k]["beta"], DP)
    out["g1"] = _pad1(mp["bn1"]["gamma"], DP)
    out["be1"] = _pad1(mp["bn1"]["beta"], DP)
    out["w4"] = _pad2(mp["l4"]["w"].T, DP, DP).astype(bf16)
    out["b4"] = _pad1(mp["l4"]["b"], DP)
    return out


def _gnn_layer(x, v, dst, src, lp):
    mw = _msg_weights(lp["msg"])
    a, b = _node_pre(x, mw["wa"], mw["wb"], mw["b1"])
    e1 = _gather_edges(a, b, dst, src)
    gs = jnp.stack([mw["g1"], mw["g2"], mw["g3"]])
    bes = jnp.stack([mw["be1"], mw["be2"], mw["be3"]])
    ws = jnp.stack([mw["w2"], mw["w3"], mw["w4"]])
    bs = jnp.stack([mw["b2"], mw["b3"], mw["b4"]])
    msg = _edge_mlp(e1, gs, bes, ws, bs)
    aggs = _scatter_msgs(msg, dst)

    up = lp["upd"]
    w1u = up["l1"]["w"]
    wv = w1u[:, :DH].T
    wx = w1u[:, DH:DH + D_FEAT].T
    wa = _pad2(w1u[:, DH + D_FEAT:].T, DP, DH)
    return _update(
        x, v, aggs,
        wv, wx, wa, _pad1(up["l1"]["b"], DH),
        up["l2"]["w"].T, _pad1(up["l2"]["b"], DH),
        up["l3"]["w"].T, _pad1(up["l3"]["b"], DH),
        up["l4"]["w"].T, _pad1(up["l4"]["b"], D_FEAT),
    )


def kernel(node_feature, edge_index, vectors, params):
    x0 = node_feature[0]
    src = edge_index[0, 0]
    dst = edge_index[0, 1]
    v = x0[:, :DH]
    stacked = jax.tree.map(lambda *a: jnp.stack(a), *params["layers"])

    def _layer_step(xc, lp):
        return _gnn_layer(xc, v, dst, src, lp), None

    x, _ = lax.scan(_layer_step, x0, stacked)

    pp = params["pre"]
    hsum = _pre_pool(
        x,
        pp["l1"]["w"].T, _pad1(pp["l1"]["b"], DH),
        pp["l2"]["w"].T, _pad1(pp["l2"]["b"], DH),
        pp["l3"]["w"].T, _pad1(pp["l3"]["b"], DH),
        pp["l4"]["w"].T, _pad1(pp["l4"]["b"], DH),
    )
    qp = params["post"]
    out = _final(hsum, x0[0:1, :DH],
                 qp["l1"]["w"].T, _pad1(qp["l1"]["b"], DH),
                 qp["l2"]["w"].T, _pad1(qp["l2"]["b"], DH))
    return out.reshape((DH,))


# speedup vs baseline: 2.7859x; 1.0218x over previous
import functools

import jax
import jax.numpy as jnp
from jax import lax
from jax.experimental import pallas as pl
from jax.experimental.pallas import tpu as pltpu
from jax.experimental.pallas import tpu_sc as plsc

N_NODES = 10000
N_EDGES = 160000
D_FEAT = 300
DH = 100
DP = 128
DP2 = 64
NC, NS = 2, 16
NW = NC * NS
EPW = N_EDGES // NW
CHUNK = 128
NFULL = EPW // CHUNK
TAIL = EPW - NFULL * CHUNK
OWN = 6000
TROWS = 6016
TSTRIPE = TROWS // NS
TRASH = TROWS - 1
EPT = N_EDGES // NS
NF2 = EPT // CHUNK
TAIL2 = EPT - NF2 * CHUNK
BE = 4000
BN = 2000
EPS = 1e-5

f32 = jnp.float32
bf16 = jnp.bfloat16


def _swish(x):
    return x * lax.logistic(x)


def _pad2(w, r, c):
    return jnp.zeros((r, c), f32).at[: w.shape[0], : w.shape[1]].set(w)


def _pad1(b, n):
    return jnp.zeros((1, n), f32).at[0, : b.shape[0]].set(b)


def _node_pre_body(x_ref, wa_ref, wb_ref, b_ref, a_ref, bb_ref):
    xb = x_ref[...]
    a_ref[...] = jnp.dot(xb, wa_ref[...], preferred_element_type=f32) + b_ref[...]
    bb_ref[...] = jnp.dot(xb[:, :DH], wb_ref[...], preferred_element_type=f32)


def _node_pre(x, wa, wb, b):
    g = N_NODES // BN
    return pl.pallas_call(
        _node_pre_body,
        grid=(g,),
        in_specs=[
            pl.BlockSpec((BN, D_FEAT), lambda i: (i, 0)),
            pl.BlockSpec((D_FEAT, DP), lambda i: (0, 0)),
            pl.BlockSpec((DH, DP), lambda i: (0, 0)),
            pl.BlockSpec((1, DP), lambda i: (0, 0)),
        ],
        out_specs=[
            pl.BlockSpec((BN, DP), lambda i: (i, 0)),
            pl.BlockSpec((BN, DP), lambda i: (i, 0)),
        ],
        out_shape=[
            jax.ShapeDtypeStruct((N_NODES, DP), f32),
            jax.ShapeDtypeStruct((N_NODES, DP), f32),
        ],
    )(x, wa, wb, b)


NBLK = N_EDGES // BE


def _edge_mlp_body(ea_ref, g_ref, be_ref, w_ref, b_ref, out_ref,
                   scr, sums, sqs):
    s = pl.program_id(0)
    b = pl.program_id(1)

    def accum(si, sv16):
        ps = jnp.sum(sv16.astype(f32), axis=0, keepdims=True)
        pq = jnp.sum((sv16 * sv16).astype(f32), axis=0, keepdims=True)

        @pl.when(b == 0)
        def _():
            sums[si] = ps
            sqs[si] = pq

        @pl.when(b != 0)
        def _():
            sums[si] = sums[si] + ps
            sqs[si] = sqs[si] + pq

    @pl.when(s == 0)
    def _():
        e16 = ea_ref[...].astype(bf16)
        s1 = e16 * lax.logistic(e16)
        scr[b] = s1
        accum(0, s1)

    @pl.when(s > 0)
    def _():
        sm1 = s - 1
        m = sums[sm1] * (1.0 / N_EDGES)
        var = sqs[sm1] * (1.0 / N_EDGES) - m * m
        scale = lax.rsqrt(var + EPS) * g_ref[0]
        shift = be_ref[0] - m * scale
        hn = scr[b] * scale.astype(bf16) + shift.astype(bf16)
        z = jnp.dot(hn, w_ref[0], preferred_element_type=f32) + b_ref[0]
        z16 = z.astype(bf16)
        sv16 = z16 * lax.logistic(z16)

        @pl.when(s < 3)
        def _():
            scr[b] = sv16
            accum(s, sv16)

        @pl.when(s == 3)
        def _():
            sv = sv16.astype(f32)
            col = lax.broadcasted_iota(jnp.int32, sv.shape, 1)
            out_ref[...] = jnp.where(col == DP - 1, 1.0, sv)


def _edge_mlp(ea, gs, bes, ws, bs):
    emap = lambda s, b: (jnp.where(s == 0, b, 0), 0)
    wm = lambda s, b: (jnp.maximum(s - 1, 0), 0, 0)
    return pl.pallas_call(
        _edge_mlp_body,
        grid=(4, NBLK),
        in_specs=[
            pl.BlockSpec((BE, DP), emap),
            pl.BlockSpec((1, 1, DP), wm),
            pl.BlockSpec((1, 1, DP), wm),
            pl.BlockSpec((1, DP, DP), wm),
            pl.BlockSpec((1, 1, DP), wm),
        ],
        out_specs=[pl.BlockSpec((BE, DP), lambda s, b: (jnp.where(s == 3, b, 0), 0))],
        out_shape=[jax.ShapeDtypeStruct((N_EDGES, DP), f32)],
        scratch_shapes=[
            pltpu.VMEM((NBLK, BE, DP), bf16),
            pltpu.VMEM((3, 1, DP), f32),
            pltpu.VMEM((3, 1, DP), f32),
        ],
        compiler_params=pltpu.CompilerParams(vmem_limit_bytes=100 * 1024 * 1024),
    )(ea, gs, bes, ws, bs)[0]


def _s1_body(ea_ref, eb_ref, sum_ref, sq_ref):
    i = pl.program_id(0)
    s = _swish(ea_ref[...].astype(f32) + eb_ref[...].astype(f32))
    ps = jnp.sum(s, axis=0, keepdims=True)
    pq = jnp.sum(s * s, axis=0, keepdims=True)

    @pl.when(i == 0)
    def _():
        sum_ref[...] = ps
        sq_ref[...] = pq

    @pl.when(i != 0)
    def _():
        sum_ref[...] = sum_ref[...] + ps
        sq_ref[...] = sq_ref[...] + pq


def _s1_stats(ea, eb):
    g = N_EDGES // BE
    return pl.pallas_call(
        _s1_body,
        grid=(g,),
        in_specs=[pl.BlockSpec((BE, DP), lambda i: (i, 0)),
                  pl.BlockSpec((BE, DP), lambda i: (i, 0))],
        out_specs=[
            pl.BlockSpec((1, DP), lambda i: (0, 0)),
            pl.BlockSpec((1, DP), lambda i: (0, 0)),
        ],
        out_shape=[
            jax.ShapeDtypeStruct((1, DP), f32),
            jax.ShapeDtypeStruct((1, DP), f32),
        ],
    )(ea, eb)


def _stage(xs, stats, gamma, beta, w, b, *, pre_swish, track_stats, ones_col=False):
    g = N_EDGES // BE
    one = pl.BlockSpec((1, DP), lambda i: (0, 0))
    n_in = len(xs)

    def body(*refs):
        in_refs = refs[:n_in]
        sum_ref, sq_ref, g_ref, be_ref, w_ref, b_ref = refs[n_in:n_in + 6]
        outs = refs[n_in + 6:]
        i = pl.program_id(0)
        h = in_refs[0][...].astype(f32)
        for r in in_refs[1:]:
            h = h + r[...].astype(f32)
        if pre_swish:
            h = _swish(h)
        m = sum_ref[...] * (1.0 / N_EDGES)
        var = sq_ref[...] * (1.0 / N_EDGES) - m * m
        h = (h - m) * lax.rsqrt(var + EPS) * g_ref[...] + be_ref[...]
        z = jnp.dot(h.astype(bf16), w_ref[...], preferred_element_type=f32) + b_ref[...]
        s = _swish(z)
        if ones_col:
            col = lax.broadcasted_iota(jnp.int32, s.shape, 1)
            s = jnp.where(col == DP - 1, 1.0, s)
        outs[0][...] = s if ones_col else s.astype(bf16)
        if track_stats:
            ps = jnp.sum(s, axis=0, keepdims=True)
            pq = jnp.sum(s * s, axis=0, keepdims=True)

            @pl.when(i == 0)
            def _():
                outs[1][...] = ps
                outs[2][...] = pq

            @pl.when(i != 0)
            def _():
                outs[1][...] = outs[1][...] + ps
                outs[2][...] = outs[2][...] + pq

    out_specs = [pl.BlockSpec((BE, DP), lambda i: (i, 0)), one, one]
    out_shape = [
        jax.ShapeDtypeStruct((N_EDGES, DP), f32 if ones_col else bf16),
        jax.ShapeDtypeStruct((1, DP), f32),
        jax.ShapeDtypeStruct((1, DP), f32),
    ]
    if not track_stats:
        out_specs, out_shape = out_specs[:1], out_shape[:1]
    return pl.pallas_call(
        body,
        grid=(g,),
        in_specs=[pl.BlockSpec((BE, DP), lambda i: (i, 0))] * n_in
                 + [one, one, one, one,
                    pl.BlockSpec((DP, DP), lambda i: (0, 0)), one],
        out_specs=out_specs,
        out_shape=out_shape,
    )(*xs, stats[0], stats[1], gamma, beta, w, b)


def _update_body(x_ref, v_ref, a_ref, wv_ref, wx_ref, wa_ref, b1_ref,
                 w2_ref, b2_ref, w3_ref, b3_ref, w4_ref, b4_ref, out_ref):
    xb = x_ref[...]
    acc = a_ref[0]
    cnt = jnp.maximum(acc[:, DP - 1 : DP], 1.0)
    agg = acc / cnt
    h = (jnp.dot(v_ref[...], wv_ref[...], preferred_element_type=f32)
         + jnp.dot(xb, wx_ref[...], preferred_element_type=f32)
         + jnp.dot(agg, wa_ref[...], preferred_element_type=f32)
         + b1_ref[...])
    h = _swish(h)
    h = _swish(jnp.dot(h, w2_ref[...], preferred_element_type=f32) + b2_ref[...])
    h = _swish(jnp.dot(h, w3_ref[...], preferred_element_type=f32) + b3_ref[...])
    upd = _swish(jnp.dot(h, w4_ref[...], preferred_element_type=f32) + b4_ref[...])
    out_ref[...] = xb + upd


def _update(x, v, aggs, wv, wx, wa, b1, w2, b2, w3, b3, w4, b4):
    g = N_NODES // BN
    oneh = pl.BlockSpec((1, DH), lambda i: (0, 0))
    return pl.pallas_call(
        _update_body,
        grid=(g,),
        in_specs=[
            pl.BlockSpec((BN, D_FEAT), lambda i: (i, 0)),
            pl.BlockSpec((BN, DH), lambda i: (i, 0)),
            pl.BlockSpec((1, BN, DP), lambda i: (i // 3, i - 3 * (i // 3), 0)),
            pl.BlockSpec((DH, DH), lambda i: (0, 0)),
            pl.BlockSpec((D_FEAT, DH), lambda i: (0, 0)),
            pl.BlockSpec((DP, DH), lambda i: (0, 0)),
            oneh,
            pl.BlockSpec((DH, DH), lambda i: (0, 0)), oneh,
            pl.BlockSpec((DH, DH), lambda i: (0, 0)), oneh,
            pl.BlockSpec((DH, D_FEAT), lambda i: (0, 0)),
            pl.BlockSpec((1, D_FEAT), lambda i: (0, 0)),
        ],
        out_specs=[pl.BlockSpec((BN, D_FEAT), lambda i: (i, 0))],
        out_shape=[jax.ShapeDtypeStruct((N_NODES, D_FEAT), f32)],
        input_output_aliases={0: 0},
    )(x, v, aggs, wv, wx, wa, b1, w2, b2, w3, b3, w4, b4)[0]


def _pre_pool_body(x_ref, w1_ref, b1_ref, w2_ref, b2_ref, w3_ref, b3_ref,
                   w4_ref, b4_ref, sum_ref):
    i = pl.program_id(0)
    h = _swish(jnp.dot(x_ref[...], w1_ref[...], preferred_element_type=f32) + b1_ref[...])
    h = _swish(jnp.dot(h, w2_ref[...], preferred_element_type=f32) + b2_ref[...])
    h = _swish(jnp.dot(h, w3_ref[...], preferred_element_type=f32) + b3_ref[...])
    h = jnp.dot(h, w4_ref[...], preferred_element_type=f32) + b4_ref[...]
    ps = jnp.sum(h, axis=0, keepdims=True)

    @pl.when(i == 0)
    def _():
        sum_ref[...] = ps

    @pl.when(i != 0)
    def _():
        sum_ref[...] = sum_ref[...] + ps


def _pre_pool(x, w1, b1, w2, b2, w3, b3, w4, b4):
    g = N_NODES // BN
    oneh = pl.BlockSpec((1, DH), lambda i: (0, 0))
    return pl.pallas_call(
        _pre_pool_body,
        grid=(g,),
        in_specs=[
            pl.BlockSpec((BN, D_FEAT), lambda i: (i, 0)),
            pl.BlockSpec((D_FEAT, DH), lambda i: (0, 0)), oneh,
            pl.BlockSpec((DH, DH), lambda i: (0, 0)), oneh,
            pl.BlockSpec((DH, DH), lambda i: (0, 0)), oneh,
            pl.BlockSpec((DH, DH), lambda i: (0, 0)), oneh,
        ],
        out_specs=[oneh],
        out_shape=[jax.ShapeDtypeStruct((1, DH), f32)],
    )(x, w1, b1, w2, b2, w3, b3, w4, b4)[0]


def _final_body(hsum_ref, v0_ref, w1_ref, b1_ref, w2_ref, b2_ref, out_ref):
    pooled = hsum_ref[...] * (1.0 / N_NODES)
    c = _swish(jnp.dot(pooled, w1_ref[...], preferred_element_type=f32) + b1_ref[...])
    coeff = jnp.dot(c, w2_ref[...], preferred_element_type=f32) + b2_ref[...]
    out_ref[...] = v0_ref[...] * coeff


def _final(hsum, v0, w1, b1, w2, b2):
    return pl.pallas_call(
        _final_body,
        out_shape=jax.ShapeDtypeStruct((1, DH), f32),
    )(hsum, v0, w1, b1, w2, b2)


@functools.cache
def _mesh():
    return plsc.VectorSubcoreMesh(core_axis_name="c", subcore_axis_name="s",
                                  num_cores=NC, num_subcores=NS)


NPAIR = (NFULL - 1) // 2


def _sc_gather_body(a_hbm, b_hbm, dst_hbm, src_hbm, oa_hbm,
                    idxd0, idxs0, idxd1, idxs1, ra0, rb0, ra1, rb1, sem_g, sem_w):
    cid = lax.axis_index("c")
    sid = lax.axis_index("s")
    wid = sid * NC + cid
    base = wid * EPW
    sets = ((idxd0, idxs0, ra0, rb0, 0), (idxd1, idxs1, ra1, rb1, 1))

    def start(ci, s):
        idxd, idxs, ra, rb, b = s
        off = base + ci * CHUNK
        pltpu.sync_copy(dst_hbm.at[pl.ds(off, CHUNK)], idxd)
        pltpu.sync_copy(src_hbm.at[pl.ds(off, CHUNK)], idxs)
        pltpu.async_copy(a_hbm.at[idxd], ra, sem_g.at[b, 0])
        pltpu.async_copy(b_hbm.at[idxs], rb, sem_g.at[b, 1])

    def wait_gather(s):
        idxd, idxs, ra, rb, b = s
        pltpu.make_async_copy(a_hbm.at[idxd], ra, sem_g.at[b, 0]).wait()
        pltpu.make_async_copy(b_hbm.at[idxs], rb, sem_g.at[b, 1]).wait()

    def add_rows(s, k):
        _, _, ra, rb, _ = s

        @functools.partial(plsc.parallel_loop, 0, k, unroll=8)
        def _(r):
            for j in range(DP // 16):
                sl = (r, pl.ds(j * 16, 16))
                ra[sl] = ra[sl] + rb[sl]

    def wstart(ci, s):
        _, _, ra, _, b = s
        off = pl.ds(base + ci * CHUNK, CHUNK)
        pltpu.async_copy(ra, oa_hbm.at[off], sem_w.at[b])

    def wwait(ci, s):
        _, _, ra, _, b = s
        off = pl.ds(base + ci * CHUNK, CHUNK)
        pltpu.make_async_copy(ra, oa_hbm.at[off], sem_w.at[b]).wait()

    start(0, sets[0])
    start(1, sets[1])

    def body(p, carry):
        c0 = 2 * p
        wait_gather(sets[0])
        add_rows(sets[0], CHUNK)
        wstart(c0, sets[0])
        wait_gather(sets[1])
        add_rows(sets[1], CHUNK)
        wstart(c0 + 1, sets[1])

        @pl.when(p < NPAIR - 1)
        def _():
            wwait(c0, sets[0])
            start(c0 + 2, sets[0])
            wwait(c0 + 1, sets[1])
            start(c0 + 3, sets[1])

        return carry

    lax.fori_loop(0, NPAIR, body, 0)
    last0 = 2 * (NPAIR - 1)
    wwait(last0, sets[0])
    wwait(last0 + 1, sets[1])

    start(NFULL - 1, sets[0])
    wait_gather(sets[0])
    add_rows(sets[0], CHUNK)
    off = pl.ds(base + (NFULL - 1) * CHUNK, CHUNK)
    pltpu.sync_copy(ra0, oa_hbm.at[off])

    if TAIL:
        offt = base + NFULL * CHUNK
        id_d = idxd1.at[pl.ds(0, TAIL)]
        id_s = idxs1.at[pl.ds(0, TAIL)]
        pltpu.sync_copy(dst_hbm.at[pl.ds(offt, TAIL)], id_d)
        pltpu.sync_copy(src_hbm.at[pl.ds(offt, TAIL)], id_s)
        pltpu.async_copy(a_hbm.at[id_d], ra1.at[pl.ds(0, TAIL)], sem_g.at[1, 0]).wait()
        pltpu.async_copy(b_hbm.at[id_s], rb1.at[pl.ds(0, TAIL)], sem_g.at[1, 1]).wait()
        add_rows(sets[1], TAIL)
        pltpu.sync_copy(ra1.at[pl.ds(0, TAIL)], oa_hbm.at[pl.ds(offt, TAIL)])


def _gather_edges(a, b, dst, src):
    return pl.kernel(
        _sc_gather_body,
        out_type=jax.ShapeDtypeStruct((N_EDGES, DP), f32),
        mesh=_mesh(),
        scratch_types=[
            pltpu.VMEM((CHUNK,), jnp.int32),
            pltpu.VMEM((CHUNK,), jnp.int32),
            pltpu.VMEM((CHUNK,), jnp.int32),
            pltpu.VMEM((CHUNK,), jnp.int32),
            pltpu.VMEM((CHUNK, DP), f32),
            pltpu.VMEM((CHUNK, DP), f32),
            pltpu.VMEM((CHUNK, DP), f32),
            pltpu.VMEM((CHUNK, DP), f32),
            pltpu.SemaphoreType.DMA((2, 2)),
            pltpu.SemaphoreType.DMA((2,)),
        ],
    )(a, b, dst, src)


def _sc_scatter_body(msg_hbm, dst_hbm, out_hbm, idx, idx_t, rows, zbuf, shared,
                     sem_r, sem_s):
    cid = lax.axis_index("c")
    sid = lax.axis_index("s")
    base = sid * EPT
    nbase = cid * OWN

    zvec = jnp.zeros((16,), f32)

    def zrow(r, carry):
        for j in range(DP // 16):
            zbuf[r, pl.ds(j * 16, 16)] = zvec
        return carry

    lax.fori_loop(0, TSTRIPE, zrow, 0)
    pltpu.sync_copy(zbuf, shared.at[pl.ds(sid * TSTRIPE, TSTRIPE)])
    plsc.subcore_barrier()

    def remap(id_buf, b, k):
        for j in range(k // 16):
            sl = (b, pl.ds(j * 16, 16))
            local = id_buf[sl] - nbase
            ok = (local >= 0) & (local < OWN)
            id_buf[sl] = jnp.where(ok, local, TRASH)

    def start_chunk(ci, b):
        off = base + ci * CHUNK
        pltpu.sync_copy(dst_hbm.at[pl.ds(off, CHUNK)], idx.at[b])
        pltpu.async_copy(msg_hbm.at[pl.ds(off, CHUNK)], rows.at[b], sem_r.at[b])
        remap(idx, b, CHUNK)

    start_chunk(0, 0)

    def body(i, carry):
        b = lax.rem(i, 2)
        nb = 1 - b

        @pl.when(i + 1 < NF2)
        def _():
            @pl.when(i >= 1)
            def _():
                pltpu.make_async_copy(rows.at[nb], shared.at[idx.at[nb]],
                                      sem_s.at[nb]).wait()

            start_chunk(i + 1, nb)

        pltpu.make_async_copy(msg_hbm.at[pl.ds(0, CHUNK)], rows.at[b],
                              sem_r.at[b]).wait()
        pltpu.async_copy(rows.at[b], shared.at[idx.at[b]], sem_s.at[b], add=True)
        return carry

    lax.fori_loop(0, NF2, body, 0)
    pb = (NF2 - 1) % 2
    pltpu.make_async_copy(rows.at[pb], shared.at[idx.at[pb]], sem_s.at[pb]).wait()
    pltpu.make_async_copy(rows.at[1 - pb], shared.at[idx.at[1 - pb]],
                          sem_s.at[1 - pb]).wait()

    if TAIL2:
        off = base + NF2 * CHUNK
        pltpu.sync_copy(dst_hbm.at[pl.ds(off, TAIL2)], idx_t)
        pltpu.sync_copy(msg_hbm.at[pl.ds(off, TAIL2)], rows.at[0, pl.ds(0, TAIL2)])
        for j in range(TAIL2 // 16):
            sl = pl.ds(j * 16, 16)
            local = idx_t[sl] - nbase
            ok = (local >= 0) & (local < OWN)
            idx_t[sl] = jnp.where(ok, local, TRASH)
        pltpu.sync_copy(rows.at[0, pl.ds(0, TAIL2)], shared.at[idx_t], add=True)

    plsc.subcore_barrier()
    pltpu.sync_copy(shared.at[pl.ds(sid * TSTRIPE, TSTRIPE)],
                    out_hbm.at[cid, pl.ds(sid * TSTRIPE, TSTRIPE)])


def _scatter_msgs(msg, dst):
    return pl.kernel(
        _sc_scatter_body,
        out_type=jax.ShapeDtypeStruct((NC, TROWS, DP), f32),
        mesh=_mesh(),
        scratch_types=[
            pltpu.VMEM((2, CHUNK), jnp.int32),
            pltpu.VMEM((TAIL2,), jnp.int32),
            pltpu.VMEM((2, CHUNK, DP), f32),
            pltpu.VMEM((TSTRIPE, DP), f32),
            pltpu.VMEM_SHARED((TROWS, DP), f32),
            pltpu.SemaphoreType.DMA((2,)),
            pltpu.SemaphoreType.DMA((2,)),
        ],
    )(msg, dst)


def _msg_weights(mp):
    w1 = mp["l1"]["w"]
    wa = _pad2(w1[:, :D_FEAT].T, D_FEAT, DP)
    wb = _pad2(w1[:, D_FEAT:].T, DH, DP)
    b1 = _pad1(mp["l1"]["b"], DP)
    out = {"wa": wa, "wb": wb, "b1": b1}
    for k in ("2", "3"):
        out["w" + k] = _pad2(mp["l" + k]["w"].T, DP, DP).astype(bf16)
        out["b" + k] = _pad1(mp["l" + k]["b"], DP)
        out["g" + k] = _pad1(mp["bn" + k]["gamma"], DP)
        out["be" + k] = _pad1(mp["bn" + k]["beta"], DP)
    out["g1"] = _pad1(mp["bn1"]["gamma"], DP)
    out["be1"] = _pad1(mp["bn1"]["beta"], DP)
    out["w4"] = _pad2(mp["l4"]["w"].T, DP, DP).astype(bf16)
    out["b4"] = _pad1(mp["l4"]["b"], DP)
    return out


def _gnn_layer(x, v, dst, src, lp):
    mw = _msg_weights(lp["msg"])
    a, b = _node_pre(x, mw["wa"], mw["wb"], mw["b1"])
    e1 = _gather_edges(a, b, dst, src)
    gs = jnp.stack([mw["g1"], mw["g2"], mw["g3"]])
    bes = jnp.stack([mw["be1"], mw["be2"], mw["be3"]])
    ws = jnp.stack([mw["w2"], mw["w3"], mw["w4"]])
    bs = jnp.stack([mw["b2"], mw["b3"], mw["b4"]])
    msg = _edge_mlp(e1, gs, bes, ws, bs)
    aggs = _scatter_msgs(msg, dst)

    up = lp["upd"]
    w1u = up["l1"]["w"]
    wv = w1u[:, :DH].T
    wx = w1u[:, DH:DH + D_FEAT].T
    wa = _pad2(w1u[:, DH + D_FEAT:].T, DP, DH)
    return _update(
        x, v, aggs,
        wv, wx, wa, _pad1(up["l1"]["b"], DH),
        up["l2"]["w"].T, _pad1(up["l2"]["b"], DH),
        up["l3"]["w"].T, _pad1(up["l3"]["b"], DH),
        up["l4"]["w"].T, _pad1(up["l4"]["b"], D_FEAT),
    )


def kernel(node_feature, edge_index, vectors, params):
    x0 = node_feature[0]
    src = edge_index[0, 0]
    dst = edge_index[0, 1]
    v = x0[:, :DH]
    stacked = jax.tree.map(lambda *a: jnp.stack(a), *params["layers"])

    def _layer_step(xc, lp):
        return _gnn_layer(xc, v, dst, src, lp), None

    x, _ = lax.scan(_layer_step, x0, stacked)

    pp = params["pre"]
    hsum = _pre_pool(
        x,
        pp["l1"]["w"].T, _pad1(pp["l1"]["b"], DH),
        pp["l2"]["w"].T, _pad1(pp["l2"]["b"], DH),
        pp["l3"]["w"].T, _pad1(pp["l3"]["b"], DH),
        pp["l4"]["w"].T, _pad1(pp["l4"]["b"], DH),
    )
    qp = params["post"]
    out = _final(hsum, x0[0:1, :DH],
                 qp["l1"]["w"].T, _pad1(qp["l1"]["b"], DH),
                 qp["l2"]["w"].T, _pad1(qp["l2"]["b"], DH))
    return out.reshape((DH,))


# speedup vs baseline: 2.7924x; 1.0024x over previous
import functools

import jax
import jax.numpy as jnp
from jax import lax
from jax.experimental import pallas as pl
from jax.experimental.pallas import tpu as pltpu
from jax.experimental.pallas import tpu_sc as plsc

N_NODES = 10000
N_EDGES = 160000
D_FEAT = 300
DH = 100
DP = 128
DP2 = 64
NC, NS = 2, 16
NW = NC * NS
EPW = N_EDGES // NW
CHUNK = 128
NFULL = EPW // CHUNK
TAIL = EPW - NFULL * CHUNK
OWN = 6000
TROWS = 6016
TSTRIPE = TROWS // NS
TRASH = TROWS - 1
EPT = N_EDGES // NS
NF2 = EPT // CHUNK
TAIL2 = EPT - NF2 * CHUNK
BE = 4000
BN = 2000
EPS = 1e-5

f32 = jnp.float32
bf16 = jnp.bfloat16


def _swish(x):
    return x * lax.logistic(x)


def _pad2(w, r, c):
    return jnp.zeros((r, c), f32).at[: w.shape[0], : w.shape[1]].set(w)


def _pad1(b, n):
    return jnp.zeros((1, n), f32).at[0, : b.shape[0]].set(b)


def _node_pre_body(x_ref, wa_ref, wb_ref, b_ref, a_ref, bb_ref):
    xb = x_ref[...]
    a_ref[...] = jnp.dot(xb, wa_ref[...], preferred_element_type=f32) + b_ref[...]
    bb_ref[...] = jnp.dot(xb[:, :DH], wb_ref[...], preferred_element_type=f32)


def _node_pre(x, wa, wb, b):
    g = N_NODES // BN
    return pl.pallas_call(
        _node_pre_body,
        grid=(g,),
        in_specs=[
            pl.BlockSpec((BN, D_FEAT), lambda i: (i, 0)),
            pl.BlockSpec((D_FEAT, DP), lambda i: (0, 0)),
            pl.BlockSpec((DH, DP), lambda i: (0, 0)),
            pl.BlockSpec((1, DP), lambda i: (0, 0)),
        ],
        out_specs=[
            pl.BlockSpec((BN, DP), lambda i: (i, 0)),
            pl.BlockSpec((BN, DP), lambda i: (i, 0)),
        ],
        out_shape=[
            jax.ShapeDtypeStruct((N_NODES, DP), f32),
            jax.ShapeDtypeStruct((N_NODES, DP), f32),
        ],
    )(x, wa, wb, b)


NBLK = N_EDGES // BE


def _edge_mlp_body(ea_ref, g_ref, be_ref, w_ref, b_ref, out_ref,
                   scr, sums, sqs):
    s = pl.program_id(0)
    b = pl.program_id(1)

    def accum(si, sv16):
        ps = jnp.sum(sv16.astype(f32), axis=0, keepdims=True)
        pq = jnp.sum((sv16 * sv16).astype(f32), axis=0, keepdims=True)

        @pl.when(b == 0)
        def _():
            sums[si] = ps
            sqs[si] = pq

        @pl.when(b != 0)
        def _():
            sums[si] = sums[si] + ps
            sqs[si] = sqs[si] + pq

    @pl.when(s == 0)
    def _():
        e16 = ea_ref[...].astype(bf16)
        s1 = e16 * lax.logistic(e16)
        scr[b] = s1
        accum(0, s1)

    @pl.when(s > 0)
    def _():
        sm1 = s - 1
        m = sums[sm1] * (1.0 / N_EDGES)
        var = sqs[sm1] * (1.0 / N_EDGES) - m * m
        scale = lax.rsqrt(var + EPS) * g_ref[0]
        shift = be_ref[0] - m * scale
        hn = scr[b] * scale.astype(bf16) + shift.astype(bf16)
        z = jnp.dot(hn, w_ref[0], preferred_element_type=f32) + b_ref[0]
        z16 = z.astype(bf16)
        sv16 = z16 * lax.logistic(z16)

        @pl.when(s < 3)
        def _():
            scr[b] = sv16
            accum(s, sv16)

        @pl.when(s == 3)
        def _():
            sv = sv16.astype(f32)
            col = lax.broadcasted_iota(jnp.int32, sv.shape, 1)
            out_ref[...] = jnp.where(col == DP - 1, 1.0, sv)


def _edge_mlp(ea, gs, bes, ws, bs):
    emap = lambda s, b: (jnp.where(s == 0, b, 0), 0)
    wm = lambda s, b: (jnp.maximum(s - 1, 0), 0, 0)
    return pl.pallas_call(
        _edge_mlp_body,
        grid=(4, NBLK),
        in_specs=[
            pl.BlockSpec((BE, DP), emap),
            pl.BlockSpec((1, 1, DP), wm),
            pl.BlockSpec((1, 1, DP), wm),
            pl.BlockSpec((1, DP, DP), wm),
            pl.BlockSpec((1, 1, DP), wm),
        ],
        out_specs=[pl.BlockSpec((BE, DP), lambda s, b: (jnp.where(s == 3, b, 0), 0))],
        out_shape=[jax.ShapeDtypeStruct((N_EDGES, DP), f32)],
        scratch_shapes=[
            pltpu.VMEM((NBLK, BE, DP), bf16),
            pltpu.VMEM((3, 1, DP), f32),
            pltpu.VMEM((3, 1, DP), f32),
        ],
        compiler_params=pltpu.CompilerParams(vmem_limit_bytes=100 * 1024 * 1024),
    )(ea, gs, bes, ws, bs)[0]


def _s1_body(ea_ref, eb_ref, sum_ref, sq_ref):
    i = pl.program_id(0)
    s = _swish(ea_ref[...].astype(f32) + eb_ref[...].astype(f32))
    ps = jnp.sum(s, axis=0, keepdims=True)
    pq = jnp.sum(s * s, axis=0, keepdims=True)

    @pl.when(i == 0)
    def _():
        sum_ref[...] = ps
        sq_ref[...] = pq

    @pl.when(i != 0)
    def _():
        sum_ref[...] = sum_ref[...] + ps
        sq_ref[...] = sq_ref[...] + pq


def _s1_stats(ea, eb):
    g = N_EDGES // BE
    return pl.pallas_call(
        _s1_body,
        grid=(g,),
        in_specs=[pl.BlockSpec((BE, DP), lambda i: (i, 0)),
                  pl.BlockSpec((BE, DP), lambda i: (i, 0))],
        out_specs=[
            pl.BlockSpec((1, DP), lambda i: (0, 0)),
            pl.BlockSpec((1, DP), lambda i: (0, 0)),
        ],
        out_shape=[
            jax.ShapeDtypeStruct((1, DP), f32),
            jax.ShapeDtypeStruct((1, DP), f32),
        ],
    )(ea, eb)


def _stage(xs, stats, gamma, beta, w, b, *, pre_swish, track_stats, ones_col=False):
    g = N_EDGES // BE
    one = pl.BlockSpec((1, DP), lambda i: (0, 0))
    n_in = len(xs)

    def body(*refs):
        in_refs = refs[:n_in]
        sum_ref, sq_ref, g_ref, be_ref, w_ref, b_ref = refs[n_in:n_in + 6]
        outs = refs[n_in + 6:]
        i = pl.program_id(0)
        h = in_refs[0][...].astype(f32)
        for r in in_refs[1:]:
            h = h + r[...].astype(f32)
        if pre_swish:
            h = _swish(h)
        m = sum_ref[...] * (1.0 / N_EDGES)
        var = sq_ref[...] * (1.0 / N_EDGES) - m * m
        h = (h - m) * lax.rsqrt(var + EPS) * g_ref[...] + be_ref[...]
        z = jnp.dot(h.astype(bf16), w_ref[...], preferred_element_type=f32) + b_ref[...]
        s = _swish(z)
        if ones_col:
            col = lax.broadcasted_iota(jnp.int32, s.shape, 1)
            s = jnp.where(col == DP - 1, 1.0, s)
        outs[0][...] = s if ones_col else s.astype(bf16)
        if track_stats:
            ps = jnp.sum(s, axis=0, keepdims=True)
            pq = jnp.sum(s * s, axis=0, keepdims=True)

            @pl.when(i == 0)
            def _():
                outs[1][...] = ps
                outs[2][...] = pq

            @pl.when(i != 0)
            def _():
                outs[1][...] = outs[1][...] + ps
                outs[2][...] = outs[2][...] + pq

    out_specs = [pl.BlockSpec((BE, DP), lambda i: (i, 0)), one, one]
    out_shape = [
        jax.ShapeDtypeStruct((N_EDGES, DP), f32 if ones_col else bf16),
        jax.ShapeDtypeStruct((1, DP), f32),
        jax.ShapeDtypeStruct((1, DP), f32),
    ]
    if not track_stats:
        out_specs, out_shape = out_specs[:1], out_shape[:1]
    return pl.pallas_call(
        body,
        grid=(g,),
        in_specs=[pl.BlockSpec((BE, DP), lambda i: (i, 0))] * n_in
                 + [one, one, one, one,
                    pl.BlockSpec((DP, DP), lambda i: (0, 0)), one],
        out_specs=out_specs,
        out_shape=out_shape,
    )(*xs, stats[0], stats[1], gamma, beta, w, b)


def _update_body(x_ref, v_ref, a_ref, wv_ref, wx_ref, wa_ref, b1_ref,
                 w2_ref, b2_ref, w3_ref, b3_ref, w4_ref, b4_ref, out_ref):
    xb = x_ref[...]
    acc = a_ref[0]
    cnt = jnp.maximum(acc[:, DP - 1 : DP], 1.0)
    agg = acc / cnt
    h = (jnp.dot(v_ref[...], wv_ref[...], preferred_element_type=f32)
         + jnp.dot(xb, wx_ref[...], preferred_element_type=f32)
         + jnp.dot(agg, wa_ref[...], preferred_element_type=f32)
         + b1_ref[...])
    h = _swish(h)
    h = _swish(jnp.dot(h, w2_ref[...], preferred_element_type=f32) + b2_ref[...])
    h = _swish(jnp.dot(h, w3_ref[...], preferred_element_type=f32) + b3_ref[...])
    upd = _swish(jnp.dot(h, w4_ref[...], preferred_element_type=f32) + b4_ref[...])
    out_ref[...] = xb + upd


def _update(x, v, aggs, wv, wx, wa, b1, w2, b2, w3, b3, w4, b4):
    g = N_NODES // BN
    oneh = pl.BlockSpec((1, DH), lambda i: (0, 0))
    return pl.pallas_call(
        _update_body,
        grid=(g,),
        in_specs=[
            pl.BlockSpec((BN, D_FEAT), lambda i: (i, 0)),
            pl.BlockSpec((BN, DH), lambda i: (i, 0)),
            pl.BlockSpec((1, BN, DP), lambda i: (i // 3, i - 3 * (i // 3), 0)),
            pl.BlockSpec((DH, DH), lambda i: (0, 0)),
            pl.BlockSpec((D_FEAT, DH), lambda i: (0, 0)),
            pl.BlockSpec((DP, DH), lambda i: (0, 0)),
            oneh,
            pl.BlockSpec((DH, DH), lambda i: (0, 0)), oneh,
            pl.BlockSpec((DH, DH), lambda i: (0, 0)), oneh,
            pl.BlockSpec((DH, D_FEAT), lambda i: (0, 0)),
            pl.BlockSpec((1, D_FEAT), lambda i: (0, 0)),
        ],
        out_specs=[pl.BlockSpec((BN, D_FEAT), lambda i: (i, 0))],
        out_shape=[jax.ShapeDtypeStruct((N_NODES, D_FEAT), f32)],
        input_output_aliases={0: 0},
    )(x, v, aggs, wv, wx, wa, b1, w2, b2, w3, b3, w4, b4)[0]


def _pre_pool_body(x_ref, w1_ref, b1_ref, w2_ref, b2_ref, w3_ref, b3_ref,
                   w4_ref, b4_ref, sum_ref):
    i = pl.program_id(0)
    h = _swish(jnp.dot(x_ref[...], w1_ref[...], preferred_element_type=f32) + b1_ref[...])
    h = _swish(jnp.dot(h, w2_ref[...], preferred_element_type=f32) + b2_ref[...])
    h = _swish(jnp.dot(h, w3_ref[...], preferred_element_type=f32) + b3_ref[...])
    h = jnp.dot(h, w4_ref[...], preferred_element_type=f32) + b4_ref[...]
    ps = jnp.sum(h, axis=0, keepdims=True)

    @pl.when(i == 0)
    def _():
        sum_ref[...] = ps

    @pl.when(i != 0)
    def _():
        sum_ref[...] = sum_ref[...] + ps


def _pre_pool(x, w1, b1, w2, b2, w3, b3, w4, b4):
    g = N_NODES // BN
    oneh = pl.BlockSpec((1, DH), lambda i: (0, 0))
    return pl.pallas_call(
        _pre_pool_body,
        grid=(g,),
        in_specs=[
            pl.BlockSpec((BN, D_FEAT), lambda i: (i, 0)),
            pl.BlockSpec((D_FEAT, DH), lambda i: (0, 0)), oneh,
            pl.BlockSpec((DH, DH), lambda i: (0, 0)), oneh,
            pl.BlockSpec((DH, DH), lambda i: (0, 0)), oneh,
            pl.BlockSpec((DH, DH), lambda i: (0, 0)), oneh,
        ],
        out_specs=[oneh],
        out_shape=[jax.ShapeDtypeStruct((1, DH), f32)],
    )(x, w1, b1, w2, b2, w3, b3, w4, b4)[0]


def _final_body(hsum_ref, v0_ref, w1_ref, b1_ref, w2_ref, b2_ref, out_ref):
    pooled = hsum_ref[...] * (1.0 / N_NODES)
    c = _swish(jnp.dot(pooled, w1_ref[...], preferred_element_type=f32) + b1_ref[...])
    coeff = jnp.dot(c, w2_ref[...], preferred_element_type=f32) + b2_ref[...]
    out_ref[...] = v0_ref[...] * coeff


def _final(hsum, v0, w1, b1, w2, b2):
    return pl.pallas_call(
        _final_body,
        out_shape=jax.ShapeDtypeStruct((1, DH), f32),
    )(hsum, v0, w1, b1, w2, b2)


@functools.cache
def _mesh():
    return plsc.VectorSubcoreMesh(core_axis_name="c", subcore_axis_name="s",
                                  num_cores=NC, num_subcores=NS)


NROWS = N_EDGES // CHUNK
GROWS = NROWS // NW
GPAIR = GROWS // 2


def _sc_gather_body(a_hbm, b_hbm, d2_hbm, s2_hbm, oa_hbm,
                    idxd0, idxs0, idxd1, idxs1, ra0, rb0, ra1, rb1, sem_g, sem_w):
    cid = lax.axis_index("c")
    sid = lax.axis_index("s")
    wid = sid * NC + cid
    rbase = wid * GROWS + jnp.minimum(wid, 2)
    sets = ((idxd0, idxs0, ra0, rb0, 0), (idxd1, idxs1, ra1, rb1, 1))

    def start(gr, s):
        idxd, idxs, ra, rb, b = s
        pltpu.sync_copy(d2_hbm.at[pl.ds(gr, 1)], idxd)
        pltpu.sync_copy(s2_hbm.at[pl.ds(gr, 1)], idxs)
        pltpu.async_copy(a_hbm.at[idxd.at[0]], ra, sem_g.at[b, 0])
        pltpu.async_copy(b_hbm.at[idxs.at[0]], rb, sem_g.at[b, 1])

    def wait_gather(s):
        idxd, idxs, ra, rb, b = s
        pltpu.make_async_copy(a_hbm.at[idxd.at[0]], ra, sem_g.at[b, 0]).wait()
        pltpu.make_async_copy(b_hbm.at[idxs.at[0]], rb, sem_g.at[b, 1]).wait()

    def add_rows(s):
        _, _, ra, rb, _ = s

        @functools.partial(plsc.parallel_loop, 0, CHUNK, unroll=8)
        def _(r):
            for j in range(DP // 16):
                sl = (r, pl.ds(j * 16, 16))
                ra[sl] = ra[sl] + rb[sl]

    def wstart(gr, s):
        _, _, ra, _, b = s
        pltpu.async_copy(ra, oa_hbm.at[pl.ds(gr * CHUNK, CHUNK)], sem_w.at[b])

    def wwait(gr, s):
        _, _, ra, _, b = s
        pltpu.make_async_copy(ra, oa_hbm.at[pl.ds(gr * CHUNK, CHUNK)],
                              sem_w.at[b]).wait()

    start(rbase, sets[0])
    start(rbase + 1, sets[1])

    def body(p, carry):
        g0 = rbase + 2 * p
        wait_gather(sets[0])
        add_rows(sets[0])
        wstart(g0, sets[0])
        wait_gather(sets[1])
        add_rows(sets[1])
        wstart(g0 + 1, sets[1])

        @pl.when(p < GPAIR - 1)
        def _():
            wwait(g0, sets[0])
            start(g0 + 2, sets[0])
            wwait(g0 + 1, sets[1])
            start(g0 + 3, sets[1])

        return carry

    lax.fori_loop(0, GPAIR, body, 0)
    g0 = rbase + 2 * (GPAIR - 1)
    wwait(g0, sets[0])
    wwait(g0 + 1, sets[1])

    def serial_row(gr, s):
        start(gr, s)
        wait_gather(s)
        add_rows(s)
        _, _, ra, _, b = s
        pltpu.sync_copy(ra, oa_hbm.at[pl.ds(gr * CHUNK, CHUNK)])

    serial_row(rbase + GROWS - 1, sets[0])

    @pl.when(wid < 2)
    def _():
        serial_row(rbase + GROWS, sets[1])


def _gather_edges(a, b, d2, s2):
    return pl.kernel(
        _sc_gather_body,
        out_type=jax.ShapeDtypeStruct((N_EDGES, DP), f32),
        mesh=_mesh(),
        scratch_types=[
            pltpu.VMEM((1, CHUNK), jnp.int32),
            pltpu.VMEM((1, CHUNK), jnp.int32),
            pltpu.VMEM((1, CHUNK), jnp.int32),
            pltpu.VMEM((1, CHUNK), jnp.int32),
            pltpu.VMEM((CHUNK, DP), f32),
            pltpu.VMEM((CHUNK, DP), f32),
            pltpu.VMEM((CHUNK, DP), f32),
            pltpu.VMEM((CHUNK, DP), f32),
            pltpu.SemaphoreType.DMA((2, 2)),
            pltpu.SemaphoreType.DMA((2,)),
        ],
    )(a, b, d2, s2)


def _sc_scatter_body(msg_hbm, d2_hbm, out_hbm, idx, rows, zbuf, shared,
                     sem_r, sem_s):
    cid = lax.axis_index("c")
    sid = lax.axis_index("s")
    nbase = cid * OWN
    srows = NROWS // NS
    rbase = sid * srows + jnp.minimum(sid, 2)
    nr = srows + jnp.where(sid < 2, 1, 0)

    zvec = jnp.zeros((16,), f32)

    def zrow(r, carry):
        for j in range(DP // 16):
            zbuf[r, pl.ds(j * 16, 16)] = zvec
        return carry

    lax.fori_loop(0, TSTRIPE, zrow, 0)
    pltpu.sync_copy(zbuf, shared.at[pl.ds(sid * TSTRIPE, TSTRIPE)])
    plsc.subcore_barrier()

    def start_chunk(gr, b):
        pltpu.sync_copy(d2_hbm.at[pl.ds(gr, 1)], idx.at[b])
        pltpu.async_copy(msg_hbm.at[pl.ds(gr * CHUNK, CHUNK)], rows.at[b],
                         sem_r.at[b])
        for j in range(CHUNK // 16):
            sl = (b, 0, pl.ds(j * 16, 16))
            local = idx[sl] - nbase
            ok = (local >= 0) & (local < OWN)
            idx[sl] = jnp.where(ok, local, TRASH)

    start_chunk(rbase, 0)

    def body(i, carry):
        b = lax.rem(i, 2)
        nb = 1 - b

        @pl.when(i + 1 < nr)
        def _():
            @pl.when(i >= 1)
            def _():
                pltpu.make_async_copy(rows.at[nb], shared.at[idx.at[nb, 0]],
                                      sem_s.at[nb]).wait()

            start_chunk(rbase + i + 1, nb)

        pltpu.make_async_copy(msg_hbm.at[pl.ds(0, CHUNK)], rows.at[b],
                              sem_r.at[b]).wait()
        pltpu.async_copy(rows.at[b], shared.at[idx.at[b, 0]], sem_s.at[b], add=True)
        return carry

    lax.fori_loop(0, nr, body, 0)
    pltpu.make_async_copy(rows.at[0], shared.at[idx.at[0, 0]], sem_s.at[0]).wait()
    pltpu.make_async_copy(rows.at[1], shared.at[idx.at[1, 0]], sem_s.at[1]).wait()

    plsc.subcore_barrier()
    pltpu.sync_copy(shared.at[pl.ds(sid * TSTRIPE, TSTRIPE)],
                    out_hbm.at[cid, pl.ds(sid * TSTRIPE, TSTRIPE)])


def _scatter_msgs(msg, d2):
    return pl.kernel(
        _sc_scatter_body,
        out_type=jax.ShapeDtypeStruct((NC, TROWS, DP), f32),
        mesh=_mesh(),
        scratch_types=[
            pltpu.VMEM((2, 1, CHUNK), jnp.int32),
            pltpu.VMEM((2, CHUNK, DP), f32),
            pltpu.VMEM((TSTRIPE, DP), f32),
            pltpu.VMEM_SHARED((TROWS, DP), f32),
            pltpu.SemaphoreType.DMA((2,)),
            pltpu.SemaphoreType.DMA((2,)),
        ],
    )(msg, d2)


def _msg_weights(mp):
    w1 = mp["l1"]["w"]
    wa = _pad2(w1[:, :D_FEAT].T, D_FEAT, DP)
    wb = _pad2(w1[:, D_FEAT:].T, DH, DP)
    b1 = _pad1(mp["l1"]["b"], DP)
    out = {"wa": wa, "wb": wb, "b1": b1}
    for k in ("2", "3"):
        out["w" + k] = _pad2(mp["l" + k]["w"].T, DP, DP).astype(bf16)
        out["b" + k] = _pad1(mp["l" + k]["b"], DP)
        out["g" + k] = _pad1(mp["bn" + k]["gamma"], DP)
        out["be" + k] = _pad1(mp["bn" + k]["beta"], DP)
    out["g1"] = _pad1(mp["bn1"]["gamma"], DP)
    out["be1"] = _pad1(mp["bn1"]["beta"], DP)
    out["w4"] = _pad2(mp["l4"]["w"].T, DP, DP).astype(bf16)
    out["b4"] = _pad1(mp["l4"]["b"], DP)
    return out


def _gnn_layer(x, v, d2, s2, lp):
    mw = _msg_weights(lp["msg"])
    a, b = _node_pre(x, mw["wa"], mw["wb"], mw["b1"])
    e1 = _gather_edges(a, b, d2, s2)
    gs = jnp.stack([mw["g1"], mw["g2"], mw["g3"]])
    bes = jnp.stack([mw["be1"], mw["be2"], mw["be3"]])
    ws = jnp.stack([mw["w2"], mw["w3"], mw["w4"]])
    bs = jnp.stack([mw["b2"], mw["b3"], mw["b4"]])
    msg = _edge_mlp(e1, gs, bes, ws, bs)
    aggs = _scatter_msgs(msg, d2)

    up = lp["upd"]
    w1u = up["l1"]["w"]
    wv = w1u[:, :DH].T
    wx = w1u[:, DH:DH + D_FEAT].T
    wa = _pad2(w1u[:, DH + D_FEAT:].T, DP, DH)
    return _update(
        x, v, aggs,
        wv, wx, wa, _pad1(up["l1"]["b"], DH),
        up["l2"]["w"].T, _pad1(up["l2"]["b"], DH),
        up["l3"]["w"].T, _pad1(up["l3"]["b"], DH),
        up["l4"]["w"].T, _pad1(up["l4"]["b"], D_FEAT),
    )


def kernel(node_feature, edge_index, vectors, params):
    x0 = node_feature[0]
    d2 = edge_index[0, 1].reshape(N_EDGES // CHUNK, CHUNK)
    s2 = edge_index[0, 0].reshape(N_EDGES // CHUNK, CHUNK)
    v = x0[:, :DH]
    stacked = jax.tree.map(lambda *a: jnp.stack(a), *params["layers"])

    def _layer_step(xc, lp):
        return _gnn_layer(xc, v, d2, s2, lp), None

    x, _ = lax.scan(_layer_step, x0, stacked)

    pp = params["pre"]
    hsum = _pre_pool(
        x,
        pp["l1"]["w"].T, _pad1(pp["l1"]["b"], DH),
        pp["l2"]["w"].T, _pad1(pp["l2"]["b"], DH),
        pp["l3"]["w"].T, _pad1(pp["l3"]["b"], DH),
        pp["l4"]["w"].T, _pad1(pp["l4"]["b"], DH),
    )
    qp = params["post"]
    out = _final(hsum, x0[0:1, :DH],
                 qp["l1"]["w"].T, _pad1(qp["l1"]["b"], DH),
                 qp["l2"]["w"].T, _pad1(qp["l2"]["b"], DH))
    return out.reshape((DH,))
